# baseline (speedup 1.0000x reference)
import jax
import jax.numpy as jnp
import numpy as np
from jax.experimental import pallas as pl

EMB = 128
NUM_BLOCKS = 3
NUM_RADIAL = 6
CUTOFF = 5.0
NUM_ABF = 7
N_ATOMS = 10000
N_EDGES = 160000
N_TRIPLETS = 320000
N_MOL = 512


def _swish(x):
    return x * jax.nn.sigmoid(x)


def _chebyshev(c, K=NUM_ABF):
    polys = [jnp.ones_like(c), c]
    for _ in range(K - 2):
        polys.append(2.0 * c * polys[-1] - polys[-2])
    return jnp.stack(polys[:K], axis=-1)


def _combine_kernel(a_ref, b_ref, cm_ref, cs_ref, o_ref):
    o_ref[...] = cm_ref[0] * a_ref[...] + cs_ref[0] * b_ref[...]


def kernel(Z, R, batch_seg, idx_i, idx_j, idx_kj, idx_ji, cosine_ijk, params):
    p = params
    Ri = jnp.take(R, idx_i, axis=0)
    Rj = jnp.take(R, idx_j, axis=0)
    d = jnp.sqrt(jnp.sum((Ri - Rj) ** 2, axis=-1) + 1e-12)
    n = jnp.arange(1, NUM_RADIAL + 1, dtype=jnp.float32)
    rbf = jnp.sqrt(2.0 / CUTOFF) * jnp.sin(n[None, :] * jnp.pi * d[:, None] / CUTOFF) / (d[:, None] + 1e-6)
    abf = _chebyshev(cosine_ijk)
    h = jnp.take(p["emb_table"], Z, axis=0)
    rbf_e = rbf @ p["emb_W_rbf"]
    x = _swish(jnp.concatenate([jnp.take(h, idx_i, axis=0), jnp.take(h, idx_j, axis=0), rbf_e], axis=-1) @ p["emb_W"] + p["emb_b"])
    x_0 = h

    def out_block(i, xe):
        t = xe * (rbf @ p["out_W_rbf"][i])
        a = jax.ops.segment_sum(t, idx_i, num_segments=N_ATOMS)
        return _swish(a @ p["out_W1"][i]) @ p["out_W2"][i]

    def sbody(i, ha):
        return _swish(ha @ p["sb_W"][i] + p["sb_b"][i])

    res_output = out_block(0, x)
    res_single = sbody(0, x_0)
    last_x = x
    for i in range(NUM_BLOCKS):
        g = rbf @ p["int_W_rbf"][i]
        m = jnp.take(last_x * g, idx_kj, axis=0) * (abf @ p["int_W_abf"][i])
        agg = jax.ops.segment_sum(m, idx_ji, num_segments=N_EDGES)
        xn = _swish((last_x + agg) @ p["int_W1"][i] + p["int_b1"][i])
        xn = last_x + _swish(xn @ p["int_W2"][i] + p["int_b2"][i])
        res_output = res_output + out_block(i + 1, xn)
        res_single = res_single + sbody(i + 1, res_single)
        last_x = xn
    res_output = jax.ops.segment_sum(res_output, batch_seg, num_segments=N_MOL)
    x_identity = jax.ops.segment_sum(res_single, batch_seg, num_segments=N_MOL)

    return pl.pallas_call(
        _combine_kernel,
        out_shape=jax.ShapeDtypeStruct((N_MOL, EMB), jnp.float32),
    )(res_output, x_identity, p["coef_mp"], p["coef_sg"])


# speedup vs baseline: 1.3296x; 1.3296x over previous
import functools

import jax
import jax.numpy as jnp
import numpy as np
from jax import lax
from jax.experimental import pallas as pl
from jax.experimental.pallas import tpu as pltpu
from jax.experimental.pallas import tpu_sc as plsc

EMB = 128
NUM_BLOCKS = 3
NUM_RADIAL = 6
CUTOFF = 5.0
NUM_ABF = 7
N_ATOMS = 10000
N_EDGES = 160000
N_TRIPLETS = 320000
N_MOL = 512

N_ATOMS_PAD = 10240
N_EDGES_PAD = 163840
N_TRI_PAD = 327680

NC = 2
NS = 16
NW = NC * NS

f32 = jnp.float32
i32 = jnp.int32

_sc_mesh = plsc.VectorSubcoreMesh(core_axis_name="c", subcore_axis_name="s")


def _bc16(x):
    return jax.lax.broadcast_in_dim(x, (16,), ())


def _swish(x):
    return x * jax.nn.sigmoid(x)


def _sc_h_gather_body(tab_h, z_h, out_h, idx_v, rows_v, sem):
    wid = lax.axis_index("s") * NC + lax.axis_index("c")
    bpw = 512

    @pl.when(wid < 20)
    def _():
        base = wid * bpw
        pltpu.sync_copy(z_h.at[pl.ds(base, bpw)], idx_v)
        pltpu.async_copy(tab_h.at[idx_v], rows_v, sem).wait()
        pltpu.sync_copy(rows_v, out_h.at[pl.ds(base, bpw)])


def _sc_h_gather(tab, zp):
    k = pl.kernel(
        _sc_h_gather_body,
        out_type=jax.ShapeDtypeStruct((N_ATOMS_PAD, EMB), f32),
        mesh=_sc_mesh,
        compiler_params=pltpu.CompilerParams(needs_layout_passes=False),
        scratch_types=[
            pltpu.VMEM((512,), i32),
            pltpu.VMEM((512, EMB), f32),
            pltpu.SemaphoreType.DMA,
        ],
    )
    return k(tab, zp)


_EG_B = 256


def _sc_edge_gather_body(hw1_h, hw2_h, ii_h, jj_h, gi_h, gj_h,
                         iv, jv, ba, bb, s0, s1):
    wid = lax.axis_index("s") * NC + lax.axis_index("c")
    bpw = N_EDGES_PAD // NW
    base = wid * bpw

    def body(k, _):
        off = k * _EG_B
        pltpu.sync_copy(ii_h.at[pl.ds(base + off, _EG_B)], iv)
        pltpu.sync_copy(jj_h.at[pl.ds(base + off, _EG_B)], jv)
        c0 = pltpu.async_copy(hw1_h.at[iv], ba, s0)
        c1 = pltpu.async_copy(hw2_h.at[jv], bb, s1)
        c0.wait(); c1.wait()
        pltpu.sync_copy(ba, gi_h.at[pl.ds(base + off, _EG_B)])
        pltpu.sync_copy(bb, gj_h.at[pl.ds(base + off, _EG_B)])
        return 0

    lax.fori_loop(0, bpw // _EG_B, body, 0)


def _sc_edge_gather(hw1, hw2, idx_i, idx_j):
    k = pl.kernel(
        _sc_edge_gather_body,
        out_type=(
            jax.ShapeDtypeStruct((N_EDGES_PAD, EMB), f32),
            jax.ShapeDtypeStruct((N_EDGES_PAD, EMB), f32),
        ),
        mesh=_sc_mesh,
        compiler_params=pltpu.CompilerParams(needs_layout_passes=False),
        scratch_types=[
            pltpu.VMEM((_EG_B,), i32),
            pltpu.VMEM((_EG_B,), i32),
            pltpu.VMEM((_EG_B, EMB), f32),
            pltpu.VMEM((_EG_B, EMB), f32),
            pltpu.SemaphoreType.DMA,
            pltpu.SemaphoreType.DMA,
        ],
    )
    return k(hw1, hw2, idx_i, idx_j)


_D_B = 256


def _sc_edge_d_body(r128_h, ii_h, jj_h, d_h, iv, jv, rib, rjb, dbuf, s0, s1):
    wid = lax.axis_index("s") * NC + lax.axis_index("c")
    bpw = N_EDGES_PAD // NW
    base = wid * bpw

    def batch(k, _):
        off = k * _D_B
        pltpu.sync_copy(ii_h.at[pl.ds(base + off, _D_B)], iv)
        pltpu.sync_copy(jj_h.at[pl.ds(base + off, _D_B)], jv)
        c0 = pltpu.async_copy(r128_h.at[iv], rib, s0)
        c1 = pltpu.async_copy(r128_h.at[jv], rjb, s1)
        c0.wait(); c1.wait()

        def row(r, _2):
            lns = lax.broadcasted_iota(i32, (16,), 0)
            cm = jnp.where(lns < jnp.full((16,), 3, i32),
                           jnp.ones((16,), f32), jnp.zeros((16,), f32))
            l0 = lns == jnp.zeros((16,), i32)
            eps = jnp.where(l0, jnp.full((16,), 1e-12, f32),
                            jnp.zeros((16,), f32))
            va = rib[r, pl.ds(0, 16)] - rjb[r, pl.ds(0, 16)]
            sq = va * va * cm + eps
            s = jnp.sum(sq)
            plsc.store_scatter(dbuf, [_bc16(off + r)], _bc16(s), mask=l0)
            return 0
        lax.fori_loop(0, _D_B, row, 0)
        return 0

    lax.fori_loop(0, bpw // _D_B, batch, 0)
    pltpu.sync_copy(dbuf, d_h.at[pl.ds(base, bpw)])


def _sc_edge_d(r128, idx_i, idx_j):
    k = pl.kernel(
        _sc_edge_d_body,
        out_type=jax.ShapeDtypeStruct((N_EDGES_PAD,), f32),
        mesh=_sc_mesh,
        compiler_params=pltpu.CompilerParams(needs_layout_passes=False),
        scratch_types=[
            pltpu.VMEM((_D_B,), i32),
            pltpu.VMEM((_D_B,), i32),
            pltpu.VMEM((_D_B, EMB), f32),
            pltpu.VMEM((_D_B, EMB), f32),
            pltpu.VMEM((N_EDGES_PAD // NW,), f32),
            pltpu.SemaphoreType.DMA,
            pltpu.SemaphoreType.DMA,
        ],
    )
    return k(r128, idx_i, idx_j)


_CH = 10000
_NCHUNK = N_EDGES // _CH
_TSL = N_TRI_PAD // NS
_SB = 2048
_DR = 128
_CAP = 2304


def _sc_triplet_body(y_h, ma_h, kj_h, ji_h, lx_h, out_h,
                     ji_s, kj_s, cd_b, ck_b, cp_b, cd2, ck2, cp2,
                     ybuf, mbuf, acc, s0, s1):
    c = lax.axis_index("c")
    s = lax.axis_index("s")
    lanes = lax.broadcasted_iota(i32, (16,), 0)

    def drain_block(off):
        for v in range(_DR // 16):
            ck2[pl.ds(v * 16, 16)] = ck_b[pl.ds(off + v * 16, 16)]
            cp2[pl.ds(v * 16, 16)] = cp_b[pl.ds(off + v * 16, 16)]
            cd2[pl.ds(v * 16, 16)] = cd_b[pl.ds(off + v * 16, 16)]
        g0 = pltpu.async_copy(y_h.at[ck2], ybuf, s0)
        g1 = pltpu.async_copy(ma_h.at[cp2], mbuf, s1)
        g0.wait(); g1.wait()

        def mulrow(r, _):
            for cc in range(EMB // 16):
                ybuf[r, pl.ds(cc * 16, 16)] = (
                    ybuf[r, pl.ds(cc * 16, 16)] * mbuf[r, pl.ds(cc * 16, 16)])
            return 0
        lax.fori_loop(0, _DR, mulrow, 0)
        pltpu.sync_copy(ybuf, acc.at[cd2], add=True)

    for kc in range(_NCHUNK // NC):
        chunk = kc * NC + c
        lo = chunk * _CH

        @pl.when(s < 10)
        def _():
            pltpu.sync_copy(lx_h.at[pl.ds(lo + s * 1000, 1000)],
                            acc.at[pl.ds(s * 1000, 1000)])
        plsc.subcore_barrier()

        def scan_batch(b, cnt):
            tbase = s * _TSL + b * _SB
            pltpu.sync_copy(ji_h.at[pl.ds(tbase, _SB)], ji_s)
            pltpu.sync_copy(kj_h.at[pl.ds(tbase, _SB)], kj_s)

            def scan_vec(v, cnt2):
                jiv = ji_s[pl.ds(v * 16, 16)]
                kjv = kj_s[pl.ds(v * 16, 16)]
                lv = jiv - _bc16(lo)
                m = (lv >= jnp.zeros((16,), i32)) & (lv < jnp.full((16,), _CH, i32))
                mi = m.astype(i32)
                csum = plsc.cumsum(mi)
                tgt = _bc16(cnt2) + csum - jnp.ones((16,), i32)
                plsc.store_scatter(cd_b, [tgt], lv, mask=m)
                plsc.store_scatter(ck_b, [tgt], kjv, mask=m)
                posv = _bc16(tbase + v * 16) + lanes
                plsc.store_scatter(cp_b, [tgt], posv, mask=m)
                return cnt2 + jnp.sum(mi)
            cnt = lax.fori_loop(0, _SB // 16, scan_vec, cnt)

            nfull = cnt // _DR

            def dr(f, _):
                drain_block(f * _DR)
                return 0
            lax.fori_loop(0, nfull, dr, 0)

            rem = cnt - nfull * _DR
            off0 = nfull * _DR
            vals = []
            for v in range(_DR // 16):
                vals.append((cd_b[pl.ds(off0 + v * 16, 16)],
                             ck_b[pl.ds(off0 + v * 16, 16)],
                             cp_b[pl.ds(off0 + v * 16, 16)]))
            for v, (a, bb2, cc2) in enumerate(vals):
                cd_b[pl.ds(v * 16, 16)] = a
                ck_b[pl.ds(v * 16, 16)] = bb2
                cp_b[pl.ds(v * 16, 16)] = cc2
            return rem

        cnt = lax.fori_loop(0, _TSL // _SB, scan_batch, jnp.int32(0))

        for v in range(_DR // 16):
            l = _bc16(jnp.int32(v * 16)) + lanes
            good = l < _bc16(cnt)
            cd_b[pl.ds(v * 16, 16)] = jnp.where(
                good, cd_b[pl.ds(v * 16, 16)], jnp.full((16,), _CH, i32))
            ck_b[pl.ds(v * 16, 16)] = jnp.where(
                good, ck_b[pl.ds(v * 16, 16)], jnp.zeros((16,), i32))
            cp_b[pl.ds(v * 16, 16)] = jnp.where(
                good, cp_b[pl.ds(v * 16, 16)], jnp.zeros((16,), i32))
        drain_block(0)

        plsc.subcore_barrier()

        @pl.when(s < 10)
        def _():
            pltpu.sync_copy(acc.at[pl.ds(s * 1000, 1000)],
                            out_h.at[pl.ds(lo + s * 1000, 1000)])
        plsc.subcore_barrier()


def _sc_triplet(y, ma, kj, ji, lx):
    k = pl.kernel(
        _sc_triplet_body,
        out_type=jax.ShapeDtypeStruct((N_EDGES_PAD, EMB), f32),
        mesh=_sc_mesh,
        compiler_params=pltpu.CompilerParams(needs_layout_passes=False),
        scratch_types=[
            pltpu.VMEM((_SB,), i32),
            pltpu.VMEM((_SB,), i32),
            pltpu.VMEM((_CAP,), i32),
            pltpu.VMEM((_CAP,), i32),
            pltpu.VMEM((_CAP,), i32),
            pltpu.VMEM((_DR,), i32),
            pltpu.VMEM((_DR,), i32),
            pltpu.VMEM((_DR,), i32),
            pltpu.VMEM((_DR, EMB), f32),
            pltpu.VMEM((_DR, EMB), f32),
            pltpu.VMEM_SHARED((_CH + 8, EMB), f32),
            pltpu.SemaphoreType.DMA,
            pltpu.SemaphoreType.DMA,
        ],
    )
    return k(y, ma, kj, ji, lx)


_E2A_B = 128
_E2A_ACC = 10240


def _sc_e2a_body(t_h, ii_h, out_h, idx_v, tbuf, acc, sem):
    c = lax.axis_index("c")
    s = lax.axis_index("s")
    span = _E2A_ACC // NS

    def zrow(r, _):
        for cc in range(EMB // 16):
            tbuf[r, pl.ds(cc * 16, 16)] = jnp.zeros((16,), f32)
        return 0
    lax.fori_loop(0, _E2A_B, zrow, 0)
    for z in range(span // _E2A_B):
        pltpu.sync_copy(tbuf, acc.at[pl.ds(s * span + z * _E2A_B, _E2A_B)])
    plsc.subcore_barrier()

    bpw = N_EDGES_PAD // NW
    base = (c * NS + s) * bpw

    def body(k, _):
        off = base + k * _E2A_B
        pltpu.sync_copy(ii_h.at[pl.ds(off, _E2A_B)], idx_v)
        pltpu.sync_copy(t_h.at[pl.ds(off, _E2A_B)], tbuf)
        pltpu.sync_copy(tbuf, acc.at[idx_v], add=True)
        return 0
    lax.fori_loop(0, bpw // _E2A_B, body, 0)

    plsc.subcore_barrier()

    @pl.when(s < 10)
    def _():
        pltpu.sync_copy(acc.at[pl.ds(s * 1000, 1000)],
                        out_h.at[c, pl.ds(s * 1000, 1000)])


def _sc_e2a(t, idx_i):
    k = pl.kernel(
        _sc_e2a_body,
        out_type=jax.ShapeDtypeStruct((NC, N_ATOMS, EMB), f32),
        mesh=_sc_mesh,
        compiler_params=pltpu.CompilerParams(needs_layout_passes=False),
        scratch_types=[
            pltpu.VMEM((_E2A_B,), i32),
            pltpu.VMEM((_E2A_B, EMB), f32),
            pltpu.VMEM_SHARED((_E2A_ACC, EMB), f32),
            pltpu.SemaphoreType.DMA,
        ],
    )
    return k(t, idx_i)


_A2M_ROWS = 1280


def _sc_a2m_body(ro_h, rs_h, seg_h, out_h, idx_v, idx2_v, buf, zbuf, acc, sem):
    c = lax.axis_index("c")
    s = lax.axis_index("s")
    wid = s * NC + c
    span = _A2M_ROWS // NS

    def zrow(r, _):
        for cc in range(EMB // 16):
            zbuf[r, pl.ds(cc * 16, 16)] = jnp.zeros((16,), f32)
        return 0
    lax.fori_loop(0, span, zrow, 0)
    pltpu.sync_copy(zbuf, acc.at[pl.ds(s * span, span)])
    plsc.subcore_barrier()

    bpw = 512

    @pl.when(wid < 20)
    def _():
        base = wid * bpw
        pltpu.sync_copy(seg_h.at[pl.ds(base, bpw)], idx_v)
        for v in range(bpw // 16):
            idx2_v[pl.ds(v * 16, 16)] = (idx_v[pl.ds(v * 16, 16)]
                                         + jnp.full((16,), 640, i32))
        pltpu.sync_copy(ro_h.at[pl.ds(base, bpw)], buf)
        pltpu.sync_copy(buf, acc.at[idx_v], add=True)
        pltpu.sync_copy(rs_h.at[pl.ds(base, bpw)], buf)
        pltpu.sync_copy(buf, acc.at[idx2_v], add=True)

    plsc.subcore_barrier()
    pltpu.sync_copy(acc.at[pl.ds(s * span, span)],
                    out_h.at[c, pl.ds(s * span, span)])


def _sc_a2m(ro, rs, seg):
    k = pl.kernel(
        _sc_a2m_body,
        out_type=jax.ShapeDtypeStruct((NC, _A2M_ROWS, EMB), f32),
        mesh=_sc_mesh,
        compiler_params=pltpu.CompilerParams(needs_layout_passes=False),
        scratch_types=[
            pltpu.VMEM((512,), i32),
            pltpu.VMEM((512,), i32),
            pltpu.VMEM((512, EMB), f32),
            pltpu.VMEM((_A2M_ROWS // NS, EMB), f32),
            pltpu.VMEM_SHARED((_A2M_ROWS, EMB), f32),
            pltpu.SemaphoreType.DMA,
        ],
    )
    return k(ro, rs, seg)


_EB = 2048
_EGRID = N_EDGES_PAD // _EB
_DB = _EB // 128


def _rbf_tiles(d2):
    d = jnp.sqrt(d2)
    scale = np.sqrt(2.0 / CUTOFF).astype(np.float32)
    inv = 1.0 / (d + 1e-6)
    return [scale * jnp.sin((k + 1) * np.pi * d / CUTOFF) * inv
            for k in range(NUM_RADIAL)]


def _rbf_mix_rows(tiles, zero_row, w6, a):
    del zero_row
    stack = jnp.concatenate([t[a:a + 1] for t in tiles], axis=0)
    return jax.lax.dot_general(stack, w6, (((0,), (0,)), ((), ())),
                               preferred_element_type=f32)


def _tc_atom_pre_body(h_ref, wa_ref, wb_ref, o1_ref, o2_ref):
    h = h_ref[...]
    o1_ref[...] = jax.lax.dot_general(h, wa_ref[...], (((1,), (0,)), ((), ())),
                                      preferred_element_type=f32)
    o2_ref[...] = jax.lax.dot_general(h, wb_ref[...], (((1,), (0,)), ((), ())),
                                      preferred_element_type=f32)


def _tc_atom_pre(h, wa, wb):
    grid = 10
    rb = N_ATOMS // grid
    return pl.pallas_call(
        _tc_atom_pre_body,
        grid=(grid,),
        in_specs=[
            pl.BlockSpec((rb, EMB), lambda s: (s, 0)),
            pl.BlockSpec((EMB, EMB), lambda s: (0, 0)),
            pl.BlockSpec((EMB, EMB), lambda s: (0, 0)),
        ],
        out_specs=[
            pl.BlockSpec((rb, EMB), lambda s: (s, 0)),
            pl.BlockSpec((rb, EMB), lambda s: (s, 0)),
        ],
        out_shape=[
            jax.ShapeDtypeStruct((N_ATOMS, EMB), f32),
            jax.ShapeDtypeStruct((N_ATOMS, EMB), f32),
        ],
    )(h, wa, wb)


def _tc_edge_init_body(gi_ref, gj_ref, d_ref, wr_ref, b_ref,
                       ow_ref, iw_ref, x_ref, t_ref, y_ref):
    tiles = _rbf_tiles(d_ref[...])
    zero_row = jnp.zeros((1, 128), f32)
    for a in range(_DB):
        rows = pl.ds(a * 128, 128)
        rbfe = _rbf_mix_rows(tiles, zero_row, wr_ref[...], a)
        xv = _swish(gi_ref[rows, :] + gj_ref[rows, :] + rbfe + b_ref[...])
        x_ref[rows, :] = xv
        t_ref[rows, :] = xv * _rbf_mix_rows(tiles, zero_row, ow_ref[...], a)
        y_ref[rows, :] = xv * _rbf_mix_rows(tiles, zero_row, iw_ref[...], a)


def _tc_edge_init(gi, gj, d2d, wr, b, ow, iw):
    espec = pl.BlockSpec((_EB, EMB), lambda s: (s, 0))
    dspec = pl.BlockSpec((_DB, 128), lambda s: (s, 0))
    wspec = pl.BlockSpec((NUM_RADIAL, EMB), lambda s: (0, 0))
    return pl.pallas_call(
        _tc_edge_init_body,
        grid=(_EGRID,),
        in_specs=[espec, espec, dspec, wspec,
                  pl.BlockSpec((1, EMB), lambda s: (0, 0)), wspec, wspec],
        out_specs=[espec, espec, espec],
        out_shape=[jax.ShapeDtypeStruct((N_EDGES_PAD, EMB), f32)] * 3,
    )(gi, gj, d2d, wr, b, ow, iw)


_MA_R = 16


def _tc_mabf_body(c_ref, w_ref, o0_ref, o1_ref, o2_ref):
    c = c_ref[...]
    polys = [jnp.ones_like(c), c]
    for _ in range(NUM_ABF - 2):
        polys.append(2.0 * c * polys[-1] - polys[-2])
    zero = jnp.zeros((1, 128), f32)
    outs = (o0_ref, o1_ref, o2_ref)
    for a in range(_MA_R):
        stack = jnp.concatenate(
            [polys[k][a:a + 1] for k in range(NUM_ABF)] + [zero],
            axis=0)
        for i in range(NUM_BLOCKS):
            outs[i][pl.ds(a * 128, 128), :] = jax.lax.dot_general(
                stack, w_ref[i], (((0,), (0,)), ((), ())),
                preferred_element_type=f32)


def _tc_mabf(cos2d, wabf8):
    grid = (N_TRI_PAD // 128) // _MA_R
    ospec = pl.BlockSpec((_MA_R * 128, EMB), lambda s: (s, 0))
    return pl.pallas_call(
        _tc_mabf_body,
        grid=(grid,),
        in_specs=[
            pl.BlockSpec((_MA_R, 128), lambda s: (s, 0)),
            pl.BlockSpec((NUM_BLOCKS, 8, EMB), lambda s: (0, 0, 0)),
        ],
        out_specs=[ospec, ospec, ospec],
        out_shape=[jax.ShapeDtypeStruct((N_TRI_PAD, EMB), f32)] * 3,
    )(cos2d, wabf8)


def _tc_mlp_body(ax_ref, lx_ref, d_ref, w1_ref, b1_ref, w2_ref,
                 b2_ref, ow_ref, iw_ref, xn_ref, t_ref, y_ref):
    u = _swish(jax.lax.dot_general(ax_ref[...], w1_ref[...],
                                   (((1,), (0,)), ((), ())),
                                   preferred_element_type=f32) + b1_ref[...])
    xn = lx_ref[...] + _swish(
        jax.lax.dot_general(u, w2_ref[...], (((1,), (0,)), ((), ())),
                            preferred_element_type=f32) + b2_ref[...])
    xn_ref[...] = xn
    tiles = _rbf_tiles(d_ref[...])
    zero_row = jnp.zeros((1, 128), f32)
    for a in range(_DB):
        rows = pl.ds(a * 128, 128)
        t_ref[rows, :] = xn[a * 128:(a + 1) * 128, :] * _rbf_mix_rows(
            tiles, zero_row, ow_ref[...], a)
        if y_ref is not None:
            y_ref[rows, :] = xn[a * 128:(a + 1) * 128, :] * _rbf_mix_rows(
                tiles, zero_row, iw_ref[...], a)


def _tc_mlp(aggx, lastx, d2d, w1, b1, w2, b2, ow, iw, want_y):
    espec = pl.BlockSpec((_EB, EMB), lambda s: (s, 0))
    dspec = pl.BlockSpec((_DB, 128), lambda s: (s, 0))
    mspec = pl.BlockSpec((EMB, EMB), lambda s: (0, 0))
    bspec = pl.BlockSpec((1, EMB), lambda s: (0, 0))
    wspec = pl.BlockSpec((NUM_RADIAL, EMB), lambda s: (0, 0))
    if want_y:
        body = _tc_mlp_body
        out_specs = [espec, espec, espec]
        out_shape = [jax.ShapeDtypeStruct((N_EDGES_PAD, EMB), f32)] * 3
    else:
        def body(ax, lx, d_, w1_, b1_, w2_, b2_, ow_, iw_, xn_, t_):
            _tc_mlp_body(ax, lx, d_, w1_, b1_, w2_, b2_, ow_, iw_,
                         xn_, t_, None)
        out_specs = [espec, espec]
        out_shape = [jax.ShapeDtypeStruct((N_EDGES_PAD, EMB), f32)] * 2
    return pl.pallas_call(
        body,
        grid=(_EGRID,),
        in_specs=[espec, espec, dspec, mspec, bspec, mspec, bspec,
                  wspec, wspec],
        out_specs=out_specs,
        out_shape=out_shape,
    )(aggx, lastx, d2d, w1, b1, w2, b2, ow, iw)


def _tc_atom_final_body(a0_ref, a1_ref, a2_ref, a3_ref, h_ref,
                        ow1_ref, ow2_ref, sbw_ref, sbb_ref,
                        ro_ref, rs_ref):
    arefs = (a0_ref, a1_ref, a2_ref, a3_ref)
    ro = None
    for i in range(NUM_BLOCKS + 1):
        a = arefs[i][0] + arefs[i][1]
        u = _swish(jax.lax.dot_general(a, ow1_ref[i], (((1,), (0,)), ((), ())),
                                       preferred_element_type=f32))
        v = jax.lax.dot_general(u, ow2_ref[i], (((1,), (0,)), ((), ())),
                                preferred_element_type=f32)
        ro = v if ro is None else ro + v
    ro_ref[...] = ro
    rs = _swish(jax.lax.dot_general(h_ref[...], sbw_ref[0],
                                    (((1,), (0,)), ((), ())),
                                    preferred_element_type=f32) + sbb_ref[0])
    for i in range(NUM_BLOCKS):
        rs = rs + _swish(
            jax.lax.dot_general(rs, sbw_ref[i + 1], (((1,), (0,)), ((), ())),
                                preferred_element_type=f32) + sbb_ref[i + 1])
    rs_ref[...] = rs


def _tc_atom_final(a_list, h, ow1, ow2, sbw, sbb):
    grid = 10
    rb = N_ATOMS // grid
    aspec = pl.BlockSpec((NC, rb, EMB), lambda s: (0, s, 0))
    nb1 = NUM_BLOCKS + 1
    return pl.pallas_call(
        _tc_atom_final_body,
        grid=(grid,),
        in_specs=[aspec, aspec, aspec, aspec,
                  pl.BlockSpec((rb, EMB), lambda s: (s, 0)),
                  pl.BlockSpec((nb1, EMB, EMB), lambda s: (0, 0, 0)),
                  pl.BlockSpec((nb1, EMB, EMB), lambda s: (0, 0, 0)),
                  pl.BlockSpec((nb1, EMB, EMB), lambda s: (0, 0, 0)),
                  pl.BlockSpec((nb1, 1, EMB), lambda s: (0, 0, 0))],
        out_specs=[pl.BlockSpec((rb, EMB), lambda s: (s, 0)),
                   pl.BlockSpec((rb, EMB), lambda s: (s, 0))],
        out_shape=[jax.ShapeDtypeStruct((N_ATOMS, EMB), f32)] * 2,
    )(*a_list, h, ow1, ow2, sbw, sbb)


def _tc_mol_body(m_ref, cm_ref, cs_ref, o_ref):
    res = m_ref[0, 0:N_MOL, :] + m_ref[1, 0:N_MOL, :]
    sing = m_ref[0, 640:640 + N_MOL, :] + m_ref[1, 640:640 + N_MOL, :]
    o_ref[...] = cm_ref[0, 0] * res + cs_ref[0, 0] * sing


def _tc_mol(mo, cm, cs):
    return pl.pallas_call(
        _tc_mol_body,
        in_specs=[pl.BlockSpec((NC, _A2M_ROWS, EMB), lambda: (0, 0, 0)),
                  pl.BlockSpec((1, 1), lambda: (0, 0)),
                  pl.BlockSpec((1, 1), lambda: (0, 0))],
        out_specs=pl.BlockSpec((N_MOL, EMB), lambda: (0, 0)),
        out_shape=jax.ShapeDtypeStruct((N_MOL, EMB), f32),
        grid=(),
    )(mo, cm, cs)


def kernel(Z, R, batch_seg, idx_i, idx_j, idx_kj, idx_ji, cosine_ijk, params):
    p = params
    ep = N_EDGES_PAD - N_EDGES
    Zp = jnp.pad(Z.astype(i32), (0, N_ATOMS_PAD - N_ATOMS))
    r128 = jnp.pad(R.astype(f32), ((0, 0), (0, EMB - 3)))
    idx_i_g = jnp.pad(idx_i.astype(i32), (0, ep))
    idx_j_g = jnp.pad(idx_j.astype(i32), (0, ep))
    idx_i_s = jnp.pad(idx_i.astype(i32), (0, ep),
                      constant_values=N_ATOMS)
    tp = N_TRI_PAD - N_TRIPLETS
    idx_kj = jnp.pad(idx_kj.astype(i32), (0, tp))
    idx_ji = jnp.pad(idx_ji.astype(i32), (0, tp),
                     constant_values=1 << 29)
    seg_p = jnp.pad(batch_seg.astype(i32), (0, N_ATOMS_PAD - N_ATOMS),
                    constant_values=N_MOL)

    emb_b = p["emb_b"].reshape(1, EMB)
    wabf8 = jnp.pad(p["int_W_abf"], ((0, 0), (0, 8 - NUM_ABF), (0, 0)))
    cos2d = jnp.pad(cosine_ijk.astype(f32),
                    (0, N_TRI_PAD - N_TRIPLETS)).reshape(N_TRI_PAD // 128, 128)

    hp = _sc_h_gather(p["emb_table"], Zp)
    h = hp[:N_ATOMS]
    hw1, hw2 = _tc_atom_pre(h, p["emb_W"][:EMB], p["emb_W"][EMB:2 * EMB])
    gi, gj = _sc_edge_gather(hw1, hw2, idx_i_g, idx_j_g)
    d2d = _sc_edge_d(r128, idx_i_g, idx_j_g).reshape(N_EDGES_PAD // 128, 128)
    wr_x = p["emb_W_rbf"] @ p["emb_W"][2 * EMB:]
    x, t0, y = _tc_edge_init(gi, gj, d2d, wr_x, emb_b,
                             p["out_W_rbf"][0], p["int_W_rbf"][0])
    ma = _tc_mabf(cos2d, wabf8)

    a_list = [_sc_e2a(t0, idx_i_s)]
    last = x
    for i in range(NUM_BLOCKS):
        aggx = _sc_triplet(y, ma[i], idx_kj, idx_ji, last)
        want_y = i < NUM_BLOCKS - 1
        ow = p["out_W_rbf"][i + 1]
        iw = p["int_W_rbf"][i + 1] if want_y else p["int_W_rbf"][i]
        outs = _tc_mlp(aggx, last, d2d, p["int_W1"][i],
                       p["int_b1"][i].reshape(1, EMB), p["int_W2"][i],
                       p["int_b2"][i].reshape(1, EMB), ow, iw, want_y)
        if want_y:
            xn, t_next, y = outs
        else:
            xn, t_next = outs
        a_list.append(_sc_e2a(t_next, idx_i_s))
        last = xn

    ro, rs = _tc_atom_final(a_list, h, p["out_W1"], p["out_W2"], p["sb_W"],
                            p["sb_b"].reshape(NUM_BLOCKS + 1, 1, EMB))
    ro_p = jnp.pad(ro, ((0, N_ATOMS_PAD - N_ATOMS), (0, 0)))
    rs_p = jnp.pad(rs, ((0, N_ATOMS_PAD - N_ATOMS), (0, 0)))
    mo = _sc_a2m(ro_p, rs_p, seg_p)
    return _tc_mol(mo, p["coef_mp"].reshape(1, 1), p["coef_sg"].reshape(1, 1))


# speedup vs baseline: 1.5887x; 1.1949x over previous
import functools

import jax
import jax.numpy as jnp
import numpy as np
from jax import lax
from jax.experimental import pallas as pl
from jax.experimental.pallas import tpu as pltpu
from jax.experimental.pallas import tpu_sc as plsc

EMB = 128
NUM_BLOCKS = 3
NUM_RADIAL = 6
CUTOFF = 5.0
NUM_ABF = 7
N_ATOMS = 10000
N_EDGES = 160000
N_TRIPLETS = 320000
N_MOL = 512

N_ATOMS_PAD = 10240
N_EDGES_PAD = 163840
N_TRI_PAD = 327680

NC = 2
NS = 16
NW = NC * NS

f32 = jnp.float32
i32 = jnp.int32

_sc_mesh = plsc.VectorSubcoreMesh(core_axis_name="c", subcore_axis_name="s")


def _bc16(x):
    return jax.lax.broadcast_in_dim(x, (16,), ())


def _swish(x):
    return x * jax.nn.sigmoid(x)


def _sc_h_gather_body(tab_h, z_h, out_h, idx_v, rows_v, sem):
    wid = lax.axis_index("s") * NC + lax.axis_index("c")
    bpw = 512

    @pl.when(wid < 20)
    def _():
        base = wid * bpw
        pltpu.sync_copy(z_h.at[pl.ds(base, bpw)], idx_v)
        pltpu.async_copy(tab_h.at[idx_v], rows_v, sem).wait()
        pltpu.sync_copy(rows_v, out_h.at[pl.ds(base, bpw)])


def _sc_h_gather(tab, zp):
    k = pl.kernel(
        _sc_h_gather_body,
        out_type=jax.ShapeDtypeStruct((N_ATOMS_PAD, EMB), f32),
        mesh=_sc_mesh,
        compiler_params=pltpu.CompilerParams(needs_layout_passes=False),
        scratch_types=[
            pltpu.VMEM((512,), i32),
            pltpu.VMEM((512, EMB), f32),
            pltpu.SemaphoreType.DMA,
        ],
    )
    return k(tab, zp)


_EG_B = 256


def _sc_edge_gather_body(hw1_h, hw2_h, ii_h, jj_h, gi_h, gj_h,
                         iv, jv, ba, bb, s0, s1):
    wid = lax.axis_index("s") * NC + lax.axis_index("c")
    bpw = N_EDGES_PAD // NW
    base = wid * bpw

    def body(k, _):
        off = k * _EG_B
        pltpu.sync_copy(ii_h.at[pl.ds(base + off, _EG_B)], iv)
        pltpu.sync_copy(jj_h.at[pl.ds(base + off, _EG_B)], jv)
        c0 = pltpu.async_copy(hw1_h.at[iv], ba, s0)
        c1 = pltpu.async_copy(hw2_h.at[jv], bb, s1)
        c0.wait(); c1.wait()
        pltpu.sync_copy(ba, gi_h.at[pl.ds(base + off, _EG_B)])
        pltpu.sync_copy(bb, gj_h.at[pl.ds(base + off, _EG_B)])
        return 0

    lax.fori_loop(0, bpw // _EG_B, body, 0)


def _sc_edge_gather(hw1, hw2, idx_i, idx_j):
    k = pl.kernel(
        _sc_edge_gather_body,
        out_type=(
            jax.ShapeDtypeStruct((N_EDGES_PAD, EMB), f32),
            jax.ShapeDtypeStruct((N_EDGES_PAD, EMB), f32),
        ),
        mesh=_sc_mesh,
        compiler_params=pltpu.CompilerParams(needs_layout_passes=False),
        scratch_types=[
            pltpu.VMEM((_EG_B,), i32),
            pltpu.VMEM((_EG_B,), i32),
            pltpu.VMEM((_EG_B, EMB), f32),
            pltpu.VMEM((_EG_B, EMB), f32),
            pltpu.SemaphoreType.DMA,
            pltpu.SemaphoreType.DMA,
        ],
    )
    return k(hw1, hw2, idx_i, idx_j)


_D_B = 256


def _sc_edge_d_body(r128_h, ii_h, jj_h, d_h, iv, jv, rib, rjb, dbuf, s0, s1):
    wid = lax.axis_index("s") * NC + lax.axis_index("c")
    bpw = N_EDGES_PAD // NW
    base = wid * bpw

    def batch(k, _):
        off = k * _D_B
        pltpu.sync_copy(ii_h.at[pl.ds(base + off, _D_B)], iv)
        pltpu.sync_copy(jj_h.at[pl.ds(base + off, _D_B)], jv)
        c0 = pltpu.async_copy(r128_h.at[iv], rib, s0)
        c1 = pltpu.async_copy(r128_h.at[jv], rjb, s1)
        c0.wait(); c1.wait()

        def row(r, _2):
            lns = lax.broadcasted_iota(i32, (16,), 0)
            cm = jnp.where(lns < jnp.full((16,), 3, i32),
                           jnp.ones((16,), f32), jnp.zeros((16,), f32))
            l0 = lns == jnp.zeros((16,), i32)
            eps = jnp.where(l0, jnp.full((16,), 1e-12, f32),
                            jnp.zeros((16,), f32))
            va = rib[r, pl.ds(0, 16)] - rjb[r, pl.ds(0, 16)]
            sq = va * va * cm + eps
            s = jnp.sum(sq)
            plsc.store_scatter(dbuf, [_bc16(off + r)], _bc16(s), mask=l0)
            return 0
        lax.fori_loop(0, _D_B, row, 0)
        return 0

    lax.fori_loop(0, bpw // _D_B, batch, 0)
    pltpu.sync_copy(dbuf, d_h.at[pl.ds(base, bpw)])


def _sc_edge_d(r128, idx_i, idx_j):
    k = pl.kernel(
        _sc_edge_d_body,
        out_type=jax.ShapeDtypeStruct((N_EDGES_PAD,), f32),
        mesh=_sc_mesh,
        compiler_params=pltpu.CompilerParams(needs_layout_passes=False),
        scratch_types=[
            pltpu.VMEM((_D_B,), i32),
            pltpu.VMEM((_D_B,), i32),
            pltpu.VMEM((_D_B, EMB), f32),
            pltpu.VMEM((_D_B, EMB), f32),
            pltpu.VMEM((N_EDGES_PAD // NW,), f32),
            pltpu.SemaphoreType.DMA,
            pltpu.SemaphoreType.DMA,
        ],
    )
    return k(r128, idx_i, idx_j)


_CH = 10000
_NCHUNK = N_EDGES // _CH
_TSL = N_TRI_PAD // NS
_SB = 2048
_DR = 64
_CAP = 2304
_PKM = (1 << 14) - 1


def _sc_triplet_body(y_h, ma_h, kj_h, ji_h, lx_h, out_h,
                     ji_s, kj_s, pk_b, cp_b, cnb,
                     cdA, ckA, cpA, cdB, ckB, cpB,
                     ybA, mbA, ybB, mbB, acc, syA, smA, syB, smB):
    c = lax.axis_index("c")
    s = lax.axis_index("s")

    def fire(off, cdx, ckx, cpx, ybx, mbx, sy, sm):
        for v in range(_DR // 16):
            pk = pk_b[pl.ds(off + v * 16, 16)]
            cdx[pl.ds(v * 16, 16)] = pk & jnp.full((16,), _PKM, i32)
            ckx[pl.ds(v * 16, 16)] = lax.shift_right_logical(
                pk, jnp.full((16,), 14, i32))
            cpx[pl.ds(v * 16, 16)] = cp_b[pl.ds(off + v * 16, 16)]
        pltpu.async_copy(y_h.at[ckx], ybx, sy)
        pltpu.async_copy(ma_h.at[cpx], mbx, sm)

    def finish(cdx, ybx, mbx, sy, sm):
        pltpu.make_async_copy(y_h.at[pl.ds(0, _DR)], ybx, sy).wait()
        pltpu.make_async_copy(ma_h.at[pl.ds(0, _DR)], mbx, sm).wait()

        def mulrow(r, _):
            for cc in range(EMB // 16):
                ybx[r, pl.ds(cc * 16, 16)] = (
                    ybx[r, pl.ds(cc * 16, 16)] * mbx[r, pl.ds(cc * 16, 16)])
            return 0
        lax.fori_loop(0, _DR, mulrow, 0)
        pltpu.sync_copy(ybx, acc.at[cdx], add=True)

    for kc in range(_NCHUNK // NC):
        chunk = kc * NC + c
        lo = chunk * _CH

        @pl.when(s < 10)
        def _():
            pltpu.sync_copy(lx_h.at[pl.ds(lo + s * 1000, 1000)],
                            acc.at[pl.ds(s * 1000, 1000)])
        plsc.subcore_barrier()

        def scan_batch(b, cnt):
            tbase = s * _TSL + b * _SB
            pltpu.sync_copy(ji_h.at[pl.ds(tbase, _SB)], ji_s)
            pltpu.sync_copy(kj_h.at[pl.ds(tbase, _SB)], kj_s)

            def scan_vec(v, cnt2):
                lanes = lax.broadcasted_iota(i32, (16,), 0)
                jiv = ji_s[pl.ds(v * 16, 16)]
                kjv = kj_s[pl.ds(v * 16, 16)]
                lv = jiv - _bc16(lo)
                m = ((lv >= jnp.zeros((16,), i32))
                     & (lv < jnp.full((16,), _CH, i32)))
                mi = m.astype(i32)
                csum = plsc.cumsum(mi)
                tgt = _bc16(cnt2) + csum - jnp.ones((16,), i32)
                pk = lv | lax.shift_left(kjv, jnp.full((16,), 14, i32))
                plsc.store_scatter(pk_b, [tgt], pk, mask=m)
                posv = _bc16(tbase + v * 16) + lanes
                plsc.store_scatter(cp_b, [tgt], posv, mask=m)
                return cnt2 + csum[15]
            cnt = lax.fori_loop(0, _SB // 16, scan_vec, cnt)

            nfull = cnt // _DR

            @pl.when(nfull > 0)
            def _():
                fire(0, cdA, ckA, cpA, ybA, mbA, syA, smA)

            def duo(g, _):
                f1 = g * 2 + 1

                @pl.when(f1 < nfull)
                def _():
                    fire(f1 * _DR, cdB, ckB, cpB, ybB, mbB, syB, smB)
                finish(cdA, ybA, mbA, syA, smA)

                @pl.when(f1 < nfull)
                def _():
                    @pl.when(f1 + 1 < nfull)
                    def _():
                        fire((f1 + 1) * _DR, cdA, ckA, cpA, ybA, mbA,
                             syA, smA)
                    finish(cdB, ybB, mbB, syB, smB)
                return 0
            lax.fori_loop(0, (nfull + 1) // 2, duo, 0)

            rem = cnt - nfull * _DR
            off0 = nfull * _DR
            vals = []
            for v in range(_DR // 16):
                vals.append((pk_b[pl.ds(off0 + v * 16, 16)],
                             cp_b[pl.ds(off0 + v * 16, 16)]))
            for v, (a, bb2) in enumerate(vals):
                pk_b[pl.ds(v * 16, 16)] = a
                cp_b[pl.ds(v * 16, 16)] = bb2
            return rem

        cnt = lax.fori_loop(0, _TSL // _SB, scan_batch, jnp.int32(0))

        for v in range(_DR // 16):
            lanes = lax.broadcasted_iota(i32, (16,), 0)
            l = _bc16(jnp.int32(v * 16)) + lanes
            good = l < _bc16(cnt)
            pk_b[pl.ds(v * 16, 16)] = jnp.where(
                good, pk_b[pl.ds(v * 16, 16)], jnp.full((16,), _CH, i32))
            cp_b[pl.ds(v * 16, 16)] = jnp.where(
                good, cp_b[pl.ds(v * 16, 16)], jnp.zeros((16,), i32))
        fire(0, cdA, ckA, cpA, ybA, mbA, syA, smA)
        finish(cdA, ybA, mbA, syA, smA)

        plsc.subcore_barrier()

        @pl.when(s < 10)
        def _():
            pltpu.sync_copy(acc.at[pl.ds(s * 1000, 1000)],
                            out_h.at[pl.ds(lo + s * 1000, 1000)])
        plsc.subcore_barrier()


def _sc_triplet(y, ma, kj, ji, lx):
    k = pl.kernel(
        _sc_triplet_body,
        out_type=jax.ShapeDtypeStruct((N_EDGES_PAD, EMB), f32),
        mesh=_sc_mesh,
        compiler_params=pltpu.CompilerParams(needs_layout_passes=False),
        scratch_types=[
            pltpu.VMEM((_SB,), i32),
            pltpu.VMEM((_SB,), i32),
            pltpu.VMEM((_CAP,), i32),
            pltpu.VMEM((_CAP,), i32),
            pltpu.VMEM((16,), i32),
            pltpu.VMEM((_DR,), i32),
            pltpu.VMEM((_DR,), i32),
            pltpu.VMEM((_DR,), i32),
            pltpu.VMEM((_DR,), i32),
            pltpu.VMEM((_DR,), i32),
            pltpu.VMEM((_DR,), i32),
            pltpu.VMEM((_DR, EMB), f32),
            pltpu.VMEM((_DR, EMB), f32),
            pltpu.VMEM((_DR, EMB), f32),
            pltpu.VMEM((_DR, EMB), f32),
            pltpu.VMEM_SHARED((_CH + 8, EMB), f32),
            pltpu.SemaphoreType.DMA,
            pltpu.SemaphoreType.DMA,
            pltpu.SemaphoreType.DMA,
            pltpu.SemaphoreType.DMA,
        ],
    )
    return k(y, ma, kj, ji, lx)


_E2A_B = 128
_E2A_ACC = 10240


def _sc_e2a_body(t_h, ii_h, out_h, idx_v, tbuf, acc, sem):
    c = lax.axis_index("c")
    s = lax.axis_index("s")
    span = _E2A_ACC // NS

    def zrow(r, _):
        for cc in range(EMB // 16):
            tbuf[r, pl.ds(cc * 16, 16)] = jnp.zeros((16,), f32)
        return 0
    lax.fori_loop(0, _E2A_B, zrow, 0)
    for z in range(span // _E2A_B):
        pltpu.sync_copy(tbuf, acc.at[pl.ds(s * span + z * _E2A_B, _E2A_B)])
    plsc.subcore_barrier()

    bpw = N_EDGES_PAD // NW
    base = (c * NS + s) * bpw

    def body(k, _):
        off = base + k * _E2A_B
        pltpu.sync_copy(ii_h.at[pl.ds(off, _E2A_B)], idx_v)
        pltpu.sync_copy(t_h.at[pl.ds(off, _E2A_B)], tbuf)
        pltpu.sync_copy(tbuf, acc.at[idx_v], add=True)
        return 0
    lax.fori_loop(0, bpw // _E2A_B, body, 0)

    plsc.subcore_barrier()

    @pl.when(s < 10)
    def _():
        pltpu.sync_copy(acc.at[pl.ds(s * 1000, 1000)],
                        out_h.at[c, pl.ds(s * 1000, 1000)])


def _sc_e2a(t, idx_i):
    k = pl.kernel(
        _sc_e2a_body,
        out_type=jax.ShapeDtypeStruct((NC, N_ATOMS, EMB), f32),
        mesh=_sc_mesh,
        compiler_params=pltpu.CompilerParams(needs_layout_passes=False),
        scratch_types=[
            pltpu.VMEM((_E2A_B,), i32),
            pltpu.VMEM((_E2A_B, EMB), f32),
            pltpu.VMEM_SHARED((_E2A_ACC, EMB), f32),
            pltpu.SemaphoreType.DMA,
        ],
    )
    return k(t, idx_i)


_A2M_ROWS = 1280


def _sc_a2m_body(ro_h, rs_h, seg_h, out_h, idx_v, idx2_v, buf, zbuf, acc, sem):
    c = lax.axis_index("c")
    s = lax.axis_index("s")
    wid = s * NC + c
    span = _A2M_ROWS // NS

    def zrow(r, _):
        for cc in range(EMB // 16):
            zbuf[r, pl.ds(cc * 16, 16)] = jnp.zeros((16,), f32)
        return 0
    lax.fori_loop(0, span, zrow, 0)
    pltpu.sync_copy(zbuf, acc.at[pl.ds(s * span, span)])
    plsc.subcore_barrier()

    bpw = 512

    @pl.when(wid < 20)
    def _():
        base = wid * bpw
        pltpu.sync_copy(seg_h.at[pl.ds(base, bpw)], idx_v)
        for v in range(bpw // 16):
            idx2_v[pl.ds(v * 16, 16)] = (idx_v[pl.ds(v * 16, 16)]
                                         + jnp.full((16,), 640, i32))
        pltpu.sync_copy(ro_h.at[pl.ds(base, bpw)], buf)
        pltpu.sync_copy(buf, acc.at[idx_v], add=True)
        pltpu.sync_copy(rs_h.at[pl.ds(base, bpw)], buf)
        pltpu.sync_copy(buf, acc.at[idx2_v], add=True)

    plsc.subcore_barrier()
    pltpu.sync_copy(acc.at[pl.ds(s * span, span)],
                    out_h.at[c, pl.ds(s * span, span)])


def _sc_a2m(ro, rs, seg):
    k = pl.kernel(
        _sc_a2m_body,
        out_type=jax.ShapeDtypeStruct((NC, _A2M_ROWS, EMB), f32),
        mesh=_sc_mesh,
        compiler_params=pltpu.CompilerParams(needs_layout_passes=False),
        scratch_types=[
            pltpu.VMEM((512,), i32),
            pltpu.VMEM((512,), i32),
            pltpu.VMEM((512, EMB), f32),
            pltpu.VMEM((_A2M_ROWS // NS, EMB), f32),
            pltpu.VMEM_SHARED((_A2M_ROWS, EMB), f32),
            pltpu.SemaphoreType.DMA,
        ],
    )
    return k(ro, rs, seg)


_EB = 2048
_EGRID = N_EDGES_PAD // _EB
_DB = _EB // 128


def _rbf_tiles(d2):
    d = jnp.sqrt(d2)
    scale = np.sqrt(2.0 / CUTOFF).astype(np.float32)
    inv = 1.0 / (d + 1e-6)
    return [scale * jnp.sin((k + 1) * np.pi * d / CUTOFF) * inv
            for k in range(NUM_RADIAL)]


def _rbf_mix_rows(tiles, zero_row, w6, a):
    del zero_row
    stack = jnp.concatenate([t[a:a + 1] for t in tiles], axis=0)
    return jax.lax.dot_general(stack, w6, (((0,), (0,)), ((), ())),
                               preferred_element_type=f32)


def _tc_atom_pre_body(h_ref, wa_ref, wb_ref, o1_ref, o2_ref):
    h = h_ref[...]
    o1_ref[...] = jax.lax.dot_general(h, wa_ref[...], (((1,), (0,)), ((), ())),
                                      preferred_element_type=f32)
    o2_ref[...] = jax.lax.dot_general(h, wb_ref[...], (((1,), (0,)), ((), ())),
                                      preferred_element_type=f32)


def _tc_atom_pre(h, wa, wb):
    grid = 10
    rb = N_ATOMS // grid
    return pl.pallas_call(
        _tc_atom_pre_body,
        grid=(grid,),
        in_specs=[
            pl.BlockSpec((rb, EMB), lambda s: (s, 0)),
            pl.BlockSpec((EMB, EMB), lambda s: (0, 0)),
            pl.BlockSpec((EMB, EMB), lambda s: (0, 0)),
        ],
        out_specs=[
            pl.BlockSpec((rb, EMB), lambda s: (s, 0)),
            pl.BlockSpec((rb, EMB), lambda s: (s, 0)),
        ],
        out_shape=[
            jax.ShapeDtypeStruct((N_ATOMS, EMB), f32),
            jax.ShapeDtypeStruct((N_ATOMS, EMB), f32),
        ],
    )(h, wa, wb)


def _tc_edge_init_body(gi_ref, gj_ref, d_ref, wr_ref, b_ref,
                       ow_ref, iw_ref, x_ref, t_ref, y_ref):
    tiles = _rbf_tiles(d_ref[...])
    zero_row = jnp.zeros((1, 128), f32)
    for a in range(_DB):
        rows = pl.ds(a * 128, 128)
        rbfe = _rbf_mix_rows(tiles, zero_row, wr_ref[...], a)
        xv = _swish(gi_ref[rows, :] + gj_ref[rows, :] + rbfe + b_ref[...])
        x_ref[rows, :] = xv
        t_ref[rows, :] = xv * _rbf_mix_rows(tiles, zero_row, ow_ref[...], a)
        y_ref[rows, :] = xv * _rbf_mix_rows(tiles, zero_row, iw_ref[...], a)


def _tc_edge_init(gi, gj, d2d, wr, b, ow, iw):
    espec = pl.BlockSpec((_EB, EMB), lambda s: (s, 0))
    dspec = pl.BlockSpec((_DB, 128), lambda s: (s, 0))
    wspec = pl.BlockSpec((NUM_RADIAL, EMB), lambda s: (0, 0))
    return pl.pallas_call(
        _tc_edge_init_body,
        grid=(_EGRID,),
        in_specs=[espec, espec, dspec, wspec,
                  pl.BlockSpec((1, EMB), lambda s: (0, 0)), wspec, wspec],
        out_specs=[espec, espec, espec],
        out_shape=[jax.ShapeDtypeStruct((N_EDGES_PAD, EMB), f32)] * 3,
    )(gi, gj, d2d, wr, b, ow, iw)


_MA_R = 16


def _tc_mabf_body(c_ref, w_ref, o0_ref, o1_ref, o2_ref):
    c = c_ref[...]
    polys = [jnp.ones_like(c), c]
    for _ in range(NUM_ABF - 2):
        polys.append(2.0 * c * polys[-1] - polys[-2])
    zero = jnp.zeros((1, 128), f32)
    outs = (o0_ref, o1_ref, o2_ref)
    for a in range(_MA_R):
        stack = jnp.concatenate(
            [polys[k][a:a + 1] for k in range(NUM_ABF)] + [zero],
            axis=0)
        for i in range(NUM_BLOCKS):
            outs[i][pl.ds(a * 128, 128), :] = jax.lax.dot_general(
                stack, w_ref[i], (((0,), (0,)), ((), ())),
                preferred_element_type=f32)


def _tc_mabf(cos2d, wabf8):
    grid = (N_TRI_PAD // 128) // _MA_R
    ospec = pl.BlockSpec((_MA_R * 128, EMB), lambda s: (s, 0))
    return pl.pallas_call(
        _tc_mabf_body,
        grid=(grid,),
        in_specs=[
            pl.BlockSpec((_MA_R, 128), lambda s: (s, 0)),
            pl.BlockSpec((NUM_BLOCKS, 8, EMB), lambda s: (0, 0, 0)),
        ],
        out_specs=[ospec, ospec, ospec],
        out_shape=[jax.ShapeDtypeStruct((N_TRI_PAD, EMB), f32)] * 3,
    )(cos2d, wabf8)


def _tc_mlp_body(ax_ref, lx_ref, d_ref, w1_ref, b1_ref, w2_ref,
                 b2_ref, ow_ref, iw_ref, xn_ref, t_ref, y_ref):
    u = _swish(jax.lax.dot_general(ax_ref[...], w1_ref[...],
                                   (((1,), (0,)), ((), ())),
                                   preferred_element_type=f32) + b1_ref[...])
    xn = lx_ref[...] + _swish(
        jax.lax.dot_general(u, w2_ref[...], (((1,), (0,)), ((), ())),
                            preferred_element_type=f32) + b2_ref[...])
    xn_ref[...] = xn
    tiles = _rbf_tiles(d_ref[...])
    zero_row = jnp.zeros((1, 128), f32)
    for a in range(_DB):
        rows = pl.ds(a * 128, 128)
        t_ref[rows, :] = xn[a * 128:(a + 1) * 128, :] * _rbf_mix_rows(
            tiles, zero_row, ow_ref[...], a)
        if y_ref is not None:
            y_ref[rows, :] = xn[a * 128:(a + 1) * 128, :] * _rbf_mix_rows(
                tiles, zero_row, iw_ref[...], a)


def _tc_mlp(aggx, lastx, d2d, w1, b1, w2, b2, ow, iw, want_y):
    espec = pl.BlockSpec((_EB, EMB), lambda s: (s, 0))
    dspec = pl.BlockSpec((_DB, 128), lambda s: (s, 0))
    mspec = pl.BlockSpec((EMB, EMB), lambda s: (0, 0))
    bspec = pl.BlockSpec((1, EMB), lambda s: (0, 0))
    wspec = pl.BlockSpec((NUM_RADIAL, EMB), lambda s: (0, 0))
    if want_y:
        body = _tc_mlp_body
        out_specs = [espec, espec, espec]
        out_shape = [jax.ShapeDtypeStruct((N_EDGES_PAD, EMB), f32)] * 3
    else:
        def body(ax, lx, d_, w1_, b1_, w2_, b2_, ow_, iw_, xn_, t_):
            _tc_mlp_body(ax, lx, d_, w1_, b1_, w2_, b2_, ow_, iw_,
                         xn_, t_, None)
        out_specs = [espec, espec]
        out_shape = [jax.ShapeDtypeStruct((N_EDGES_PAD, EMB), f32)] * 2
    return pl.pallas_call(
        body,
        grid=(_EGRID,),
        in_specs=[espec, espec, dspec, mspec, bspec, mspec, bspec,
                  wspec, wspec],
        out_specs=out_specs,
        out_shape=out_shape,
    )(aggx, lastx, d2d, w1, b1, w2, b2, ow, iw)


def _tc_atom_final_body(a0_ref, a1_ref, a2_ref, a3_ref, h_ref,
                        ow1_ref, ow2_ref, sbw_ref, sbb_ref,
                        ro_ref, rs_ref):
    arefs = (a0_ref, a1_ref, a2_ref, a3_ref)
    ro = None
    for i in range(NUM_BLOCKS + 1):
        a = arefs[i][0] + arefs[i][1]
        u = _swish(jax.lax.dot_general(a, ow1_ref[i], (((1,), (0,)), ((), ())),
                                       preferred_element_type=f32))
        v = jax.lax.dot_general(u, ow2_ref[i], (((1,), (0,)), ((), ())),
                                preferred_element_type=f32)
        ro = v if ro is None else ro + v
    ro_ref[...] = ro
    rs = _swish(jax.lax.dot_general(h_ref[...], sbw_ref[0],
                                    (((1,), (0,)), ((), ())),
                                    preferred_element_type=f32) + sbb_ref[0])
    for i in range(NUM_BLOCKS):
        rs = rs + _swish(
            jax.lax.dot_general(rs, sbw_ref[i + 1], (((1,), (0,)), ((), ())),
                                preferred_element_type=f32) + sbb_ref[i + 1])
    rs_ref[...] = rs


def _tc_atom_final(a_list, h, ow1, ow2, sbw, sbb):
    grid = 10
    rb = N_ATOMS // grid
    aspec = pl.BlockSpec((NC, rb, EMB), lambda s: (0, s, 0))
    nb1 = NUM_BLOCKS + 1
    return pl.pallas_call(
        _tc_atom_final_body,
        grid=(grid,),
        in_specs=[aspec, aspec, aspec, aspec,
                  pl.BlockSpec((rb, EMB), lambda s: (s, 0)),
                  pl.BlockSpec((nb1, EMB, EMB), lambda s: (0, 0, 0)),
                  pl.BlockSpec((nb1, EMB, EMB), lambda s: (0, 0, 0)),
                  pl.BlockSpec((nb1, EMB, EMB), lambda s: (0, 0, 0)),
                  pl.BlockSpec((nb1, 1, EMB), lambda s: (0, 0, 0))],
        out_specs=[pl.BlockSpec((rb, EMB), lambda s: (s, 0)),
                   pl.BlockSpec((rb, EMB), lambda s: (s, 0))],
        out_shape=[jax.ShapeDtypeStruct((N_ATOMS, EMB), f32)] * 2,
    )(*a_list, h, ow1, ow2, sbw, sbb)


def _tc_mol_body(m_ref, cm_ref, cs_ref, o_ref):
    res = m_ref[0, 0:N_MOL, :] + m_ref[1, 0:N_MOL, :]
    sing = m_ref[0, 640:640 + N_MOL, :] + m_ref[1, 640:640 + N_MOL, :]
    o_ref[...] = cm_ref[0, 0] * res + cs_ref[0, 0] * sing


def _tc_mol(mo, cm, cs):
    return pl.pallas_call(
        _tc_mol_body,
        in_specs=[pl.BlockSpec((NC, _A2M_ROWS, EMB), lambda: (0, 0, 0)),
                  pl.BlockSpec((1, 1), lambda: (0, 0)),
                  pl.BlockSpec((1, 1), lambda: (0, 0))],
        out_specs=pl.BlockSpec((N_MOL, EMB), lambda: (0, 0)),
        out_shape=jax.ShapeDtypeStruct((N_MOL, EMB), f32),
        grid=(),
    )(mo, cm, cs)


def kernel(Z, R, batch_seg, idx_i, idx_j, idx_kj, idx_ji, cosine_ijk, params):
    p = params
    ep = N_EDGES_PAD - N_EDGES
    Zp = jnp.pad(Z.astype(i32), (0, N_ATOMS_PAD - N_ATOMS))
    r128 = jnp.pad(R.astype(f32), ((0, 0), (0, EMB - 3)))
    idx_i_g = jnp.pad(idx_i.astype(i32), (0, ep))
    idx_j_g = jnp.pad(idx_j.astype(i32), (0, ep))
    idx_i_s = jnp.pad(idx_i.astype(i32), (0, ep),
                      constant_values=N_ATOMS)
    tp = N_TRI_PAD - N_TRIPLETS
    idx_kj = jnp.pad(idx_kj.astype(i32), (0, tp))
    idx_ji = jnp.pad(idx_ji.astype(i32), (0, tp),
                     constant_values=1 << 29)
    seg_p = jnp.pad(batch_seg.astype(i32), (0, N_ATOMS_PAD - N_ATOMS),
                    constant_values=N_MOL)

    emb_b = p["emb_b"].reshape(1, EMB)
    wabf8 = jnp.pad(p["int_W_abf"], ((0, 0), (0, 8 - NUM_ABF), (0, 0)))
    cos2d = jnp.pad(cosine_ijk.astype(f32),
                    (0, N_TRI_PAD - N_TRIPLETS)).reshape(N_TRI_PAD // 128, 128)

    hp = _sc_h_gather(p["emb_table"], Zp)
    h = hp[:N_ATOMS]
    hw1, hw2 = _tc_atom_pre(h, p["emb_W"][:EMB], p["emb_W"][EMB:2 * EMB])
    gi, gj = _sc_edge_gather(hw1, hw2, idx_i_g, idx_j_g)
    d2d = _sc_edge_d(r128, idx_i_g, idx_j_g).reshape(N_EDGES_PAD // 128, 128)
    wr_x = p["emb_W_rbf"] @ p["emb_W"][2 * EMB:]
    x, t0, y = _tc_edge_init(gi, gj, d2d, wr_x, emb_b,
                             p["out_W_rbf"][0], p["int_W_rbf"][0])
    ma = _tc_mabf(cos2d, wabf8)

    a_list = [_sc_e2a(t0, idx_i_s)]
    last = x
    for i in range(NUM_BLOCKS):
        aggx = _sc_triplet(y, ma[i], idx_kj, idx_ji, last)
        want_y = i < NUM_BLOCKS - 1
        ow = p["out_W_rbf"][i + 1]
        iw = p["int_W_rbf"][i + 1] if want_y else p["int_W_rbf"][i]
        outs = _tc_mlp(aggx, last, d2d, p["int_W1"][i],
                       p["int_b1"][i].reshape(1, EMB), p["int_W2"][i],
                       p["int_b2"][i].reshape(1, EMB), ow, iw, want_y)
        if want_y:
            xn, t_next, y = outs
        else:
            xn, t_next = outs
        a_list.append(_sc_e2a(t_next, idx_i_s))
        last = xn

    ro, rs = _tc_atom_final(a_list, h, p["out_W1"], p["out_W2"], p["sb_W"],
                            p["sb_b"].reshape(NUM_BLOCKS + 1, 1, EMB))
    ro_p = jnp.pad(ro, ((0, N_ATOMS_PAD - N_ATOMS), (0, 0)))
    rs_p = jnp.pad(rs, ((0, N_ATOMS_PAD - N_ATOMS), (0, 0)))
    mo = _sc_a2m(ro_p, rs_p, seg_p)
    return _tc_mol(mo, p["coef_mp"].reshape(1, 1), p["coef_sg"].reshape(1, 1))


# speedup vs baseline: 1.6657x; 1.0485x over previous
import functools

import jax
import jax.numpy as jnp
import numpy as np
from jax import lax
from jax.experimental import pallas as pl
from jax.experimental.pallas import tpu as pltpu
from jax.experimental.pallas import tpu_sc as plsc

EMB = 128
NUM_BLOCKS = 3
NUM_RADIAL = 6
CUTOFF = 5.0
NUM_ABF = 7
N_ATOMS = 10000
N_EDGES = 160000
N_TRIPLETS = 320000
N_MOL = 512

N_ATOMS_PAD = 10240
N_EDGES_PAD = 163840
N_TRI_PAD = 327680

NC = 2
NS = 16
NW = NC * NS

f32 = jnp.float32
i32 = jnp.int32

_sc_mesh = plsc.VectorSubcoreMesh(core_axis_name="c", subcore_axis_name="s")


def _bc16(x):
    return jax.lax.broadcast_in_dim(x, (16,), ())


def _swish(x):
    return x * jax.nn.sigmoid(x)


def _sc_h_gather_body(tab_h, z_h, out_h, idx_v, rows_v, sem):
    wid = lax.axis_index("s") * NC + lax.axis_index("c")
    bpw = 512

    @pl.when(wid < 20)
    def _():
        base = wid * bpw
        pltpu.sync_copy(z_h.at[pl.ds(base, bpw)], idx_v)
        pltpu.async_copy(tab_h.at[idx_v], rows_v, sem).wait()
        pltpu.sync_copy(rows_v, out_h.at[pl.ds(base, bpw)])


def _sc_h_gather(tab, zp):
    k = pl.kernel(
        _sc_h_gather_body,
        out_type=jax.ShapeDtypeStruct((N_ATOMS_PAD, EMB), f32),
        mesh=_sc_mesh,
        compiler_params=pltpu.CompilerParams(needs_layout_passes=False),
        scratch_types=[
            pltpu.VMEM((512,), i32),
            pltpu.VMEM((512, EMB), f32),
            pltpu.SemaphoreType.DMA,
        ],
    )
    return k(tab, zp)


_EG_B = 128


def _sc_edge_gather_body(hw1_h, hw2_h, ii_h, jj_h, gi_h, gj_h,
                         ivA, jvA, baA, bbA, ivB, jvB, baB, bbB,
                         s0A, s1A, s0B, s1B):
    wid = lax.axis_index("s") * NC + lax.axis_index("c")
    bpw = N_EDGES_PAD // NW
    base = wid * bpw
    nb = bpw // _EG_B

    def fire(k, ivx, jvx, bax, bbx, s0x, s1x):
        off = base + k * _EG_B
        pltpu.sync_copy(ii_h.at[pl.ds(off, _EG_B)], ivx)
        pltpu.sync_copy(jj_h.at[pl.ds(off, _EG_B)], jvx)
        pltpu.async_copy(hw1_h.at[ivx], bax, s0x)
        pltpu.async_copy(hw2_h.at[jvx], bbx, s1x)

    def finish(k, bax, bbx, s0x, s1x):
        off = base + k * _EG_B
        pltpu.make_async_copy(hw1_h.at[pl.ds(0, _EG_B)], bax, s0x).wait()
        pltpu.make_async_copy(hw2_h.at[pl.ds(0, _EG_B)], bbx, s1x).wait()
        pltpu.sync_copy(bax, gi_h.at[pl.ds(off, _EG_B)])
        pltpu.sync_copy(bbx, gj_h.at[pl.ds(off, _EG_B)])

    fire(0, ivA, jvA, baA, bbA, s0A, s1A)

    def duo(g, _):
        k0 = g * 2
        fire(k0 + 1, ivB, jvB, baB, bbB, s0B, s1B)
        finish(k0, baA, bbA, s0A, s1A)

        @pl.when(k0 + 2 < nb)
        def _():
            fire(k0 + 2, ivA, jvA, baA, bbA, s0A, s1A)
        finish(k0 + 1, baB, bbB, s0B, s1B)
        return 0
    lax.fori_loop(0, nb // 2, duo, 0)


def _sc_edge_gather(hw1, hw2, idx_i, idx_j):
    k = pl.kernel(
        _sc_edge_gather_body,
        out_type=(
            jax.ShapeDtypeStruct((N_EDGES_PAD, EMB), f32),
            jax.ShapeDtypeStruct((N_EDGES_PAD, EMB), f32),
        ),
        mesh=_sc_mesh,
        compiler_params=pltpu.CompilerParams(needs_layout_passes=False),
        scratch_types=[
            pltpu.VMEM((_EG_B,), i32),
            pltpu.VMEM((_EG_B,), i32),
            pltpu.VMEM((_EG_B, EMB), f32),
            pltpu.VMEM((_EG_B, EMB), f32),
            pltpu.VMEM((_EG_B,), i32),
            pltpu.VMEM((_EG_B,), i32),
            pltpu.VMEM((_EG_B, EMB), f32),
            pltpu.VMEM((_EG_B, EMB), f32),
            pltpu.SemaphoreType.DMA,
            pltpu.SemaphoreType.DMA,
            pltpu.SemaphoreType.DMA,
            pltpu.SemaphoreType.DMA,
        ],
    )
    return k(hw1, hw2, idx_i, idx_j)


_D_B = 256


def _sc_edge_d_body(r128_h, ii_h, jj_h, d_h, iv, jv, rib, rjb, dbuf, s0, s1):
    wid = lax.axis_index("s") * NC + lax.axis_index("c")
    bpw = N_EDGES_PAD // NW
    base = wid * bpw

    def batch(k, _):
        off = k * _D_B
        pltpu.sync_copy(ii_h.at[pl.ds(base + off, _D_B)], iv)
        pltpu.sync_copy(jj_h.at[pl.ds(base + off, _D_B)], jv)
        c0 = pltpu.async_copy(r128_h.at[iv], rib, s0)
        c1 = pltpu.async_copy(r128_h.at[jv], rjb, s1)
        c0.wait(); c1.wait()

        def row(r, _2):
            lns = lax.broadcasted_iota(i32, (16,), 0)
            cm = jnp.where(lns < jnp.full((16,), 3, i32),
                           jnp.ones((16,), f32), jnp.zeros((16,), f32))
            l0 = lns == jnp.zeros((16,), i32)
            eps = jnp.where(l0, jnp.full((16,), 1e-12, f32),
                            jnp.zeros((16,), f32))
            va = rib[r, pl.ds(0, 16)] - rjb[r, pl.ds(0, 16)]
            sq = va * va * cm + eps
            s = jnp.sum(sq)
            plsc.store_scatter(dbuf, [_bc16(off + r)], _bc16(s), mask=l0)
            return 0
        lax.fori_loop(0, _D_B, row, 0)
        return 0

    lax.fori_loop(0, bpw // _D_B, batch, 0)
    pltpu.sync_copy(dbuf, d_h.at[pl.ds(base, bpw)])


def _sc_edge_d(r128, idx_i, idx_j):
    k = pl.kernel(
        _sc_edge_d_body,
        out_type=jax.ShapeDtypeStruct((N_EDGES_PAD,), f32),
        mesh=_sc_mesh,
        compiler_params=pltpu.CompilerParams(needs_layout_passes=False),
        scratch_types=[
            pltpu.VMEM((_D_B,), i32),
            pltpu.VMEM((_D_B,), i32),
            pltpu.VMEM((_D_B, EMB), f32),
            pltpu.VMEM((_D_B, EMB), f32),
            pltpu.VMEM((N_EDGES_PAD // NW,), f32),
            pltpu.SemaphoreType.DMA,
            pltpu.SemaphoreType.DMA,
        ],
    )
    return k(r128, idx_i, idx_j)


_CH = 10000
_NCHUNK = N_EDGES // _CH
_TSL = N_TRI_PAD // NS
_SB = 2048
_DR = 64
_CAP = 2304
_PKM = (1 << 14) - 1


def _sc_triplet_body(y_h, ma_h, kj_h, ji_h, lx_h, out_h,
                     ji_s, kj_s, pk_b, cp_b, cnb,
                     cdA, ckA, cpA, cdB, ckB, cpB,
                     ybA, mbA, ybB, mbB, acc, syA, smA, syB, smB):
    c = lax.axis_index("c")
    s = lax.axis_index("s")

    def fire(off, cdx, ckx, cpx, ybx, mbx, sy, sm):
        for v in range(_DR // 16):
            pk = pk_b[pl.ds(off + v * 16, 16)]
            cdx[pl.ds(v * 16, 16)] = pk & jnp.full((16,), _PKM, i32)
            ckx[pl.ds(v * 16, 16)] = lax.shift_right_logical(
                pk, jnp.full((16,), 14, i32))
            cpx[pl.ds(v * 16, 16)] = cp_b[pl.ds(off + v * 16, 16)]
        pltpu.async_copy(y_h.at[ckx], ybx, sy)
        pltpu.async_copy(ma_h.at[cpx], mbx, sm)

    def finish(cdx, ybx, mbx, sy, sm):
        pltpu.make_async_copy(y_h.at[pl.ds(0, _DR)], ybx, sy).wait()
        pltpu.make_async_copy(ma_h.at[pl.ds(0, _DR)], mbx, sm).wait()

        def mulrow(r, _):
            for cc in range(EMB // 16):
                ybx[r, pl.ds(cc * 16, 16)] = (
                    ybx[r, pl.ds(cc * 16, 16)] * mbx[r, pl.ds(cc * 16, 16)])
            return 0
        lax.fori_loop(0, _DR, mulrow, 0)
        pltpu.sync_copy(ybx, acc.at[cdx], add=True)

    for kc in range(_NCHUNK // NC):
        chunk = kc * NC + c
        lo = chunk * _CH

        @pl.when(s < 10)
        def _():
            pltpu.sync_copy(lx_h.at[pl.ds(lo + s * 1000, 1000)],
                            acc.at[pl.ds(s * 1000, 1000)])
        plsc.subcore_barrier()

        def scan_batch(b, cnt):
            tbase = s * _TSL + b * _SB
            pltpu.sync_copy(ji_h.at[pl.ds(tbase, _SB)], ji_s)
            pltpu.sync_copy(kj_h.at[pl.ds(tbase, _SB)], kj_s)

            def scan_vec(v, cnt2):
                lanes = lax.broadcasted_iota(i32, (16,), 0)
                jiv = ji_s[pl.ds(v * 16, 16)]
                kjv = kj_s[pl.ds(v * 16, 16)]
                lv = jiv - _bc16(lo)
                m = ((lv >= jnp.zeros((16,), i32))
                     & (lv < jnp.full((16,), _CH, i32)))
                mi = m.astype(i32)
                csum = plsc.cumsum(mi)
                tgt = _bc16(cnt2) + csum - jnp.ones((16,), i32)
                pk = lv | lax.shift_left(kjv, jnp.full((16,), 14, i32))
                plsc.store_scatter(pk_b, [tgt], pk, mask=m)
                posv = _bc16(tbase + v * 16) + lanes
                plsc.store_scatter(cp_b, [tgt], posv, mask=m)
                return cnt2 + csum[15]
            cnt = lax.fori_loop(0, _SB // 16, scan_vec, cnt)

            nfull = cnt // _DR

            @pl.when(nfull > 0)
            def _():
                fire(0, cdA, ckA, cpA, ybA, mbA, syA, smA)

            def duo(g, _):
                f1 = g * 2 + 1

                @pl.when(f1 < nfull)
                def _():
                    fire(f1 * _DR, cdB, ckB, cpB, ybB, mbB, syB, smB)
                finish(cdA, ybA, mbA, syA, smA)

                @pl.when(f1 < nfull)
                def _():
                    @pl.when(f1 + 1 < nfull)
                    def _():
                        fire((f1 + 1) * _DR, cdA, ckA, cpA, ybA, mbA,
                             syA, smA)
                    finish(cdB, ybB, mbB, syB, smB)
                return 0
            lax.fori_loop(0, (nfull + 1) // 2, duo, 0)

            rem = cnt - nfull * _DR
            off0 = nfull * _DR
            vals = []
            for v in range(_DR // 16):
                vals.append((pk_b[pl.ds(off0 + v * 16, 16)],
                             cp_b[pl.ds(off0 + v * 16, 16)]))
            for v, (a, bb2) in enumerate(vals):
                pk_b[pl.ds(v * 16, 16)] = a
                cp_b[pl.ds(v * 16, 16)] = bb2
            return rem

        cnt = lax.fori_loop(0, _TSL // _SB, scan_batch, jnp.int32(0))

        for v in range(_DR // 16):
            lanes = lax.broadcasted_iota(i32, (16,), 0)
            l = _bc16(jnp.int32(v * 16)) + lanes
            good = l < _bc16(cnt)
            pk_b[pl.ds(v * 16, 16)] = jnp.where(
                good, pk_b[pl.ds(v * 16, 16)], jnp.full((16,), _CH, i32))
            cp_b[pl.ds(v * 16, 16)] = jnp.where(
                good, cp_b[pl.ds(v * 16, 16)], jnp.zeros((16,), i32))
        fire(0, cdA, ckA, cpA, ybA, mbA, syA, smA)
        finish(cdA, ybA, mbA, syA, smA)

        plsc.subcore_barrier()

        @pl.when(s < 10)
        def _():
            pltpu.sync_copy(acc.at[pl.ds(s * 1000, 1000)],
                            out_h.at[pl.ds(lo + s * 1000, 1000)])
        plsc.subcore_barrier()


def _sc_triplet(y, ma, kj, ji, lx):
    k = pl.kernel(
        _sc_triplet_body,
        out_type=jax.ShapeDtypeStruct((N_EDGES_PAD, EMB), f32),
        mesh=_sc_mesh,
        compiler_params=pltpu.CompilerParams(needs_layout_passes=False),
        scratch_types=[
            pltpu.VMEM((_SB,), i32),
            pltpu.VMEM((_SB,), i32),
            pltpu.VMEM((_CAP,), i32),
            pltpu.VMEM((_CAP,), i32),
            pltpu.VMEM((16,), i32),
            pltpu.VMEM((_DR,), i32),
            pltpu.VMEM((_DR,), i32),
            pltpu.VMEM((_DR,), i32),
            pltpu.VMEM((_DR,), i32),
            pltpu.VMEM((_DR,), i32),
            pltpu.VMEM((_DR,), i32),
            pltpu.VMEM((_DR, EMB), f32),
            pltpu.VMEM((_DR, EMB), f32),
            pltpu.VMEM((_DR, EMB), f32),
            pltpu.VMEM((_DR, EMB), f32),
            pltpu.VMEM_SHARED((_CH + 8, EMB), f32),
            pltpu.SemaphoreType.DMA,
            pltpu.SemaphoreType.DMA,
            pltpu.SemaphoreType.DMA,
            pltpu.SemaphoreType.DMA,
        ],
    )
    return k(y, ma, kj, ji, lx)


_E2A_B = 128
_E2A_ACC = 10240


def _sc_e2a_body(t_h, ii_h, out_h, ivA, tbA, ivB, tbB, acc, sA, sB):
    c = lax.axis_index("c")
    s = lax.axis_index("s")
    span = _E2A_ACC // NS

    def zrow(r, _):
        for cc in range(EMB // 16):
            tbA[r, pl.ds(cc * 16, 16)] = jnp.zeros((16,), f32)
        return 0
    lax.fori_loop(0, _E2A_B, zrow, 0)
    for z in range(span // _E2A_B):
        pltpu.sync_copy(tbA, acc.at[pl.ds(s * span + z * _E2A_B, _E2A_B)])
    plsc.subcore_barrier()

    bpw = N_EDGES_PAD // NW
    base = (c * NS + s) * bpw
    nb = bpw // _E2A_B

    def fire(k, ivx, tbx, sx):
        off = base + k * _E2A_B
        pltpu.sync_copy(ii_h.at[pl.ds(off, _E2A_B)], ivx)
        pltpu.async_copy(t_h.at[pl.ds(off, _E2A_B)], tbx, sx)

    def finish(ivx, tbx, sx):
        pltpu.make_async_copy(t_h.at[pl.ds(0, _E2A_B)], tbx, sx).wait()
        pltpu.sync_copy(tbx, acc.at[ivx], add=True)

    fire(0, ivA, tbA, sA)

    def duo(g, _):
        k0 = g * 2
        fire(k0 + 1, ivB, tbB, sB)
        finish(ivA, tbA, sA)

        @pl.when(k0 + 2 < nb)
        def _():
            fire(k0 + 2, ivA, tbA, sA)
        finish(ivB, tbB, sB)
        return 0
    lax.fori_loop(0, nb // 2, duo, 0)

    plsc.subcore_barrier()

    @pl.when(s < 10)
    def _():
        pltpu.sync_copy(acc.at[pl.ds(s * 1000, 1000)],
                        out_h.at[c, pl.ds(s * 1000, 1000)])


def _sc_e2a(t, idx_i):
    k = pl.kernel(
        _sc_e2a_body,
        out_type=jax.ShapeDtypeStruct((NC, N_ATOMS, EMB), f32),
        mesh=_sc_mesh,
        compiler_params=pltpu.CompilerParams(needs_layout_passes=False),
        scratch_types=[
            pltpu.VMEM((_E2A_B,), i32),
            pltpu.VMEM((_E2A_B, EMB), f32),
            pltpu.VMEM((_E2A_B,), i32),
            pltpu.VMEM((_E2A_B, EMB), f32),
            pltpu.VMEM_SHARED((_E2A_ACC, EMB), f32),
            pltpu.SemaphoreType.DMA,
            pltpu.SemaphoreType.DMA,
        ],
    )
    return k(t, idx_i)


_A2M_ROWS = 1280


def _sc_a2m_body(ro_h, rs_h, seg_h, out_h, idx_v, idx2_v, buf, zbuf, acc, sem):
    c = lax.axis_index("c")
    s = lax.axis_index("s")
    wid = s * NC + c
    span = _A2M_ROWS // NS

    def zrow(r, _):
        for cc in range(EMB // 16):
            zbuf[r, pl.ds(cc * 16, 16)] = jnp.zeros((16,), f32)
        return 0
    lax.fori_loop(0, span, zrow, 0)
    pltpu.sync_copy(zbuf, acc.at[pl.ds(s * span, span)])
    plsc.subcore_barrier()

    bpw = 512

    @pl.when(wid < 20)
    def _():
        base = wid * bpw
        pltpu.sync_copy(seg_h.at[pl.ds(base, bpw)], idx_v)
        for v in range(bpw // 16):
            idx2_v[pl.ds(v * 16, 16)] = (idx_v[pl.ds(v * 16, 16)]
                                         + jnp.full((16,), 640, i32))
        pltpu.sync_copy(ro_h.at[pl.ds(base, bpw)], buf)
        pltpu.sync_copy(buf, acc.at[idx_v], add=True)
        pltpu.sync_copy(rs_h.at[pl.ds(base, bpw)], buf)
        pltpu.sync_copy(buf, acc.at[idx2_v], add=True)

    plsc.subcore_barrier()
    pltpu.sync_copy(acc.at[pl.ds(s * span, span)],
                    out_h.at[c, pl.ds(s * span, span)])


def _sc_a2m(ro, rs, seg):
    k = pl.kernel(
        _sc_a2m_body,
        out_type=jax.ShapeDtypeStruct((NC, _A2M_ROWS, EMB), f32),
        mesh=_sc_mesh,
        compiler_params=pltpu.CompilerParams(needs_layout_passes=False),
        scratch_types=[
            pltpu.VMEM((512,), i32),
            pltpu.VMEM((512,), i32),
            pltpu.VMEM((512, EMB), f32),
            pltpu.VMEM((_A2M_ROWS // NS, EMB), f32),
            pltpu.VMEM_SHARED((_A2M_ROWS, EMB), f32),
            pltpu.SemaphoreType.DMA,
        ],
    )
    return k(ro, rs, seg)


_EB = 2048
_EGRID = N_EDGES_PAD // _EB
_DB = _EB // 128


def _rbf_tiles(d2):
    d = jnp.sqrt(d2)
    scale = np.sqrt(2.0 / CUTOFF).astype(np.float32)
    inv = 1.0 / (d + 1e-6)
    return [scale * jnp.sin((k + 1) * np.pi * d / CUTOFF) * inv
            for k in range(NUM_RADIAL)]


def _rbf_mix_rows(tiles, zero_row, w6, a):
    del zero_row
    stack = jnp.concatenate([t[a:a + 1] for t in tiles], axis=0)
    return jax.lax.dot_general(stack, w6, (((0,), (0,)), ((), ())),
                               preferred_element_type=f32)


def _tc_atom_pre_body(h_ref, wa_ref, wb_ref, o1_ref, o2_ref):
    h = h_ref[...]
    o1_ref[...] = jax.lax.dot_general(h, wa_ref[...], (((1,), (0,)), ((), ())),
                                      preferred_element_type=f32)
    o2_ref[...] = jax.lax.dot_general(h, wb_ref[...], (((1,), (0,)), ((), ())),
                                      preferred_element_type=f32)


def _tc_atom_pre(h, wa, wb):
    grid = 10
    rb = N_ATOMS // grid
    return pl.pallas_call(
        _tc_atom_pre_body,
        grid=(grid,),
        in_specs=[
            pl.BlockSpec((rb, EMB), lambda s: (s, 0)),
            pl.BlockSpec((EMB, EMB), lambda s: (0, 0)),
            pl.BlockSpec((EMB, EMB), lambda s: (0, 0)),
        ],
        out_specs=[
            pl.BlockSpec((rb, EMB), lambda s: (s, 0)),
            pl.BlockSpec((rb, EMB), lambda s: (s, 0)),
        ],
        out_shape=[
            jax.ShapeDtypeStruct((N_ATOMS, EMB), f32),
            jax.ShapeDtypeStruct((N_ATOMS, EMB), f32),
        ],
    )(h, wa, wb)


def _tc_edge_init_body(gi_ref, gj_ref, d_ref, wr_ref, b_ref,
                       ow_ref, iw_ref, x_ref, t_ref, y_ref):
    tiles = _rbf_tiles(d_ref[...])
    zero_row = jnp.zeros((1, 128), f32)
    for a in range(_DB):
        rows = pl.ds(a * 128, 128)
        rbfe = _rbf_mix_rows(tiles, zero_row, wr_ref[...], a)
        xv = _swish(gi_ref[rows, :] + gj_ref[rows, :] + rbfe + b_ref[...])
        x_ref[rows, :] = xv
        t_ref[rows, :] = xv * _rbf_mix_rows(tiles, zero_row, ow_ref[...], a)
        y_ref[rows, :] = xv * _rbf_mix_rows(tiles, zero_row, iw_ref[...], a)


def _tc_edge_init(gi, gj, d2d, wr, b, ow, iw):
    espec = pl.BlockSpec((_EB, EMB), lambda s: (s, 0))
    dspec = pl.BlockSpec((_DB, 128), lambda s: (s, 0))
    wspec = pl.BlockSpec((NUM_RADIAL, EMB), lambda s: (0, 0))
    return pl.pallas_call(
        _tc_edge_init_body,
        grid=(_EGRID,),
        in_specs=[espec, espec, dspec, wspec,
                  pl.BlockSpec((1, EMB), lambda s: (0, 0)), wspec, wspec],
        out_specs=[espec, espec, espec],
        out_shape=[jax.ShapeDtypeStruct((N_EDGES_PAD, EMB), f32)] * 3,
    )(gi, gj, d2d, wr, b, ow, iw)


_MA_R = 16


def _tc_mabf_body(c_ref, w_ref, o0_ref, o1_ref, o2_ref):
    c = c_ref[...]
    polys = [jnp.ones_like(c), c]
    for _ in range(NUM_ABF - 2):
        polys.append(2.0 * c * polys[-1] - polys[-2])
    zero = jnp.zeros((1, 128), f32)
    outs = (o0_ref, o1_ref, o2_ref)
    for a in range(_MA_R):
        stack = jnp.concatenate(
            [polys[k][a:a + 1] for k in range(NUM_ABF)] + [zero],
            axis=0)
        for i in range(NUM_BLOCKS):
            outs[i][pl.ds(a * 128, 128), :] = jax.lax.dot_general(
                stack, w_ref[i], (((0,), (0,)), ((), ())),
                preferred_element_type=f32)


def _tc_mabf(cos2d, wabf8):
    grid = (N_TRI_PAD // 128) // _MA_R
    ospec = pl.BlockSpec((_MA_R * 128, EMB), lambda s: (s, 0))
    return pl.pallas_call(
        _tc_mabf_body,
        grid=(grid,),
        in_specs=[
            pl.BlockSpec((_MA_R, 128), lambda s: (s, 0)),
            pl.BlockSpec((NUM_BLOCKS, 8, EMB), lambda s: (0, 0, 0)),
        ],
        out_specs=[ospec, ospec, ospec],
        out_shape=[jax.ShapeDtypeStruct((N_TRI_PAD, EMB), f32)] * 3,
    )(cos2d, wabf8)


def _tc_mlp_body(ax_ref, lx_ref, d_ref, w1_ref, b1_ref, w2_ref,
                 b2_ref, ow_ref, iw_ref, xn_ref, t_ref, y_ref):
    u = _swish(jax.lax.dot_general(ax_ref[...], w1_ref[...],
                                   (((1,), (0,)), ((), ())),
                                   preferred_element_type=f32) + b1_ref[...])
    xn = lx_ref[...] + _swish(
        jax.lax.dot_general(u, w2_ref[...], (((1,), (0,)), ((), ())),
                            preferred_element_type=f32) + b2_ref[...])
    xn_ref[...] = xn
    tiles = _rbf_tiles(d_ref[...])
    zero_row = jnp.zeros((1, 128), f32)
    for a in range(_DB):
        rows = pl.ds(a * 128, 128)
        t_ref[rows, :] = xn[a * 128:(a + 1) * 128, :] * _rbf_mix_rows(
            tiles, zero_row, ow_ref[...], a)
        if y_ref is not None:
            y_ref[rows, :] = xn[a * 128:(a + 1) * 128, :] * _rbf_mix_rows(
                tiles, zero_row, iw_ref[...], a)


def _tc_mlp(aggx, lastx, d2d, w1, b1, w2, b2, ow, iw, want_y):
    espec = pl.BlockSpec((_EB, EMB), lambda s: (s, 0))
    dspec = pl.BlockSpec((_DB, 128), lambda s: (s, 0))
    mspec = pl.BlockSpec((EMB, EMB), lambda s: (0, 0))
    bspec = pl.BlockSpec((1, EMB), lambda s: (0, 0))
    wspec = pl.BlockSpec((NUM_RADIAL, EMB), lambda s: (0, 0))
    if want_y:
        body = _tc_mlp_body
        out_specs = [espec, espec, espec]
        out_shape = [jax.ShapeDtypeStruct((N_EDGES_PAD, EMB), f32)] * 3
    else:
        def body(ax, lx, d_, w1_, b1_, w2_, b2_, ow_, iw_, xn_, t_):
            _tc_mlp_body(ax, lx, d_, w1_, b1_, w2_, b2_, ow_, iw_,
                         xn_, t_, None)
        out_specs = [espec, espec]
        out_shape = [jax.ShapeDtypeStruct((N_EDGES_PAD, EMB), f32)] * 2
    return pl.pallas_call(
        body,
        grid=(_EGRID,),
        in_specs=[espec, espec, dspec, mspec, bspec, mspec, bspec,
                  wspec, wspec],
        out_specs=out_specs,
        out_shape=out_shape,
    )(aggx, lastx, d2d, w1, b1, w2, b2, ow, iw)


def _tc_atom_final_body(a0_ref, a1_ref, a2_ref, a3_ref, h_ref,
                        ow1_ref, ow2_ref, sbw_ref, sbb_ref,
                        ro_ref, rs_ref):
    arefs = (a0_ref, a1_ref, a2_ref, a3_ref)
    ro = None
    for i in range(NUM_BLOCKS + 1):
        a = arefs[i][0] + arefs[i][1]
        u = _swish(jax.lax.dot_general(a, ow1_ref[i], (((1,), (0,)), ((), ())),
                                       preferred_element_type=f32))
        v = jax.lax.dot_general(u, ow2_ref[i], (((1,), (0,)), ((), ())),
                                preferred_element_type=f32)
        ro = v if ro is None else ro + v
    ro_ref[...] = ro
    rs = _swish(jax.lax.dot_general(h_ref[...], sbw_ref[0],
                                    (((1,), (0,)), ((), ())),
                                    preferred_element_type=f32) + sbb_ref[0])
    for i in range(NUM_BLOCKS):
        rs = rs + _swish(
            jax.lax.dot_general(rs, sbw_ref[i + 1], (((1,), (0,)), ((), ())),
                                preferred_element_type=f32) + sbb_ref[i + 1])
    rs_ref[...] = rs


def _tc_atom_final(a_list, h, ow1, ow2, sbw, sbb):
    grid = 10
    rb = N_ATOMS // grid
    aspec = pl.BlockSpec((NC, rb, EMB), lambda s: (0, s, 0))
    nb1 = NUM_BLOCKS + 1
    return pl.pallas_call(
        _tc_atom_final_body,
        grid=(grid,),
        in_specs=[aspec, aspec, aspec, aspec,
                  pl.BlockSpec((rb, EMB), lambda s: (s, 0)),
                  pl.BlockSpec((nb1, EMB, EMB), lambda s: (0, 0, 0)),
                  pl.BlockSpec((nb1, EMB, EMB), lambda s: (0, 0, 0)),
                  pl.BlockSpec((nb1, EMB, EMB), lambda s: (0, 0, 0)),
                  pl.BlockSpec((nb1, 1, EMB), lambda s: (0, 0, 0))],
        out_specs=[pl.BlockSpec((rb, EMB), lambda s: (s, 0)),
                   pl.BlockSpec((rb, EMB), lambda s: (s, 0))],
        out_shape=[jax.ShapeDtypeStruct((N_ATOMS, EMB), f32)] * 2,
    )(*a_list, h, ow1, ow2, sbw, sbb)


def _tc_mol_body(m_ref, cm_ref, cs_ref, o_ref):
    res = m_ref[0, 0:N_MOL, :] + m_ref[1, 0:N_MOL, :]
    sing = m_ref[0, 640:640 + N_MOL, :] + m_ref[1, 640:640 + N_MOL, :]
    o_ref[...] = cm_ref[0, 0] * res + cs_ref[0, 0] * sing


def _tc_mol(mo, cm, cs):
    return pl.pallas_call(
        _tc_mol_body,
        in_specs=[pl.BlockSpec((NC, _A2M_ROWS, EMB), lambda: (0, 0, 0)),
                  pl.BlockSpec((1, 1), lambda: (0, 0)),
                  pl.BlockSpec((1, 1), lambda: (0, 0))],
        out_specs=pl.BlockSpec((N_MOL, EMB), lambda: (0, 0)),
        out_shape=jax.ShapeDtypeStruct((N_MOL, EMB), f32),
        grid=(),
    )(mo, cm, cs)


def kernel(Z, R, batch_seg, idx_i, idx_j, idx_kj, idx_ji, cosine_ijk, params):
    p = params
    ep = N_EDGES_PAD - N_EDGES
    Zp = jnp.pad(Z.astype(i32), (0, N_ATOMS_PAD - N_ATOMS))
    r128 = jnp.pad(R.astype(f32), ((0, 0), (0, EMB - 3)))
    idx_i_g = jnp.pad(idx_i.astype(i32), (0, ep))
    idx_j_g = jnp.pad(idx_j.astype(i32), (0, ep))
    idx_i_s = jnp.pad(idx_i.astype(i32), (0, ep),
                      constant_values=N_ATOMS)
    tp = N_TRI_PAD - N_TRIPLETS
    idx_kj = jnp.pad(idx_kj.astype(i32), (0, tp))
    idx_ji = jnp.pad(idx_ji.astype(i32), (0, tp),
                     constant_values=1 << 29)
    seg_p = jnp.pad(batch_seg.astype(i32), (0, N_ATOMS_PAD - N_ATOMS),
                    constant_values=N_MOL)

    emb_b = p["emb_b"].reshape(1, EMB)
    wabf8 = jnp.pad(p["int_W_abf"], ((0, 0), (0, 8 - NUM_ABF), (0, 0)))
    cos2d = jnp.pad(cosine_ijk.astype(f32),
                    (0, N_TRI_PAD - N_TRIPLETS)).reshape(N_TRI_PAD // 128, 128)

    hp = _sc_h_gather(p["emb_table"], Zp)
    h = hp[:N_ATOMS]
    hw1, hw2 = _tc_atom_pre(h, p["emb_W"][:EMB], p["emb_W"][EMB:2 * EMB])
    gi, gj = _sc_edge_gather(hw1, hw2, idx_i_g, idx_j_g)
    d2d = _sc_edge_d(r128, idx_i_g, idx_j_g).reshape(N_EDGES_PAD // 128, 128)
    wr_x = p["emb_W_rbf"] @ p["emb_W"][2 * EMB:]
    x, t0, y = _tc_edge_init(gi, gj, d2d, wr_x, emb_b,
                             p["out_W_rbf"][0], p["int_W_rbf"][0])
    ma = _tc_mabf(cos2d, wabf8)

    a_list = [_sc_e2a(t0, idx_i_s)]
    last = x
    for i in range(NUM_BLOCKS):
        aggx = _sc_triplet(y, ma[i], idx_kj, idx_ji, last)
        want_y = i < NUM_BLOCKS - 1
        ow = p["out_W_rbf"][i + 1]
        iw = p["int_W_rbf"][i + 1] if want_y else p["int_W_rbf"][i]
        outs = _tc_mlp(aggx, last, d2d, p["int_W1"][i],
                       p["int_b1"][i].reshape(1, EMB), p["int_W2"][i],
                       p["int_b2"][i].reshape(1, EMB), ow, iw, want_y)
        if want_y:
            xn, t_next, y = outs
        else:
            xn, t_next = outs
        a_list.append(_sc_e2a(t_next, idx_i_s))
        last = xn

    ro, rs = _tc_atom_final(a_list, h, p["out_W1"], p["out_W2"], p["sb_W"],
                            p["sb_b"].reshape(NUM_BLOCKS + 1, 1, EMB))
    ro_p = jnp.pad(ro, ((0, N_ATOMS_PAD - N_ATOMS), (0, 0)))
    rs_p = jnp.pad(rs, ((0, N_ATOMS_PAD - N_ATOMS), (0, 0)))
    mo = _sc_a2m(ro_p, rs_p, seg_p)
    return _tc_mol(mo, p["coef_mp"].reshape(1, 1), p["coef_sg"].reshape(1, 1))


# speedup vs baseline: 1.8417x; 1.1056x over previous
import functools

import jax
import jax.numpy as jnp
import numpy as np
from jax import lax
from jax.experimental import pallas as pl
from jax.experimental.pallas import tpu as pltpu
from jax.experimental.pallas import tpu_sc as plsc

EMB = 128
NUM_BLOCKS = 3
NUM_RADIAL = 6
CUTOFF = 5.0
NUM_ABF = 7
N_ATOMS = 10000
N_EDGES = 160000
N_TRIPLETS = 320000
N_MOL = 512

N_ATOMS_PAD = 10240
N_EDGES_PAD = 163840
N_TRI_PAD = 327680

NC = 2
NS = 16
NW = NC * NS

f32 = jnp.float32
i32 = jnp.int32

_sc_mesh = plsc.VectorSubcoreMesh(core_axis_name="c", subcore_axis_name="s")


def _bc16(x):
    return jax.lax.broadcast_in_dim(x, (16,), ())


def _swish(x):
    return x * jax.nn.sigmoid(x)


def _sc_h_gather_body(tab_h, z_h, out_h, idx_v, rows_v, sem):
    wid = lax.axis_index("s") * NC + lax.axis_index("c")
    bpw = 512

    @pl.when(wid < 20)
    def _():
        base = wid * bpw
        pltpu.sync_copy(z_h.at[pl.ds(base, bpw)], idx_v)
        pltpu.async_copy(tab_h.at[idx_v], rows_v, sem).wait()
        pltpu.sync_copy(rows_v, out_h.at[pl.ds(base, bpw)])


def _sc_h_gather(tab, zp):
    k = pl.kernel(
        _sc_h_gather_body,
        out_type=jax.ShapeDtypeStruct((N_ATOMS_PAD, EMB), f32),
        mesh=_sc_mesh,
        compiler_params=pltpu.CompilerParams(needs_layout_passes=False),
        scratch_types=[
            pltpu.VMEM((512,), i32),
            pltpu.VMEM((512, EMB), f32),
            pltpu.SemaphoreType.DMA,
        ],
    )
    return k(tab, zp)


_EG_B = 128


def _sc_edge_gather_body(hw1_h, hw2_h, ii_h, jj_h, gi_h, gj_h,
                         ivA, jvA, baA, bbA, ivB, jvB, baB, bbB,
                         s0A, s1A, s0B, s1B):
    wid = lax.axis_index("s") * NC + lax.axis_index("c")
    bpw = N_EDGES_PAD // NW
    base = wid * bpw
    nb = bpw // _EG_B

    def fire(k, ivx, jvx, bax, bbx, s0x, s1x):
        off = base + k * _EG_B
        pltpu.sync_copy(ii_h.at[pl.ds(off, _EG_B)], ivx)
        pltpu.sync_copy(jj_h.at[pl.ds(off, _EG_B)], jvx)
        pltpu.async_copy(hw1_h.at[ivx], bax, s0x)
        pltpu.async_copy(hw2_h.at[jvx], bbx, s1x)

    def finish(k, bax, bbx, s0x, s1x):
        off = base + k * _EG_B
        pltpu.make_async_copy(hw1_h.at[pl.ds(0, _EG_B)], bax, s0x).wait()
        pltpu.make_async_copy(hw2_h.at[pl.ds(0, _EG_B)], bbx, s1x).wait()
        pltpu.sync_copy(bax, gi_h.at[pl.ds(off, _EG_B)])
        pltpu.sync_copy(bbx, gj_h.at[pl.ds(off, _EG_B)])

    fire(0, ivA, jvA, baA, bbA, s0A, s1A)

    def duo(g, _):
        k0 = g * 2
        fire(k0 + 1, ivB, jvB, baB, bbB, s0B, s1B)
        finish(k0, baA, bbA, s0A, s1A)

        @pl.when(k0 + 2 < nb)
        def _():
            fire(k0 + 2, ivA, jvA, baA, bbA, s0A, s1A)
        finish(k0 + 1, baB, bbB, s0B, s1B)
        return 0
    lax.fori_loop(0, nb // 2, duo, 0)


def _sc_edge_gather(hw1, hw2, idx_i, idx_j):
    k = pl.kernel(
        _sc_edge_gather_body,
        out_type=(
            jax.ShapeDtypeStruct((N_EDGES_PAD, EMB), f32),
            jax.ShapeDtypeStruct((N_EDGES_PAD, EMB), f32),
        ),
        mesh=_sc_mesh,
        compiler_params=pltpu.CompilerParams(needs_layout_passes=False),
        scratch_types=[
            pltpu.VMEM((_EG_B,), i32),
            pltpu.VMEM((_EG_B,), i32),
            pltpu.VMEM((_EG_B, EMB), f32),
            pltpu.VMEM((_EG_B, EMB), f32),
            pltpu.VMEM((_EG_B,), i32),
            pltpu.VMEM((_EG_B,), i32),
            pltpu.VMEM((_EG_B, EMB), f32),
            pltpu.VMEM((_EG_B, EMB), f32),
            pltpu.SemaphoreType.DMA,
            pltpu.SemaphoreType.DMA,
            pltpu.SemaphoreType.DMA,
            pltpu.SemaphoreType.DMA,
        ],
    )
    return k(hw1, hw2, idx_i, idx_j)


_D_B = 256


def _sc_edge_d_body(r128_h, ii_h, jj_h, d_h, iv, jv, rib, rjb, dbuf, s0, s1):
    wid = lax.axis_index("s") * NC + lax.axis_index("c")
    bpw = N_EDGES_PAD // NW
    base = wid * bpw

    def batch(k, _):
        off = k * _D_B
        pltpu.sync_copy(ii_h.at[pl.ds(base + off, _D_B)], iv)
        pltpu.sync_copy(jj_h.at[pl.ds(base + off, _D_B)], jv)
        c0 = pltpu.async_copy(r128_h.at[iv], rib, s0)
        c1 = pltpu.async_copy(r128_h.at[jv], rjb, s1)
        c0.wait(); c1.wait()

        def row(r, _2):
            lns = lax.broadcasted_iota(i32, (16,), 0)
            cm = jnp.where(lns < jnp.full((16,), 3, i32),
                           jnp.ones((16,), f32), jnp.zeros((16,), f32))
            l0 = lns == jnp.zeros((16,), i32)
            eps = jnp.where(l0, jnp.full((16,), 1e-12, f32),
                            jnp.zeros((16,), f32))
            va = rib[r, pl.ds(0, 16)] - rjb[r, pl.ds(0, 16)]
            sq = va * va * cm + eps
            s = jnp.sum(sq)
            plsc.store_scatter(dbuf, [_bc16(off + r)], _bc16(s), mask=l0)
            return 0
        lax.fori_loop(0, _D_B, row, 0)
        return 0

    lax.fori_loop(0, bpw // _D_B, batch, 0)
    pltpu.sync_copy(dbuf, d_h.at[pl.ds(base, bpw)])


def _sc_edge_d(r128, idx_i, idx_j):
    k = pl.kernel(
        _sc_edge_d_body,
        out_type=jax.ShapeDtypeStruct((N_EDGES_PAD,), f32),
        mesh=_sc_mesh,
        compiler_params=pltpu.CompilerParams(needs_layout_passes=False),
        scratch_types=[
            pltpu.VMEM((_D_B,), i32),
            pltpu.VMEM((_D_B,), i32),
            pltpu.VMEM((_D_B, EMB), f32),
            pltpu.VMEM((_D_B, EMB), f32),
            pltpu.VMEM((N_EDGES_PAD // NW,), f32),
            pltpu.SemaphoreType.DMA,
            pltpu.SemaphoreType.DMA,
        ],
    )
    return k(r128, idx_i, idx_j)


_CH = 10000
_NCHUNK = N_EDGES // _CH
_TSL = N_TRI_PAD // NS
_SB = 4096
_DR = 64
_CAP = 4224
_PKM = (1 << 14) - 1


def _sc_triplet_body(y_h, ma_h, kj_h, ji_h, lx_h, out_h,
                     ji_s, kj_s, pk_b, cp_b, cnb,
                     cdA, ckA, cpA, cdB, ckB, cpB,
                     ybA, mbA, ybB, mbB, acc, syA, smA, syB, smB):
    c = lax.axis_index("c")
    s = lax.axis_index("s")

    def fire(off, cdx, ckx, cpx, ybx, mbx, sy, sm):
        for v in range(_DR // 16):
            pk = pk_b[pl.ds(off + v * 16, 16)]
            cdx[pl.ds(v * 16, 16)] = pk & jnp.full((16,), _PKM, i32)
            ckx[pl.ds(v * 16, 16)] = lax.shift_right_logical(
                pk, jnp.full((16,), 14, i32))
            cpx[pl.ds(v * 16, 16)] = cp_b[pl.ds(off + v * 16, 16)]
        pltpu.async_copy(y_h.at[ckx], ybx, sy)
        pltpu.async_copy(ma_h.at[cpx], mbx, sm)

    def finish(cdx, ybx, mbx, sy, sm):
        pltpu.make_async_copy(y_h.at[pl.ds(0, _DR)], ybx, sy).wait()
        pltpu.make_async_copy(ma_h.at[pl.ds(0, _DR)], mbx, sm).wait()

        def mulrow(r, _):
            for cc in range(EMB // 16):
                ybx[r, pl.ds(cc * 16, 16)] = (
                    ybx[r, pl.ds(cc * 16, 16)] * mbx[r, pl.ds(cc * 16, 16)])
            return 0
        lax.fori_loop(0, _DR, mulrow, 0)
        pltpu.sync_copy(ybx, acc.at[cdx], add=True)

    for kc in range(_NCHUNK // NC):
        chunk = kc * NC + c
        lo = chunk * _CH

        @pl.when(s < 10)
        def _():
            pltpu.sync_copy(lx_h.at[pl.ds(lo + s * 1000, 1000)],
                            acc.at[pl.ds(s * 1000, 1000)])
        plsc.subcore_barrier()

        def scan_batch(b, cnt):
            tbase = s * _TSL + b * _SB
            pltpu.sync_copy(ji_h.at[pl.ds(tbase, _SB)], ji_s)
            pltpu.sync_copy(kj_h.at[pl.ds(tbase, _SB)], kj_s)

            def scan_quad(q, cnt2):
                lanes = lax.broadcasted_iota(i32, (16,), 0)
                parts = []
                for u in range(4):
                    voff = q * 64 + u * 16
                    jiv = ji_s[pl.ds(voff, 16)]
                    kjv = kj_s[pl.ds(voff, 16)]
                    lv = jiv - _bc16(lo)
                    m = ((lv >= jnp.zeros((16,), i32))
                         & (lv < jnp.full((16,), _CH, i32)))
                    mi = m.astype(i32)
                    csum = plsc.cumsum(mi)
                    pk = lv | lax.shift_left(kjv, jnp.full((16,), 14, i32))
                    posv = _bc16(tbase + q * 64 + u * 16) + lanes
                    parts.append((m, csum, pk, posv))
                for m, csum, pk, posv in parts:
                    tgt = _bc16(cnt2) + csum - jnp.ones((16,), i32)
                    plsc.store_scatter(pk_b, [tgt], pk, mask=m)
                    plsc.store_scatter(cp_b, [tgt], posv, mask=m)
                    cnt2 = cnt2 + csum[15]
                return cnt2
            cnt = lax.fori_loop(0, _SB // 64, scan_quad, cnt)

            nfull = cnt // _DR

            @pl.when(nfull > 0)
            def _():
                fire(0, cdA, ckA, cpA, ybA, mbA, syA, smA)

            def duo(g, _):
                f1 = g * 2 + 1

                @pl.when(f1 < nfull)
                def _():
                    fire(f1 * _DR, cdB, ckB, cpB, ybB, mbB, syB, smB)
                finish(cdA, ybA, mbA, syA, smA)

                @pl.when(f1 < nfull)
                def _():
                    @pl.when(f1 + 1 < nfull)
                    def _():
                        fire((f1 + 1) * _DR, cdA, ckA, cpA, ybA, mbA,
                             syA, smA)
                    finish(cdB, ybB, mbB, syB, smB)
                return 0
            lax.fori_loop(0, (nfull + 1) // 2, duo, 0)

            rem = cnt - nfull * _DR
            off0 = nfull * _DR
            vals = []
            for v in range(_DR // 16):
                vals.append((pk_b[pl.ds(off0 + v * 16, 16)],
                             cp_b[pl.ds(off0 + v * 16, 16)]))
            for v, (a, bb2) in enumerate(vals):
                pk_b[pl.ds(v * 16, 16)] = a
                cp_b[pl.ds(v * 16, 16)] = bb2
            return rem

        cnt = lax.fori_loop(0, _TSL // _SB, scan_batch, jnp.int32(0))

        for v in range(_DR // 16):
            lanes = lax.broadcasted_iota(i32, (16,), 0)
            l = _bc16(jnp.int32(v * 16)) + lanes
            good = l < _bc16(cnt)
            pk_b[pl.ds(v * 16, 16)] = jnp.where(
                good, pk_b[pl.ds(v * 16, 16)], jnp.full((16,), _CH, i32))
            cp_b[pl.ds(v * 16, 16)] = jnp.where(
                good, cp_b[pl.ds(v * 16, 16)], jnp.zeros((16,), i32))
        fire(0, cdA, ckA, cpA, ybA, mbA, syA, smA)
        finish(cdA, ybA, mbA, syA, smA)

        plsc.subcore_barrier()

        @pl.when(s < 10)
        def _():
            pltpu.sync_copy(acc.at[pl.ds(s * 1000, 1000)],
                            out_h.at[pl.ds(lo + s * 1000, 1000)])
        plsc.subcore_barrier()


def _sc_triplet(y, ma, kj, ji, lx):
    k = pl.kernel(
        _sc_triplet_body,
        out_type=jax.ShapeDtypeStruct((N_EDGES_PAD, EMB), f32),
        mesh=_sc_mesh,
        compiler_params=pltpu.CompilerParams(needs_layout_passes=False),
        scratch_types=[
            pltpu.VMEM((_SB,), i32),
            pltpu.VMEM((_SB,), i32),
            pltpu.VMEM((_CAP,), i32),
            pltpu.VMEM((_CAP,), i32),
            pltpu.VMEM((16,), i32),
            pltpu.VMEM((_DR,), i32),
            pltpu.VMEM((_DR,), i32),
            pltpu.VMEM((_DR,), i32),
            pltpu.VMEM((_DR,), i32),
            pltpu.VMEM((_DR,), i32),
            pltpu.VMEM((_DR,), i32),
            pltpu.VMEM((_DR, EMB), f32),
            pltpu.VMEM((_DR, EMB), f32),
            pltpu.VMEM((_DR, EMB), f32),
            pltpu.VMEM((_DR, EMB), f32),
            pltpu.VMEM_SHARED((_CH + 8, EMB), f32),
            pltpu.SemaphoreType.DMA,
            pltpu.SemaphoreType.DMA,
            pltpu.SemaphoreType.DMA,
            pltpu.SemaphoreType.DMA,
        ],
    )
    return k(y, ma, kj, ji, lx)


_E2A_B = 128
_E2A_ACC = 10240


def _sc_e2a_body(t_h, ii_h, out_h, ivA, tbA, ivB, tbB, acc, sA, sB):
    c = lax.axis_index("c")
    s = lax.axis_index("s")
    span = _E2A_ACC // NS

    def zrow(r, _):
        for cc in range(EMB // 16):
            tbA[r, pl.ds(cc * 16, 16)] = jnp.zeros((16,), f32)
        return 0
    lax.fori_loop(0, _E2A_B, zrow, 0)
    for z in range(span // _E2A_B):
        pltpu.sync_copy(tbA, acc.at[pl.ds(s * span + z * _E2A_B, _E2A_B)])
    plsc.subcore_barrier()

    bpw = N_EDGES_PAD // NW
    base = (c * NS + s) * bpw
    nb = bpw // _E2A_B

    def fire(k, ivx, tbx, sx):
        off = base + k * _E2A_B
        pltpu.sync_copy(ii_h.at[pl.ds(off, _E2A_B)], ivx)
        pltpu.async_copy(t_h.at[pl.ds(off, _E2A_B)], tbx, sx)

    def finish(ivx, tbx, sx):
        pltpu.make_async_copy(t_h.at[pl.ds(0, _E2A_B)], tbx, sx).wait()
        pltpu.sync_copy(tbx, acc.at[ivx], add=True)

    fire(0, ivA, tbA, sA)

    def duo(g, _):
        k0 = g * 2
        fire(k0 + 1, ivB, tbB, sB)
        finish(ivA, tbA, sA)

        @pl.when(k0 + 2 < nb)
        def _():
            fire(k0 + 2, ivA, tbA, sA)
        finish(ivB, tbB, sB)
        return 0
    lax.fori_loop(0, nb // 2, duo, 0)

    plsc.subcore_barrier()

    @pl.when(s < 10)
    def _():
        pltpu.sync_copy(acc.at[pl.ds(s * 1000, 1000)],
                        out_h.at[c, pl.ds(s * 1000, 1000)])


def _sc_e2a(t, idx_i):
    k = pl.kernel(
        _sc_e2a_body,
        out_type=jax.ShapeDtypeStruct((NC, N_ATOMS, EMB), f32),
        mesh=_sc_mesh,
        compiler_params=pltpu.CompilerParams(needs_layout_passes=False),
        scratch_types=[
            pltpu.VMEM((_E2A_B,), i32),
            pltpu.VMEM((_E2A_B, EMB), f32),
            pltpu.VMEM((_E2A_B,), i32),
            pltpu.VMEM((_E2A_B, EMB), f32),
            pltpu.VMEM_SHARED((_E2A_ACC, EMB), f32),
            pltpu.SemaphoreType.DMA,
            pltpu.SemaphoreType.DMA,
        ],
    )
    return k(t, idx_i)


_A2M_ROWS = 1280


def _sc_a2m_body(ro_h, rs_h, seg_h, out_h, idx_v, idx2_v, buf, zbuf, acc, sem):
    c = lax.axis_index("c")
    s = lax.axis_index("s")
    wid = s * NC + c
    span = _A2M_ROWS // NS

    def zrow(r, _):
        for cc in range(EMB // 16):
            zbuf[r, pl.ds(cc * 16, 16)] = jnp.zeros((16,), f32)
        return 0
    lax.fori_loop(0, span, zrow, 0)
    pltpu.sync_copy(zbuf, acc.at[pl.ds(s * span, span)])
    plsc.subcore_barrier()

    bpw = 512

    @pl.when(wid < 20)
    def _():
        base = wid * bpw
        pltpu.sync_copy(seg_h.at[pl.ds(base, bpw)], idx_v)
        for v in range(bpw // 16):
            idx2_v[pl.ds(v * 16, 16)] = (idx_v[pl.ds(v * 16, 16)]
                                         + jnp.full((16,), 640, i32))
        pltpu.sync_copy(ro_h.at[pl.ds(base, bpw)], buf)
        pltpu.sync_copy(buf, acc.at[idx_v], add=True)
        pltpu.sync_copy(rs_h.at[pl.ds(base, bpw)], buf)
        pltpu.sync_copy(buf, acc.at[idx2_v], add=True)

    plsc.subcore_barrier()
    pltpu.sync_copy(acc.at[pl.ds(s * span, span)],
                    out_h.at[c, pl.ds(s * span, span)])


def _sc_a2m(ro, rs, seg):
    k = pl.kernel(
        _sc_a2m_body,
        out_type=jax.ShapeDtypeStruct((NC, _A2M_ROWS, EMB), f32),
        mesh=_sc_mesh,
        compiler_params=pltpu.CompilerParams(needs_layout_passes=False),
        scratch_types=[
            pltpu.VMEM((512,), i32),
            pltpu.VMEM((512,), i32),
            pltpu.VMEM((512, EMB), f32),
            pltpu.VMEM((_A2M_ROWS // NS, EMB), f32),
            pltpu.VMEM_SHARED((_A2M_ROWS, EMB), f32),
            pltpu.SemaphoreType.DMA,
        ],
    )
    return k(ro, rs, seg)


_EB = 2048
_EGRID = N_EDGES_PAD // _EB
_DB = _EB // 128


def _rbf_tiles(d2):
    d = jnp.sqrt(d2)
    scale = np.sqrt(2.0 / CUTOFF).astype(np.float32)
    inv = 1.0 / (d + 1e-6)
    return [scale * jnp.sin((k + 1) * np.pi * d / CUTOFF) * inv
            for k in range(NUM_RADIAL)]


def _rbf_mix_rows(tiles, zero_row, w6, a):
    del zero_row
    stack = jnp.concatenate([t[a:a + 1] for t in tiles], axis=0)
    return jax.lax.dot_general(stack, w6, (((0,), (0,)), ((), ())),
                               preferred_element_type=f32)


def _tc_atom_pre_body(h_ref, wa_ref, wb_ref, o1_ref, o2_ref):
    h = h_ref[...]
    o1_ref[...] = jax.lax.dot_general(h, wa_ref[...], (((1,), (0,)), ((), ())),
                                      preferred_element_type=f32)
    o2_ref[...] = jax.lax.dot_general(h, wb_ref[...], (((1,), (0,)), ((), ())),
                                      preferred_element_type=f32)


def _tc_atom_pre(h, wa, wb):
    grid = 10
    rb = N_ATOMS // grid
    return pl.pallas_call(
        _tc_atom_pre_body,
        grid=(grid,),
        in_specs=[
            pl.BlockSpec((rb, EMB), lambda s: (s, 0)),
            pl.BlockSpec((EMB, EMB), lambda s: (0, 0)),
            pl.BlockSpec((EMB, EMB), lambda s: (0, 0)),
        ],
        out_specs=[
            pl.BlockSpec((rb, EMB), lambda s: (s, 0)),
            pl.BlockSpec((rb, EMB), lambda s: (s, 0)),
        ],
        out_shape=[
            jax.ShapeDtypeStruct((N_ATOMS, EMB), f32),
            jax.ShapeDtypeStruct((N_ATOMS, EMB), f32),
        ],
    )(h, wa, wb)


def _tc_edge_init_body(gi_ref, gj_ref, d_ref, wr_ref, b_ref,
                       ow_ref, iw_ref, x_ref, t_ref, y_ref):
    tiles = _rbf_tiles(d_ref[...])
    zero_row = jnp.zeros((1, 128), f32)
    for a in range(_DB):
        rows = pl.ds(a * 128, 128)
        rbfe = _rbf_mix_rows(tiles, zero_row, wr_ref[...], a)
        xv = _swish(gi_ref[rows, :] + gj_ref[rows, :] + rbfe + b_ref[...])
        x_ref[rows, :] = xv
        t_ref[rows, :] = xv * _rbf_mix_rows(tiles, zero_row, ow_ref[...], a)
        y_ref[rows, :] = xv * _rbf_mix_rows(tiles, zero_row, iw_ref[...], a)


def _tc_edge_init(gi, gj, d2d, wr, b, ow, iw):
    espec = pl.BlockSpec((_EB, EMB), lambda s: (s, 0))
    dspec = pl.BlockSpec((_DB, 128), lambda s: (s, 0))
    wspec = pl.BlockSpec((NUM_RADIAL, EMB), lambda s: (0, 0))
    return pl.pallas_call(
        _tc_edge_init_body,
        grid=(_EGRID,),
        in_specs=[espec, espec, dspec, wspec,
                  pl.BlockSpec((1, EMB), lambda s: (0, 0)), wspec, wspec],
        out_specs=[espec, espec, espec],
        out_shape=[jax.ShapeDtypeStruct((N_EDGES_PAD, EMB), f32)] * 3,
    )(gi, gj, d2d, wr, b, ow, iw)


_MA_R = 16


def _tc_mabf_body(c_ref, w_ref, o0_ref, o1_ref, o2_ref):
    c = c_ref[...]
    polys = [jnp.ones_like(c), c]
    for _ in range(NUM_ABF - 2):
        polys.append(2.0 * c * polys[-1] - polys[-2])
    zero = jnp.zeros((1, 128), f32)
    outs = (o0_ref, o1_ref, o2_ref)
    for a in range(_MA_R):
        stack = jnp.concatenate(
            [polys[k][a:a + 1] for k in range(NUM_ABF)] + [zero],
            axis=0)
        for i in range(NUM_BLOCKS):
            outs[i][pl.ds(a * 128, 128), :] = jax.lax.dot_general(
                stack, w_ref[i], (((0,), (0,)), ((), ())),
                preferred_element_type=f32)


def _tc_mabf(cos2d, wabf8):
    grid = (N_TRI_PAD // 128) // _MA_R
    ospec = pl.BlockSpec((_MA_R * 128, EMB), lambda s: (s, 0))
    return pl.pallas_call(
        _tc_mabf_body,
        grid=(grid,),
        in_specs=[
            pl.BlockSpec((_MA_R, 128), lambda s: (s, 0)),
            pl.BlockSpec((NUM_BLOCKS, 8, EMB), lambda s: (0, 0, 0)),
        ],
        out_specs=[ospec, ospec, ospec],
        out_shape=[jax.ShapeDtypeStruct((N_TRI_PAD, EMB), f32)] * 3,
    )(cos2d, wabf8)


def _tc_mlp_body(ax_ref, lx_ref, d_ref, w1_ref, b1_ref, w2_ref,
                 b2_ref, ow_ref, iw_ref, xn_ref, t_ref, y_ref):
    u = _swish(jax.lax.dot_general(ax_ref[...], w1_ref[...],
                                   (((1,), (0,)), ((), ())),
                                   preferred_element_type=f32) + b1_ref[...])
    xn = lx_ref[...] + _swish(
        jax.lax.dot_general(u, w2_ref[...], (((1,), (0,)), ((), ())),
                            preferred_element_type=f32) + b2_ref[...])
    xn_ref[...] = xn
    tiles = _rbf_tiles(d_ref[...])
    zero_row = jnp.zeros((1, 128), f32)
    for a in range(_DB):
        rows = pl.ds(a * 128, 128)
        t_ref[rows, :] = xn[a * 128:(a + 1) * 128, :] * _rbf_mix_rows(
            tiles, zero_row, ow_ref[...], a)
        if y_ref is not None:
            y_ref[rows, :] = xn[a * 128:(a + 1) * 128, :] * _rbf_mix_rows(
                tiles, zero_row, iw_ref[...], a)


def _tc_mlp(aggx, lastx, d2d, w1, b1, w2, b2, ow, iw, want_y):
    espec = pl.BlockSpec((_EB, EMB), lambda s: (s, 0))
    dspec = pl.BlockSpec((_DB, 128), lambda s: (s, 0))
    mspec = pl.BlockSpec((EMB, EMB), lambda s: (0, 0))
    bspec = pl.BlockSpec((1, EMB), lambda s: (0, 0))
    wspec = pl.BlockSpec((NUM_RADIAL, EMB), lambda s: (0, 0))
    if want_y:
        body = _tc_mlp_body
        out_specs = [espec, espec, espec]
        out_shape = [jax.ShapeDtypeStruct((N_EDGES_PAD, EMB), f32)] * 3
    else:
        def body(ax, lx, d_, w1_, b1_, w2_, b2_, ow_, iw_, xn_, t_):
            _tc_mlp_body(ax, lx, d_, w1_, b1_, w2_, b2_, ow_, iw_,
                         xn_, t_, None)
        out_specs = [espec, espec]
        out_shape = [jax.ShapeDtypeStruct((N_EDGES_PAD, EMB), f32)] * 2
    return pl.pallas_call(
        body,
        grid=(_EGRID,),
        in_specs=[espec, espec, dspec, mspec, bspec, mspec, bspec,
                  wspec, wspec],
        out_specs=out_specs,
        out_shape=out_shape,
    )(aggx, lastx, d2d, w1, b1, w2, b2, ow, iw)


def _tc_atom_final_body(a0_ref, a1_ref, a2_ref, a3_ref, h_ref,
                        ow1_ref, ow2_ref, sbw_ref, sbb_ref,
                        ro_ref, rs_ref):
    arefs = (a0_ref, a1_ref, a2_ref, a3_ref)
    ro = None
    for i in range(NUM_BLOCKS + 1):
        a = arefs[i][0] + arefs[i][1]
        u = _swish(jax.lax.dot_general(a, ow1_ref[i], (((1,), (0,)), ((), ())),
                                       preferred_element_type=f32))
        v = jax.lax.dot_general(u, ow2_ref[i], (((1,), (0,)), ((), ())),
                                preferred_element_type=f32)
        ro = v if ro is None else ro + v
    ro_ref[...] = ro
    rs = _swish(jax.lax.dot_general(h_ref[...], sbw_ref[0],
                                    (((1,), (0,)), ((), ())),
                                    preferred_element_type=f32) + sbb_ref[0])
    for i in range(NUM_BLOCKS):
        rs = rs + _swish(
            jax.lax.dot_general(rs, sbw_ref[i + 1], (((1,), (0,)), ((), ())),
                                preferred_element_type=f32) + sbb_ref[i + 1])
    rs_ref[...] = rs


def _tc_atom_final(a_list, h, ow1, ow2, sbw, sbb):
    grid = 10
    rb = N_ATOMS // grid
    aspec = pl.BlockSpec((NC, rb, EMB), lambda s: (0, s, 0))
    nb1 = NUM_BLOCKS + 1
    return pl.pallas_call(
        _tc_atom_final_body,
        grid=(grid,),
        in_specs=[aspec, aspec, aspec, aspec,
                  pl.BlockSpec((rb, EMB), lambda s: (s, 0)),
                  pl.BlockSpec((nb1, EMB, EMB), lambda s: (0, 0, 0)),
                  pl.BlockSpec((nb1, EMB, EMB), lambda s: (0, 0, 0)),
                  pl.BlockSpec((nb1, EMB, EMB), lambda s: (0, 0, 0)),
                  pl.BlockSpec((nb1, 1, EMB), lambda s: (0, 0, 0))],
        out_specs=[pl.BlockSpec((rb, EMB), lambda s: (s, 0)),
                   pl.BlockSpec((rb, EMB), lambda s: (s, 0))],
        out_shape=[jax.ShapeDtypeStruct((N_ATOMS, EMB), f32)] * 2,
    )(*a_list, h, ow1, ow2, sbw, sbb)


def _tc_mol_body(m_ref, cm_ref, cs_ref, o_ref):
    res = m_ref[0, 0:N_MOL, :] + m_ref[1, 0:N_MOL, :]
    sing = m_ref[0, 640:640 + N_MOL, :] + m_ref[1, 640:640 + N_MOL, :]
    o_ref[...] = cm_ref[0, 0] * res + cs_ref[0, 0] * sing


def _tc_mol(mo, cm, cs):
    return pl.pallas_call(
        _tc_mol_body,
        in_specs=[pl.BlockSpec((NC, _A2M_ROWS, EMB), lambda: (0, 0, 0)),
                  pl.BlockSpec((1, 1), lambda: (0, 0)),
                  pl.BlockSpec((1, 1), lambda: (0, 0))],
        out_specs=pl.BlockSpec((N_MOL, EMB), lambda: (0, 0)),
        out_shape=jax.ShapeDtypeStruct((N_MOL, EMB), f32),
        grid=(),
    )(mo, cm, cs)


def kernel(Z, R, batch_seg, idx_i, idx_j, idx_kj, idx_ji, cosine_ijk, params):
    p = params
    ep = N_EDGES_PAD - N_EDGES
    Zp = jnp.pad(Z.astype(i32), (0, N_ATOMS_PAD - N_ATOMS))
    r128 = jnp.pad(R.astype(f32), ((0, 0), (0, EMB - 3)))
    idx_i_g = jnp.pad(idx_i.astype(i32), (0, ep))
    idx_j_g = jnp.pad(idx_j.astype(i32), (0, ep))
    idx_i_s = jnp.pad(idx_i.astype(i32), (0, ep),
                      constant_values=N_ATOMS)
    tp = N_TRI_PAD - N_TRIPLETS
    idx_kj = jnp.pad(idx_kj.astype(i32), (0, tp))
    idx_ji = jnp.pad(idx_ji.astype(i32), (0, tp),
                     constant_values=1 << 29)
    seg_p = jnp.pad(batch_seg.astype(i32), (0, N_ATOMS_PAD - N_ATOMS),
                    constant_values=N_MOL)

    emb_b = p["emb_b"].reshape(1, EMB)
    wabf8 = jnp.pad(p["int_W_abf"], ((0, 0), (0, 8 - NUM_ABF), (0, 0)))
    cos2d = jnp.pad(cosine_ijk.astype(f32),
                    (0, N_TRI_PAD - N_TRIPLETS)).reshape(N_TRI_PAD // 128, 128)

    hp = _sc_h_gather(p["emb_table"], Zp)
    h = hp[:N_ATOMS]
    hw1, hw2 = _tc_atom_pre(h, p["emb_W"][:EMB], p["emb_W"][EMB:2 * EMB])
    gi, gj = _sc_edge_gather(hw1, hw2, idx_i_g, idx_j_g)
    d2d = _sc_edge_d(r128, idx_i_g, idx_j_g).reshape(N_EDGES_PAD // 128, 128)
    wr_x = p["emb_W_rbf"] @ p["emb_W"][2 * EMB:]
    x, t0, y = _tc_edge_init(gi, gj, d2d, wr_x, emb_b,
                             p["out_W_rbf"][0], p["int_W_rbf"][0])
    ma = _tc_mabf(cos2d, wabf8)

    a_list = [_sc_e2a(t0, idx_i_s)]
    last = x
    for i in range(NUM_BLOCKS):
        aggx = _sc_triplet(y, ma[i], idx_kj, idx_ji, last)
        want_y = i < NUM_BLOCKS - 1
        ow = p["out_W_rbf"][i + 1]
        iw = p["int_W_rbf"][i + 1] if want_y else p["int_W_rbf"][i]
        outs = _tc_mlp(aggx, last, d2d, p["int_W1"][i],
                       p["int_b1"][i].reshape(1, EMB), p["int_W2"][i],
                       p["int_b2"][i].reshape(1, EMB), ow, iw, want_y)
        if want_y:
            xn, t_next, y = outs
        else:
            xn, t_next = outs
        a_list.append(_sc_e2a(t_next, idx_i_s))
        last = xn

    ro, rs = _tc_atom_final(a_list, h, p["out_W1"], p["out_W2"], p["sb_W"],
                            p["sb_b"].reshape(NUM_BLOCKS + 1, 1, EMB))
    ro_p = jnp.pad(ro, ((0, N_ATOMS_PAD - N_ATOMS), (0, 0)))
    rs_p = jnp.pad(rs, ((0, N_ATOMS_PAD - N_ATOMS), (0, 0)))
    mo = _sc_a2m(ro_p, rs_p, seg_p)
    return _tc_mol(mo, p["coef_mp"].reshape(1, 1), p["coef_sg"].reshape(1, 1))


# speedup vs baseline: 1.8738x; 1.0174x over previous
import functools

import jax
import jax.numpy as jnp
import numpy as np
from jax import lax
from jax.experimental import pallas as pl
from jax.experimental.pallas import tpu as pltpu
from jax.experimental.pallas import tpu_sc as plsc

EMB = 128
NUM_BLOCKS = 3
NUM_RADIAL = 6
CUTOFF = 5.0
NUM_ABF = 7
N_ATOMS = 10000
N_EDGES = 160000
N_TRIPLETS = 320000
N_MOL = 512

N_ATOMS_PAD = 10240
N_EDGES_PAD = 163840
N_TRI_PAD = 327680

NC = 2
NS = 16
NW = NC * NS

f32 = jnp.float32
i32 = jnp.int32

_sc_mesh = plsc.VectorSubcoreMesh(core_axis_name="c", subcore_axis_name="s")


def _bc16(x):
    return jax.lax.broadcast_in_dim(x, (16,), ())


def _swish(x):
    return x * jax.nn.sigmoid(x)


def _sc_h_gather_body(tab_h, z_h, out_h, idx_v, rows_v, sem):
    wid = lax.axis_index("s") * NC + lax.axis_index("c")
    bpw = 512

    @pl.when(wid < 20)
    def _():
        base = wid * bpw
        pltpu.sync_copy(z_h.at[pl.ds(base, bpw)], idx_v)
        pltpu.async_copy(tab_h.at[idx_v], rows_v, sem).wait()
        pltpu.sync_copy(rows_v, out_h.at[pl.ds(base, bpw)])


def _sc_h_gather(tab, zp):
    k = pl.kernel(
        _sc_h_gather_body,
        out_type=jax.ShapeDtypeStruct((N_ATOMS_PAD, EMB), f32),
        mesh=_sc_mesh,
        compiler_params=pltpu.CompilerParams(needs_layout_passes=False),
        scratch_types=[
            pltpu.VMEM((512,), i32),
            pltpu.VMEM((512, EMB), f32),
            pltpu.SemaphoreType.DMA,
        ],
    )
    return k(tab, zp)


_EG_B = 128


def _sc_edge_gather_body(hw1_h, hw2_h, ii_h, jj_h, gi_h, gj_h,
                         ivA, jvA, baA, bbA, ivB, jvB, baB, bbB,
                         s0A, s1A, s0B, s1B):
    wid = lax.axis_index("s") * NC + lax.axis_index("c")
    bpw = N_EDGES_PAD // NW
    base = wid * bpw
    nb = bpw // _EG_B

    def fire(k, ivx, jvx, bax, bbx, s0x, s1x):
        off = base + k * _EG_B
        pltpu.sync_copy(ii_h.at[pl.ds(off, _EG_B)], ivx)
        pltpu.sync_copy(jj_h.at[pl.ds(off, _EG_B)], jvx)
        pltpu.async_copy(hw1_h.at[ivx], bax, s0x)
        pltpu.async_copy(hw2_h.at[jvx], bbx, s1x)

    def finish(k, bax, bbx, s0x, s1x):
        off = base + k * _EG_B
        pltpu.make_async_copy(hw1_h.at[pl.ds(0, _EG_B)], bax, s0x).wait()
        pltpu.make_async_copy(hw2_h.at[pl.ds(0, _EG_B)], bbx, s1x).wait()
        pltpu.sync_copy(bax, gi_h.at[pl.ds(off, _EG_B)])
        pltpu.sync_copy(bbx, gj_h.at[pl.ds(off, _EG_B)])

    fire(0, ivA, jvA, baA, bbA, s0A, s1A)

    def duo(g, _):
        k0 = g * 2
        fire(k0 + 1, ivB, jvB, baB, bbB, s0B, s1B)
        finish(k0, baA, bbA, s0A, s1A)

        @pl.when(k0 + 2 < nb)
        def _():
            fire(k0 + 2, ivA, jvA, baA, bbA, s0A, s1A)
        finish(k0 + 1, baB, bbB, s0B, s1B)
        return 0
    lax.fori_loop(0, nb // 2, duo, 0)


def _sc_edge_gather(hw1, hw2, idx_i, idx_j):
    k = pl.kernel(
        _sc_edge_gather_body,
        out_type=(
            jax.ShapeDtypeStruct((N_EDGES_PAD, EMB), f32),
            jax.ShapeDtypeStruct((N_EDGES_PAD, EMB), f32),
        ),
        mesh=_sc_mesh,
        compiler_params=pltpu.CompilerParams(needs_layout_passes=False),
        scratch_types=[
            pltpu.VMEM((_EG_B,), i32),
            pltpu.VMEM((_EG_B,), i32),
            pltpu.VMEM((_EG_B, EMB), f32),
            pltpu.VMEM((_EG_B, EMB), f32),
            pltpu.VMEM((_EG_B,), i32),
            pltpu.VMEM((_EG_B,), i32),
            pltpu.VMEM((_EG_B, EMB), f32),
            pltpu.VMEM((_EG_B, EMB), f32),
            pltpu.SemaphoreType.DMA,
            pltpu.SemaphoreType.DMA,
            pltpu.SemaphoreType.DMA,
            pltpu.SemaphoreType.DMA,
        ],
    )
    return k(hw1, hw2, idx_i, idx_j)


_D_B = 256


def _sc_edge_d_body(r128_h, ii_h, jj_h, d_h, iv, jv, rib, rjb, dbuf, s0, s1):
    wid = lax.axis_index("s") * NC + lax.axis_index("c")
    bpw = N_EDGES_PAD // NW
    base = wid * bpw

    def batch(k, _):
        off = k * _D_B
        pltpu.sync_copy(ii_h.at[pl.ds(base + off, _D_B)], iv)
        pltpu.sync_copy(jj_h.at[pl.ds(base + off, _D_B)], jv)
        c0 = pltpu.async_copy(r128_h.at[iv], rib, s0)
        c1 = pltpu.async_copy(r128_h.at[jv], rjb, s1)
        c0.wait(); c1.wait()

        def row(r, _2):
            lns = lax.broadcasted_iota(i32, (16,), 0)
            cm = jnp.where(lns < jnp.full((16,), 3, i32),
                           jnp.ones((16,), f32), jnp.zeros((16,), f32))
            l0 = lns == jnp.zeros((16,), i32)
            eps = jnp.where(l0, jnp.full((16,), 1e-12, f32),
                            jnp.zeros((16,), f32))
            va = rib[r, pl.ds(0, 16)] - rjb[r, pl.ds(0, 16)]
            sq = va * va * cm + eps
            s = jnp.sum(sq)
            plsc.store_scatter(dbuf, [_bc16(off + r)], _bc16(s), mask=l0)
            return 0
        lax.fori_loop(0, _D_B, row, 0)
        return 0

    lax.fori_loop(0, bpw // _D_B, batch, 0)
    pltpu.sync_copy(dbuf, d_h.at[pl.ds(base, bpw)])


def _sc_edge_d(r128, idx_i, idx_j):
    k = pl.kernel(
        _sc_edge_d_body,
        out_type=jax.ShapeDtypeStruct((N_EDGES_PAD,), f32),
        mesh=_sc_mesh,
        compiler_params=pltpu.CompilerParams(needs_layout_passes=False),
        scratch_types=[
            pltpu.VMEM((_D_B,), i32),
            pltpu.VMEM((_D_B,), i32),
            pltpu.VMEM((_D_B, EMB), f32),
            pltpu.VMEM((_D_B, EMB), f32),
            pltpu.VMEM((N_EDGES_PAD // NW,), f32),
            pltpu.SemaphoreType.DMA,
            pltpu.SemaphoreType.DMA,
        ],
    )
    return k(r128, idx_i, idx_j)


_CH = 10000
_NCHUNK = N_EDGES // _CH
_TSL = N_TRI_PAD // NS
_SB = 4096
_DR = 64
_CAP = 4224
_PKM = (1 << 14) - 1


def _sc_triplet_body(y_h, ma_h, kj_h, ji_h, lx_h, out_h,
                     ji_s, kj_s, pk_b, cp_b, cnb,
                     cdA, ckA, cpA, cdB, ckB, cpB,
                     ybA, mbA, ybB, mbB, acc, syA, smA, syB, smB):
    c = lax.axis_index("c")
    s = lax.axis_index("s")

    def fire(off, cdx, ckx, cpx, ybx, mbx, sy, sm):
        for v in range(_DR // 16):
            pk = pk_b[pl.ds(off + v * 16, 16)]
            cdx[pl.ds(v * 16, 16)] = pk & jnp.full((16,), _PKM, i32)
            ckx[pl.ds(v * 16, 16)] = lax.shift_right_logical(
                pk, jnp.full((16,), 14, i32))
            cpx[pl.ds(v * 16, 16)] = cp_b[pl.ds(off + v * 16, 16)]
        pltpu.async_copy(y_h.at[ckx], ybx, sy)
        pltpu.async_copy(ma_h.at[cpx], mbx, sm)

    def finish(cdx, ybx, mbx, sy, sm):
        pltpu.make_async_copy(y_h.at[pl.ds(0, _DR)], ybx, sy).wait()
        pltpu.make_async_copy(ma_h.at[pl.ds(0, _DR)], mbx, sm).wait()

        def mulrow(r, _):
            for cc in range(EMB // 16):
                ybx[r, pl.ds(cc * 16, 16)] = (
                    ybx[r, pl.ds(cc * 16, 16)] * mbx[r, pl.ds(cc * 16, 16)])
            return 0
        lax.fori_loop(0, _DR, mulrow, 0)
        pltpu.sync_copy(ybx, acc.at[cdx], add=True)

    for kc in range(_NCHUNK // NC):
        chunk = kc * NC + c
        lo = chunk * _CH

        @pl.when(s < 10)
        def _():
            pltpu.sync_copy(lx_h.at[pl.ds(lo + s * 1000, 1000)],
                            acc.at[pl.ds(s * 1000, 1000)])
        plsc.subcore_barrier()

        def scan_batch(b, cnt):
            tbase = s * _TSL + b * _SB
            pltpu.sync_copy(ji_h.at[pl.ds(tbase, _SB)], ji_s)
            pltpu.sync_copy(kj_h.at[pl.ds(tbase, _SB)], kj_s)

            def scan_quad(q, cnt2):
                lanes = lax.broadcasted_iota(i32, (16,), 0)
                parts = []
                for u in range(8):
                    voff = q * 128 + u * 16
                    jiv = ji_s[pl.ds(voff, 16)]
                    kjv = kj_s[pl.ds(voff, 16)]
                    lv = jiv - _bc16(lo)
                    m = ((lv >= jnp.zeros((16,), i32))
                         & (lv < jnp.full((16,), _CH, i32)))
                    mi = m.astype(i32)
                    csum = plsc.cumsum(mi)
                    pk = lv | lax.shift_left(kjv, jnp.full((16,), 14, i32))
                    posv = _bc16(tbase + q * 128 + u * 16) + lanes
                    parts.append((m, csum, pk, posv))
                for m, csum, pk, posv in parts:
                    tgt = _bc16(cnt2) + csum - jnp.ones((16,), i32)
                    plsc.store_scatter(pk_b, [tgt], pk, mask=m)
                    plsc.store_scatter(cp_b, [tgt], posv, mask=m)
                    cnt2 = cnt2 + csum[15]
                return cnt2
            cnt = lax.fori_loop(0, _SB // 128, scan_quad, cnt)

            nfull = cnt // _DR

            @pl.when(nfull > 0)
            def _():
                fire(0, cdA, ckA, cpA, ybA, mbA, syA, smA)

            def duo(g, _):
                f1 = g * 2 + 1

                @pl.when(f1 < nfull)
                def _():
                    fire(f1 * _DR, cdB, ckB, cpB, ybB, mbB, syB, smB)
                finish(cdA, ybA, mbA, syA, smA)

                @pl.when(f1 < nfull)
                def _():
                    @pl.when(f1 + 1 < nfull)
                    def _():
                        fire((f1 + 1) * _DR, cdA, ckA, cpA, ybA, mbA,
                             syA, smA)
                    finish(cdB, ybB, mbB, syB, smB)
                return 0
            lax.fori_loop(0, (nfull + 1) // 2, duo, 0)

            rem = cnt - nfull * _DR
            off0 = nfull * _DR
            vals = []
            for v in range(_DR // 16):
                vals.append((pk_b[pl.ds(off0 + v * 16, 16)],
                             cp_b[pl.ds(off0 + v * 16, 16)]))
            for v, (a, bb2) in enumerate(vals):
                pk_b[pl.ds(v * 16, 16)] = a
                cp_b[pl.ds(v * 16, 16)] = bb2
            return rem

        cnt = lax.fori_loop(0, _TSL // _SB, scan_batch, jnp.int32(0))

        for v in range(_DR // 16):
            lanes = lax.broadcasted_iota(i32, (16,), 0)
            l = _bc16(jnp.int32(v * 16)) + lanes
            good = l < _bc16(cnt)
            pk_b[pl.ds(v * 16, 16)] = jnp.where(
                good, pk_b[pl.ds(v * 16, 16)], jnp.full((16,), _CH, i32))
            cp_b[pl.ds(v * 16, 16)] = jnp.where(
                good, cp_b[pl.ds(v * 16, 16)], jnp.zeros((16,), i32))
        fire(0, cdA, ckA, cpA, ybA, mbA, syA, smA)
        finish(cdA, ybA, mbA, syA, smA)

        plsc.subcore_barrier()

        @pl.when(s < 10)
        def _():
            pltpu.sync_copy(acc.at[pl.ds(s * 1000, 1000)],
                            out_h.at[pl.ds(lo + s * 1000, 1000)])
        plsc.subcore_barrier()


def _sc_triplet(y, ma, kj, ji, lx):
    k = pl.kernel(
        _sc_triplet_body,
        out_type=jax.ShapeDtypeStruct((N_EDGES_PAD, EMB), f32),
        mesh=_sc_mesh,
        compiler_params=pltpu.CompilerParams(needs_layout_passes=False),
        scratch_types=[
            pltpu.VMEM((_SB,), i32),
            pltpu.VMEM((_SB,), i32),
            pltpu.VMEM((_CAP,), i32),
            pltpu.VMEM((_CAP,), i32),
            pltpu.VMEM((16,), i32),
            pltpu.VMEM((_DR,), i32),
            pltpu.VMEM((_DR,), i32),
            pltpu.VMEM((_DR,), i32),
            pltpu.VMEM((_DR,), i32),
            pltpu.VMEM((_DR,), i32),
            pltpu.VMEM((_DR,), i32),
            pltpu.VMEM((_DR, EMB), f32),
            pltpu.VMEM((_DR, EMB), f32),
            pltpu.VMEM((_DR, EMB), f32),
            pltpu.VMEM((_DR, EMB), f32),
            pltpu.VMEM_SHARED((_CH + 8, EMB), f32),
            pltpu.SemaphoreType.DMA,
            pltpu.SemaphoreType.DMA,
            pltpu.SemaphoreType.DMA,
            pltpu.SemaphoreType.DMA,
        ],
    )
    return k(y, ma, kj, ji, lx)


_E2A_B = 128
_E2A_ACC = 10240


def _sc_e2a_body(t_h, ii_h, out_h, ivA, tbA, ivB, tbB, acc, sA, sB):
    c = lax.axis_index("c")
    s = lax.axis_index("s")
    span = _E2A_ACC // NS

    def zrow(r, _):
        for cc in range(EMB // 16):
            tbA[r, pl.ds(cc * 16, 16)] = jnp.zeros((16,), f32)
        return 0
    lax.fori_loop(0, _E2A_B, zrow, 0)
    for z in range(span // _E2A_B):
        pltpu.sync_copy(tbA, acc.at[pl.ds(s * span + z * _E2A_B, _E2A_B)])
    plsc.subcore_barrier()

    bpw = N_EDGES_PAD // NW
    base = (c * NS + s) * bpw
    nb = bpw // _E2A_B

    def fire(k, ivx, tbx, sx):
        off = base + k * _E2A_B
        pltpu.sync_copy(ii_h.at[pl.ds(off, _E2A_B)], ivx)
        pltpu.async_copy(t_h.at[pl.ds(off, _E2A_B)], tbx, sx)

    def finish(ivx, tbx, sx):
        pltpu.make_async_copy(t_h.at[pl.ds(0, _E2A_B)], tbx, sx).wait()
        pltpu.sync_copy(tbx, acc.at[ivx], add=True)

    fire(0, ivA, tbA, sA)

    def duo(g, _):
        k0 = g * 2
        fire(k0 + 1, ivB, tbB, sB)
        finish(ivA, tbA, sA)

        @pl.when(k0 + 2 < nb)
        def _():
            fire(k0 + 2, ivA, tbA, sA)
        finish(ivB, tbB, sB)
        return 0
    lax.fori_loop(0, nb // 2, duo, 0)

    plsc.subcore_barrier()

    @pl.when(s < 10)
    def _():
        pltpu.sync_copy(acc.at[pl.ds(s * 1000, 1000)],
                        out_h.at[c, pl.ds(s * 1000, 1000)])


def _sc_e2a(t, idx_i):
    k = pl.kernel(
        _sc_e2a_body,
        out_type=jax.ShapeDtypeStruct((NC, N_ATOMS, EMB), f32),
        mesh=_sc_mesh,
        compiler_params=pltpu.CompilerParams(needs_layout_passes=False),
        scratch_types=[
            pltpu.VMEM((_E2A_B,), i32),
            pltpu.VMEM((_E2A_B, EMB), f32),
            pltpu.VMEM((_E2A_B,), i32),
            pltpu.VMEM((_E2A_B, EMB), f32),
            pltpu.VMEM_SHARED((_E2A_ACC, EMB), f32),
            pltpu.SemaphoreType.DMA,
            pltpu.SemaphoreType.DMA,
        ],
    )
    return k(t, idx_i)


_A2M_ROWS = 1280


def _sc_a2m_body(ro_h, rs_h, seg_h, out_h, idx_v, idx2_v, buf, zbuf, acc, sem):
    c = lax.axis_index("c")
    s = lax.axis_index("s")
    wid = s * NC + c
    span = _A2M_ROWS // NS

    def zrow(r, _):
        for cc in range(EMB // 16):
            zbuf[r, pl.ds(cc * 16, 16)] = jnp.zeros((16,), f32)
        return 0
    lax.fori_loop(0, span, zrow, 0)
    pltpu.sync_copy(zbuf, acc.at[pl.ds(s * span, span)])
    plsc.subcore_barrier()

    bpw = 512

    @pl.when(wid < 20)
    def _():
        base = wid * bpw
        pltpu.sync_copy(seg_h.at[pl.ds(base, bpw)], idx_v)
        for v in range(bpw // 16):
            idx2_v[pl.ds(v * 16, 16)] = (idx_v[pl.ds(v * 16, 16)]
                                         + jnp.full((16,), 640, i32))
        pltpu.sync_copy(ro_h.at[pl.ds(base, bpw)], buf)
        pltpu.sync_copy(buf, acc.at[idx_v], add=True)
        pltpu.sync_copy(rs_h.at[pl.ds(base, bpw)], buf)
        pltpu.sync_copy(buf, acc.at[idx2_v], add=True)

    plsc.subcore_barrier()
    pltpu.sync_copy(acc.at[pl.ds(s * span, span)],
                    out_h.at[c, pl.ds(s * span, span)])


def _sc_a2m(ro, rs, seg):
    k = pl.kernel(
        _sc_a2m_body,
        out_type=jax.ShapeDtypeStruct((NC, _A2M_ROWS, EMB), f32),
        mesh=_sc_mesh,
        compiler_params=pltpu.CompilerParams(needs_layout_passes=False),
        scratch_types=[
            pltpu.VMEM((512,), i32),
            pltpu.VMEM((512,), i32),
            pltpu.VMEM((512, EMB), f32),
            pltpu.VMEM((_A2M_ROWS // NS, EMB), f32),
            pltpu.VMEM_SHARED((_A2M_ROWS, EMB), f32),
            pltpu.SemaphoreType.DMA,
        ],
    )
    return k(ro, rs, seg)


_EB = 2048
_EGRID = N_EDGES_PAD // _EB
_DB = _EB // 128


def _rbf_tiles(d2):
    d = jnp.sqrt(d2)
    scale = np.sqrt(2.0 / CUTOFF).astype(np.float32)
    inv = 1.0 / (d + 1e-6)
    return [scale * jnp.sin((k + 1) * np.pi * d / CUTOFF) * inv
            for k in range(NUM_RADIAL)]


def _rbf_mix_rows(tiles, zero_row, w6, a):
    del zero_row
    stack = jnp.concatenate([t[a:a + 1] for t in tiles], axis=0)
    return jax.lax.dot_general(stack, w6, (((0,), (0,)), ((), ())),
                               preferred_element_type=f32)


def _tc_atom_pre_body(h_ref, wa_ref, wb_ref, o1_ref, o2_ref):
    h = h_ref[...]
    o1_ref[...] = jax.lax.dot_general(h, wa_ref[...], (((1,), (0,)), ((), ())),
                                      preferred_element_type=f32)
    o2_ref[...] = jax.lax.dot_general(h, wb_ref[...], (((1,), (0,)), ((), ())),
                                      preferred_element_type=f32)


def _tc_atom_pre(h, wa, wb):
    grid = 10
    rb = N_ATOMS // grid
    return pl.pallas_call(
        _tc_atom_pre_body,
        grid=(grid,),
        in_specs=[
            pl.BlockSpec((rb, EMB), lambda s: (s, 0)),
            pl.BlockSpec((EMB, EMB), lambda s: (0, 0)),
            pl.BlockSpec((EMB, EMB), lambda s: (0, 0)),
        ],
        out_specs=[
            pl.BlockSpec((rb, EMB), lambda s: (s, 0)),
            pl.BlockSpec((rb, EMB), lambda s: (s, 0)),
        ],
        out_shape=[
            jax.ShapeDtypeStruct((N_ATOMS, EMB), f32),
            jax.ShapeDtypeStruct((N_ATOMS, EMB), f32),
        ],
    )(h, wa, wb)


def _tc_edge_init_body(gi_ref, gj_ref, d_ref, wr_ref, b_ref,
                       ow_ref, iw_ref, x_ref, t_ref, y_ref):
    tiles = _rbf_tiles(d_ref[...])
    zero_row = jnp.zeros((1, 128), f32)
    for a in range(_DB):
        rows = pl.ds(a * 128, 128)
        rbfe = _rbf_mix_rows(tiles, zero_row, wr_ref[...], a)
        xv = _swish(gi_ref[rows, :] + gj_ref[rows, :] + rbfe + b_ref[...])
        x_ref[rows, :] = xv
        t_ref[rows, :] = xv * _rbf_mix_rows(tiles, zero_row, ow_ref[...], a)
        y_ref[rows, :] = xv * _rbf_mix_rows(tiles, zero_row, iw_ref[...], a)


def _tc_edge_init(gi, gj, d2d, wr, b, ow, iw):
    espec = pl.BlockSpec((_EB, EMB), lambda s: (s, 0))
    dspec = pl.BlockSpec((_DB, 128), lambda s: (s, 0))
    wspec = pl.BlockSpec((NUM_RADIAL, EMB), lambda s: (0, 0))
    return pl.pallas_call(
        _tc_edge_init_body,
        grid=(_EGRID,),
        in_specs=[espec, espec, dspec, wspec,
                  pl.BlockSpec((1, EMB), lambda s: (0, 0)), wspec, wspec],
        out_specs=[espec, espec, espec],
        out_shape=[jax.ShapeDtypeStruct((N_EDGES_PAD, EMB), f32)] * 3,
    )(gi, gj, d2d, wr, b, ow, iw)


_MA_R = 16


def _tc_mabf_body(c_ref, w_ref, o0_ref, o1_ref, o2_ref):
    c = c_ref[...]
    polys = [jnp.ones_like(c), c]
    for _ in range(NUM_ABF - 2):
        polys.append(2.0 * c * polys[-1] - polys[-2])
    zero = jnp.zeros((1, 128), f32)
    outs = (o0_ref, o1_ref, o2_ref)
    for a in range(_MA_R):
        stack = jnp.concatenate(
            [polys[k][a:a + 1] for k in range(NUM_ABF)] + [zero],
            axis=0)
        for i in range(NUM_BLOCKS):
            outs[i][pl.ds(a * 128, 128), :] = jax.lax.dot_general(
                stack, w_ref[i], (((0,), (0,)), ((), ())),
                preferred_element_type=f32)


def _tc_mabf(cos2d, wabf8):
    grid = (N_TRI_PAD // 128) // _MA_R
    ospec = pl.BlockSpec((_MA_R * 128, EMB), lambda s: (s, 0))
    return pl.pallas_call(
        _tc_mabf_body,
        grid=(grid,),
        in_specs=[
            pl.BlockSpec((_MA_R, 128), lambda s: (s, 0)),
            pl.BlockSpec((NUM_BLOCKS, 8, EMB), lambda s: (0, 0, 0)),
        ],
        out_specs=[ospec, ospec, ospec],
        out_shape=[jax.ShapeDtypeStruct((N_TRI_PAD, EMB), f32)] * 3,
    )(cos2d, wabf8)


def _tc_mlp_body(ax_ref, lx_ref, d_ref, w1_ref, b1_ref, w2_ref,
                 b2_ref, ow_ref, iw_ref, xn_ref, t_ref, y_ref):
    u = _swish(jax.lax.dot_general(ax_ref[...], w1_ref[...],
                                   (((1,), (0,)), ((), ())),
                                   preferred_element_type=f32) + b1_ref[...])
    xn = lx_ref[...] + _swish(
        jax.lax.dot_general(u, w2_ref[...], (((1,), (0,)), ((), ())),
                            preferred_element_type=f32) + b2_ref[...])
    xn_ref[...] = xn
    tiles = _rbf_tiles(d_ref[...])
    zero_row = jnp.zeros((1, 128), f32)
    for a in range(_DB):
        rows = pl.ds(a * 128, 128)
        t_ref[rows, :] = xn[a * 128:(a + 1) * 128, :] * _rbf_mix_rows(
            tiles, zero_row, ow_ref[...], a)
        if y_ref is not None:
            y_ref[rows, :] = xn[a * 128:(a + 1) * 128, :] * _rbf_mix_rows(
                tiles, zero_row, iw_ref[...], a)


def _tc_mlp(aggx, lastx, d2d, w1, b1, w2, b2, ow, iw, want_y):
    espec = pl.BlockSpec((_EB, EMB), lambda s: (s, 0))
    dspec = pl.BlockSpec((_DB, 128), lambda s: (s, 0))
    mspec = pl.BlockSpec((EMB, EMB), lambda s: (0, 0))
    bspec = pl.BlockSpec((1, EMB), lambda s: (0, 0))
    wspec = pl.BlockSpec((NUM_RADIAL, EMB), lambda s: (0, 0))
    if want_y:
        body = _tc_mlp_body
        out_specs = [espec, espec, espec]
        out_shape = [jax.ShapeDtypeStruct((N_EDGES_PAD, EMB), f32)] * 3
    else:
        def body(ax, lx, d_, w1_, b1_, w2_, b2_, ow_, iw_, xn_, t_):
            _tc_mlp_body(ax, lx, d_, w1_, b1_, w2_, b2_, ow_, iw_,
                         xn_, t_, None)
        out_specs = [espec, espec]
        out_shape = [jax.ShapeDtypeStruct((N_EDGES_PAD, EMB), f32)] * 2
    return pl.pallas_call(
        body,
        grid=(_EGRID,),
        in_specs=[espec, espec, dspec, mspec, bspec, mspec, bspec,
                  wspec, wspec],
        out_specs=out_specs,
        out_shape=out_shape,
    )(aggx, lastx, d2d, w1, b1, w2, b2, ow, iw)


def _tc_atom_final_body(a0_ref, a1_ref, a2_ref, a3_ref, h_ref,
                        ow1_ref, ow2_ref, sbw_ref, sbb_ref,
                        ro_ref, rs_ref):
    arefs = (a0_ref, a1_ref, a2_ref, a3_ref)
    ro = None
    for i in range(NUM_BLOCKS + 1):
        a = arefs[i][0] + arefs[i][1]
        u = _swish(jax.lax.dot_general(a, ow1_ref[i], (((1,), (0,)), ((), ())),
                                       preferred_element_type=f32))
        v = jax.lax.dot_general(u, ow2_ref[i], (((1,), (0,)), ((), ())),
                                preferred_element_type=f32)
        ro = v if ro is None else ro + v
    ro_ref[...] = ro
    rs = _swish(jax.lax.dot_general(h_ref[...], sbw_ref[0],
                                    (((1,), (0,)), ((), ())),
                                    preferred_element_type=f32) + sbb_ref[0])
    for i in range(NUM_BLOCKS):
        rs = rs + _swish(
            jax.lax.dot_general(rs, sbw_ref[i + 1], (((1,), (0,)), ((), ())),
                                preferred_element_type=f32) + sbb_ref[i + 1])
    rs_ref[...] = rs


def _tc_atom_final(a_list, h, ow1, ow2, sbw, sbb):
    grid = 10
    rb = N_ATOMS // grid
    aspec = pl.BlockSpec((NC, rb, EMB), lambda s: (0, s, 0))
    nb1 = NUM_BLOCKS + 1
    return pl.pallas_call(
        _tc_atom_final_body,
        grid=(grid,),
        in_specs=[aspec, aspec, aspec, aspec,
                  pl.BlockSpec((rb, EMB), lambda s: (s, 0)),
                  pl.BlockSpec((nb1, EMB, EMB), lambda s: (0, 0, 0)),
                  pl.BlockSpec((nb1, EMB, EMB), lambda s: (0, 0, 0)),
                  pl.BlockSpec((nb1, EMB, EMB), lambda s: (0, 0, 0)),
                  pl.BlockSpec((nb1, 1, EMB), lambda s: (0, 0, 0))],
        out_specs=[pl.BlockSpec((rb, EMB), lambda s: (s, 0)),
                   pl.BlockSpec((rb, EMB), lambda s: (s, 0))],
        out_shape=[jax.ShapeDtypeStruct((N_ATOMS, EMB), f32)] * 2,
    )(*a_list, h, ow1, ow2, sbw, sbb)


def _tc_mol_body(m_ref, cm_ref, cs_ref, o_ref):
    res = m_ref[0, 0:N_MOL, :] + m_ref[1, 0:N_MOL, :]
    sing = m_ref[0, 640:640 + N_MOL, :] + m_ref[1, 640:640 + N_MOL, :]
    o_ref[...] = cm_ref[0, 0] * res + cs_ref[0, 0] * sing


def _tc_mol(mo, cm, cs):
    return pl.pallas_call(
        _tc_mol_body,
        in_specs=[pl.BlockSpec((NC, _A2M_ROWS, EMB), lambda: (0, 0, 0)),
                  pl.BlockSpec((1, 1), lambda: (0, 0)),
                  pl.BlockSpec((1, 1), lambda: (0, 0))],
        out_specs=pl.BlockSpec((N_MOL, EMB), lambda: (0, 0)),
        out_shape=jax.ShapeDtypeStruct((N_MOL, EMB), f32),
        grid=(),
    )(mo, cm, cs)


def kernel(Z, R, batch_seg, idx_i, idx_j, idx_kj, idx_ji, cosine_ijk, params):
    p = params
    ep = N_EDGES_PAD - N_EDGES
    Zp = jnp.pad(Z.astype(i32), (0, N_ATOMS_PAD - N_ATOMS))
    r128 = jnp.pad(R.astype(f32), ((0, 0), (0, EMB - 3)))
    idx_i_g = jnp.pad(idx_i.astype(i32), (0, ep))
    idx_j_g = jnp.pad(idx_j.astype(i32), (0, ep))
    idx_i_s = jnp.pad(idx_i.astype(i32), (0, ep),
                      constant_values=N_ATOMS)
    tp = N_TRI_PAD - N_TRIPLETS
    idx_kj = jnp.pad(idx_kj.astype(i32), (0, tp))
    idx_ji = jnp.pad(idx_ji.astype(i32), (0, tp),
                     constant_values=1 << 29)
    seg_p = jnp.pad(batch_seg.astype(i32), (0, N_ATOMS_PAD - N_ATOMS),
                    constant_values=N_MOL)

    emb_b = p["emb_b"].reshape(1, EMB)
    wabf8 = jnp.pad(p["int_W_abf"], ((0, 0), (0, 8 - NUM_ABF), (0, 0)))
    cos2d = jnp.pad(cosine_ijk.astype(f32),
                    (0, N_TRI_PAD - N_TRIPLETS)).reshape(N_TRI_PAD // 128, 128)

    hp = _sc_h_gather(p["emb_table"], Zp)
    h = hp[:N_ATOMS]
    hw1, hw2 = _tc_atom_pre(h, p["emb_W"][:EMB], p["emb_W"][EMB:2 * EMB])
    gi, gj = _sc_edge_gather(hw1, hw2, idx_i_g, idx_j_g)
    d2d = _sc_edge_d(r128, idx_i_g, idx_j_g).reshape(N_EDGES_PAD // 128, 128)
    wr_x = p["emb_W_rbf"] @ p["emb_W"][2 * EMB:]
    x, t0, y = _tc_edge_init(gi, gj, d2d, wr_x, emb_b,
                             p["out_W_rbf"][0], p["int_W_rbf"][0])
    ma = _tc_mabf(cos2d, wabf8)

    a_list = [_sc_e2a(t0, idx_i_s)]
    last = x
    for i in range(NUM_BLOCKS):
        aggx = _sc_triplet(y, ma[i], idx_kj, idx_ji, last)
        want_y = i < NUM_BLOCKS - 1
        ow = p["out_W_rbf"][i + 1]
        iw = p["int_W_rbf"][i + 1] if want_y else p["int_W_rbf"][i]
        outs = _tc_mlp(aggx, last, d2d, p["int_W1"][i],
                       p["int_b1"][i].reshape(1, EMB), p["int_W2"][i],
                       p["int_b2"][i].reshape(1, EMB), ow, iw, want_y)
        if want_y:
            xn, t_next, y = outs
        else:
            xn, t_next = outs
        a_list.append(_sc_e2a(t_next, idx_i_s))
        last = xn

    ro, rs = _tc_atom_final(a_list, h, p["out_W1"], p["out_W2"], p["sb_W"],
                            p["sb_b"].reshape(NUM_BLOCKS + 1, 1, EMB))
    ro_p = jnp.pad(ro, ((0, N_ATOMS_PAD - N_ATOMS), (0, 0)))
    rs_p = jnp.pad(rs, ((0, N_ATOMS_PAD - N_ATOMS), (0, 0)))
    mo = _sc_a2m(ro_p, rs_p, seg_p)
    return _tc_mol(mo, p["coef_mp"].reshape(1, 1), p["coef_sg"].reshape(1, 1))


# speedup vs baseline: 1.8911x; 1.0092x over previous
import functools

import jax
import jax.numpy as jnp
import numpy as np
from jax import lax
from jax.experimental import pallas as pl
from jax.experimental.pallas import tpu as pltpu
from jax.experimental.pallas import tpu_sc as plsc

EMB = 128
NUM_BLOCKS = 3
NUM_RADIAL = 6
CUTOFF = 5.0
NUM_ABF = 7
N_ATOMS = 10000
N_EDGES = 160000
N_TRIPLETS = 320000
N_MOL = 512

N_ATOMS_PAD = 10240
N_EDGES_PAD = 163840
N_TRI_PAD = 327680

NC = 2
NS = 16
NW = NC * NS

f32 = jnp.float32
i32 = jnp.int32

_sc_mesh = plsc.VectorSubcoreMesh(core_axis_name="c", subcore_axis_name="s")


def _bc16(x):
    return jax.lax.broadcast_in_dim(x, (16,), ())


def _swish(x):
    return x * jax.nn.sigmoid(x)


def _sc_h_gather_body(tab_h, z_h, out_h, idx_v, rows_v, sem):
    wid = lax.axis_index("s") * NC + lax.axis_index("c")
    bpw = 512

    @pl.when(wid < 20)
    def _():
        base = wid * bpw
        pltpu.sync_copy(z_h.at[pl.ds(base, bpw)], idx_v)
        pltpu.async_copy(tab_h.at[idx_v], rows_v, sem).wait()
        pltpu.sync_copy(rows_v, out_h.at[pl.ds(base, bpw)])


def _sc_h_gather(tab, zp):
    k = pl.kernel(
        _sc_h_gather_body,
        out_type=jax.ShapeDtypeStruct((N_ATOMS_PAD, EMB), f32),
        mesh=_sc_mesh,
        compiler_params=pltpu.CompilerParams(needs_layout_passes=False),
        scratch_types=[
            pltpu.VMEM((512,), i32),
            pltpu.VMEM((512, EMB), f32),
            pltpu.SemaphoreType.DMA,
        ],
    )
    return k(tab, zp)


_EG_B = 128


def _sc_edge_gather_body(hw1_h, hw2_h, ii_h, jj_h, gi_h, gj_h,
                         ivA, jvA, baA, bbA, ivB, jvB, baB, bbB,
                         s0A, s1A, s0B, s1B):
    wid = lax.axis_index("s") * NC + lax.axis_index("c")
    bpw = N_EDGES_PAD // NW
    base = wid * bpw
    nb = bpw // _EG_B

    def fire(k, ivx, jvx, bax, bbx, s0x, s1x):
        off = base + k * _EG_B
        pltpu.sync_copy(ii_h.at[pl.ds(off, _EG_B)], ivx)
        pltpu.sync_copy(jj_h.at[pl.ds(off, _EG_B)], jvx)
        pltpu.async_copy(hw1_h.at[ivx], bax, s0x)
        pltpu.async_copy(hw2_h.at[jvx], bbx, s1x)

    def finish(k, bax, bbx, s0x, s1x):
        off = base + k * _EG_B
        pltpu.make_async_copy(hw1_h.at[pl.ds(0, _EG_B)], bax, s0x).wait()
        pltpu.make_async_copy(hw2_h.at[pl.ds(0, _EG_B)], bbx, s1x).wait()
        pltpu.sync_copy(bax, gi_h.at[pl.ds(off, _EG_B)])
        pltpu.sync_copy(bbx, gj_h.at[pl.ds(off, _EG_B)])

    fire(0, ivA, jvA, baA, bbA, s0A, s1A)

    def duo(g, _):
        k0 = g * 2
        fire(k0 + 1, ivB, jvB, baB, bbB, s0B, s1B)
        finish(k0, baA, bbA, s0A, s1A)

        @pl.when(k0 + 2 < nb)
        def _():
            fire(k0 + 2, ivA, jvA, baA, bbA, s0A, s1A)
        finish(k0 + 1, baB, bbB, s0B, s1B)
        return 0
    lax.fori_loop(0, nb // 2, duo, 0)


def _sc_edge_gather(hw1, hw2, idx_i, idx_j):
    k = pl.kernel(
        _sc_edge_gather_body,
        out_type=(
            jax.ShapeDtypeStruct((N_EDGES_PAD, EMB), f32),
            jax.ShapeDtypeStruct((N_EDGES_PAD, EMB), f32),
        ),
        mesh=_sc_mesh,
        compiler_params=pltpu.CompilerParams(needs_layout_passes=False),
        scratch_types=[
            pltpu.VMEM((_EG_B,), i32),
            pltpu.VMEM((_EG_B,), i32),
            pltpu.VMEM((_EG_B, EMB), f32),
            pltpu.VMEM((_EG_B, EMB), f32),
            pltpu.VMEM((_EG_B,), i32),
            pltpu.VMEM((_EG_B,), i32),
            pltpu.VMEM((_EG_B, EMB), f32),
            pltpu.VMEM((_EG_B, EMB), f32),
            pltpu.SemaphoreType.DMA,
            pltpu.SemaphoreType.DMA,
            pltpu.SemaphoreType.DMA,
            pltpu.SemaphoreType.DMA,
        ],
    )
    return k(hw1, hw2, idx_i, idx_j)


_D_B = 256


def _sc_edge_d_body(r128_h, ii_h, jj_h, d_h, iv, jv, rib, rjb, dbuf, s0, s1):
    wid = lax.axis_index("s") * NC + lax.axis_index("c")
    bpw = N_EDGES_PAD // NW
    base = wid * bpw

    def batch(k, _):
        off = k * _D_B
        pltpu.sync_copy(ii_h.at[pl.ds(base + off, _D_B)], iv)
        pltpu.sync_copy(jj_h.at[pl.ds(base + off, _D_B)], jv)
        c0 = pltpu.async_copy(r128_h.at[iv], rib, s0)
        c1 = pltpu.async_copy(r128_h.at[jv], rjb, s1)
        c0.wait(); c1.wait()

        def row(r, _2):
            lns = lax.broadcasted_iota(i32, (16,), 0)
            cm = jnp.where(lns < jnp.full((16,), 3, i32),
                           jnp.ones((16,), f32), jnp.zeros((16,), f32))
            l0 = lns == jnp.zeros((16,), i32)
            eps = jnp.where(l0, jnp.full((16,), 1e-12, f32),
                            jnp.zeros((16,), f32))
            va = rib[r, pl.ds(0, 16)] - rjb[r, pl.ds(0, 16)]
            sq = va * va * cm + eps
            s = jnp.sum(sq)
            plsc.store_scatter(dbuf, [_bc16(off + r)], _bc16(s), mask=l0)
            return 0
        lax.fori_loop(0, _D_B, row, 0)
        return 0

    lax.fori_loop(0, bpw // _D_B, batch, 0)
    pltpu.sync_copy(dbuf, d_h.at[pl.ds(base, bpw)])


def _sc_edge_d(r128, idx_i, idx_j):
    k = pl.kernel(
        _sc_edge_d_body,
        out_type=jax.ShapeDtypeStruct((N_EDGES_PAD,), f32),
        mesh=_sc_mesh,
        compiler_params=pltpu.CompilerParams(needs_layout_passes=False),
        scratch_types=[
            pltpu.VMEM((_D_B,), i32),
            pltpu.VMEM((_D_B,), i32),
            pltpu.VMEM((_D_B, EMB), f32),
            pltpu.VMEM((_D_B, EMB), f32),
            pltpu.VMEM((N_EDGES_PAD // NW,), f32),
            pltpu.SemaphoreType.DMA,
            pltpu.SemaphoreType.DMA,
        ],
    )
    return k(r128, idx_i, idx_j)


_CH = 10000
_NCHUNK = N_EDGES // _CH
_TSL = N_TRI_PAD // NS
_SB = 4096
_DR = 64
_CAP = 4224
_PKM = (1 << 14) - 1


def _sc_triplet_body(y_h, ma_h, kj_h, ji_h, lx_h, out_h,
                     ji_s, kj_s, pk_b, cp_b, cnb,
                     cdA, ckA, cpA, cdB, ckB, cpB,
                     ybA, mbA, ybB, mbB, acc, syA, smA, syB, smB, ssA, ssB):
    c = lax.axis_index("c")
    s = lax.axis_index("s")

    def fire(off, cdx, ckx, cpx, ybx, mbx, sy, sm, ss):
        pltpu.make_async_copy(ybx, acc.at[cdx], ss).wait()
        for v in range(_DR // 16):
            pk = pk_b[pl.ds(off + v * 16, 16)]
            cdx[pl.ds(v * 16, 16)] = pk & jnp.full((16,), _PKM, i32)
            ckx[pl.ds(v * 16, 16)] = lax.shift_right_logical(
                pk, jnp.full((16,), 14, i32))
            cpx[pl.ds(v * 16, 16)] = cp_b[pl.ds(off + v * 16, 16)]
        pltpu.async_copy(y_h.at[ckx], ybx, sy)
        pltpu.async_copy(ma_h.at[cpx], mbx, sm)

    def finish(cdx, ybx, mbx, sy, sm, ss):
        pltpu.make_async_copy(y_h.at[pl.ds(0, _DR)], ybx, sy).wait()
        pltpu.make_async_copy(ma_h.at[pl.ds(0, _DR)], mbx, sm).wait()

        def mulrow(r, _):
            for cc in range(EMB // 16):
                ybx[r, pl.ds(cc * 16, 16)] = (
                    ybx[r, pl.ds(cc * 16, 16)] * mbx[r, pl.ds(cc * 16, 16)])
            return 0
        lax.fori_loop(0, _DR, mulrow, 0)
        pltpu.async_copy(ybx, acc.at[cdx], ss, add=True)

    def prime(cdx, ybx, ss):
        for v in range(_DR // 16):
            cdx[pl.ds(v * 16, 16)] = jnp.full((16,), _CH, i32)
        pltpu.async_copy(ybx, acc.at[cdx], ss, add=True)

    for kc in range(_NCHUNK // NC):
        chunk = kc * NC + c
        lo = chunk * _CH

        @pl.when(s < 10)
        def _():
            pltpu.sync_copy(lx_h.at[pl.ds(lo + s * 1000, 1000)],
                            acc.at[pl.ds(s * 1000, 1000)])
        prime(cdA, ybA, ssA)
        prime(cdB, ybB, ssB)
        plsc.subcore_barrier()

        def scan_batch(b, cnt):
            tbase = s * _TSL + b * _SB
            pltpu.sync_copy(ji_h.at[pl.ds(tbase, _SB)], ji_s)
            pltpu.sync_copy(kj_h.at[pl.ds(tbase, _SB)], kj_s)

            def scan_quad(q, cnt2):
                lanes = lax.broadcasted_iota(i32, (16,), 0)
                parts = []
                for u in range(8):
                    voff = q * 128 + u * 16
                    jiv = ji_s[pl.ds(voff, 16)]
                    kjv = kj_s[pl.ds(voff, 16)]
                    lv = jiv - _bc16(lo)
                    m = ((lv >= jnp.zeros((16,), i32))
                         & (lv < jnp.full((16,), _CH, i32)))
                    mi = m.astype(i32)
                    csum = plsc.cumsum(mi)
                    pk = lv | lax.shift_left(kjv, jnp.full((16,), 14, i32))
                    posv = _bc16(tbase + q * 128 + u * 16) + lanes
                    parts.append((m, csum, pk, posv))
                for m, csum, pk, posv in parts:
                    tgt = _bc16(cnt2) + csum - jnp.ones((16,), i32)
                    plsc.store_scatter(pk_b, [tgt], pk, mask=m)
                    plsc.store_scatter(cp_b, [tgt], posv, mask=m)
                    cnt2 = cnt2 + csum[15]
                return cnt2
            cnt = lax.fori_loop(0, _SB // 128, scan_quad, cnt)

            nfull = cnt // _DR

            @pl.when(nfull > 0)
            def _():
                fire(0, cdA, ckA, cpA, ybA, mbA, syA, smA, ssA)

            def duo(g, _):
                f1 = g * 2 + 1

                @pl.when(f1 < nfull)
                def _():
                    fire(f1 * _DR, cdB, ckB, cpB, ybB, mbB, syB, smB, ssB)
                finish(cdA, ybA, mbA, syA, smA, ssA)

                @pl.when(f1 < nfull)
                def _():
                    @pl.when(f1 + 1 < nfull)
                    def _():
                        fire((f1 + 1) * _DR, cdA, ckA, cpA, ybA, mbA,
                             syA, smA, ssA)
                    finish(cdB, ybB, mbB, syB, smB, ssB)
                return 0
            lax.fori_loop(0, (nfull + 1) // 2, duo, 0)

            rem = cnt - nfull * _DR
            off0 = nfull * _DR
            vals = []
            for v in range(_DR // 16):
                vals.append((pk_b[pl.ds(off0 + v * 16, 16)],
                             cp_b[pl.ds(off0 + v * 16, 16)]))
            for v, (a, bb2) in enumerate(vals):
                pk_b[pl.ds(v * 16, 16)] = a
                cp_b[pl.ds(v * 16, 16)] = bb2
            return rem

        cnt = lax.fori_loop(0, _TSL // _SB, scan_batch, jnp.int32(0))

        for v in range(_DR // 16):
            lanes = lax.broadcasted_iota(i32, (16,), 0)
            l = _bc16(jnp.int32(v * 16)) + lanes
            good = l < _bc16(cnt)
            pk_b[pl.ds(v * 16, 16)] = jnp.where(
                good, pk_b[pl.ds(v * 16, 16)], jnp.full((16,), _CH, i32))
            cp_b[pl.ds(v * 16, 16)] = jnp.where(
                good, cp_b[pl.ds(v * 16, 16)], jnp.zeros((16,), i32))
        fire(0, cdA, ckA, cpA, ybA, mbA, syA, smA, ssA)
        finish(cdA, ybA, mbA, syA, smA, ssA)

        pltpu.make_async_copy(ybA, acc.at[cdA], ssA).wait()
        pltpu.make_async_copy(ybB, acc.at[cdB], ssB).wait()
        plsc.subcore_barrier()

        @pl.when(s < 10)
        def _():
            pltpu.sync_copy(acc.at[pl.ds(s * 1000, 1000)],
                            out_h.at[pl.ds(lo + s * 1000, 1000)])
        plsc.subcore_barrier()


def _sc_triplet(y, ma, kj, ji, lx):
    k = pl.kernel(
        _sc_triplet_body,
        out_type=jax.ShapeDtypeStruct((N_EDGES_PAD, EMB), f32),
        mesh=_sc_mesh,
        compiler_params=pltpu.CompilerParams(needs_layout_passes=False),
        scratch_types=[
            pltpu.VMEM((_SB,), i32),
            pltpu.VMEM((_SB,), i32),
            pltpu.VMEM((_CAP,), i32),
            pltpu.VMEM((_CAP,), i32),
            pltpu.VMEM((16,), i32),
            pltpu.VMEM((_DR,), i32),
            pltpu.VMEM((_DR,), i32),
            pltpu.VMEM((_DR,), i32),
            pltpu.VMEM((_DR,), i32),
            pltpu.VMEM((_DR,), i32),
            pltpu.VMEM((_DR,), i32),
            pltpu.VMEM((_DR, EMB), f32),
            pltpu.VMEM((_DR, EMB), f32),
            pltpu.VMEM((_DR, EMB), f32),
            pltpu.VMEM((_DR, EMB), f32),
            pltpu.VMEM_SHARED((_CH + 8, EMB), f32),
            pltpu.SemaphoreType.DMA,
            pltpu.SemaphoreType.DMA,
            pltpu.SemaphoreType.DMA,
            pltpu.SemaphoreType.DMA,
            pltpu.SemaphoreType.DMA,
            pltpu.SemaphoreType.DMA,
        ],
    )
    return k(y, ma, kj, ji, lx)


_E2A_B = 128
_E2A_ACC = 10240


def _sc_e2a_body(t_h, ii_h, out_h, ivA, tbA, ivB, tbB, acc, sA, sB):
    c = lax.axis_index("c")
    s = lax.axis_index("s")
    span = _E2A_ACC // NS

    def zrow(r, _):
        for cc in range(EMB // 16):
            tbA[r, pl.ds(cc * 16, 16)] = jnp.zeros((16,), f32)
        return 0
    lax.fori_loop(0, _E2A_B, zrow, 0)
    for z in range(span // _E2A_B):
        pltpu.sync_copy(tbA, acc.at[pl.ds(s * span + z * _E2A_B, _E2A_B)])
    plsc.subcore_barrier()

    bpw = N_EDGES_PAD // NW
    base = (c * NS + s) * bpw
    nb = bpw // _E2A_B

    def fire(k, ivx, tbx, sx):
        off = base + k * _E2A_B
        pltpu.sync_copy(ii_h.at[pl.ds(off, _E2A_B)], ivx)
        pltpu.async_copy(t_h.at[pl.ds(off, _E2A_B)], tbx, sx)

    def finish(ivx, tbx, sx):
        pltpu.make_async_copy(t_h.at[pl.ds(0, _E2A_B)], tbx, sx).wait()
        pltpu.sync_copy(tbx, acc.at[ivx], add=True)

    fire(0, ivA, tbA, sA)

    def duo(g, _):
        k0 = g * 2
        fire(k0 + 1, ivB, tbB, sB)
        finish(ivA, tbA, sA)

        @pl.when(k0 + 2 < nb)
        def _():
            fire(k0 + 2, ivA, tbA, sA)
        finish(ivB, tbB, sB)
        return 0
    lax.fori_loop(0, nb // 2, duo, 0)

    plsc.subcore_barrier()

    @pl.when(s < 10)
    def _():
        pltpu.sync_copy(acc.at[pl.ds(s * 1000, 1000)],
                        out_h.at[c, pl.ds(s * 1000, 1000)])


def _sc_e2a(t, idx_i):
    k = pl.kernel(
        _sc_e2a_body,
        out_type=jax.ShapeDtypeStruct((NC, N_ATOMS, EMB), f32),
        mesh=_sc_mesh,
        compiler_params=pltpu.CompilerParams(needs_layout_passes=False),
        scratch_types=[
            pltpu.VMEM((_E2A_B,), i32),
            pltpu.VMEM((_E2A_B, EMB), f32),
            pltpu.VMEM((_E2A_B,), i32),
            pltpu.VMEM((_E2A_B, EMB), f32),
            pltpu.VMEM_SHARED((_E2A_ACC, EMB), f32),
            pltpu.SemaphoreType.DMA,
            pltpu.SemaphoreType.DMA,
        ],
    )
    return k(t, idx_i)


_A2M_ROWS = 1280


def _sc_a2m_body(ro_h, rs_h, seg_h, out_h, idx_v, idx2_v, buf, zbuf, acc, sem):
    c = lax.axis_index("c")
    s = lax.axis_index("s")
    wid = s * NC + c
    span = _A2M_ROWS // NS

    def zrow(r, _):
        for cc in range(EMB // 16):
            zbuf[r, pl.ds(cc * 16, 16)] = jnp.zeros((16,), f32)
        return 0
    lax.fori_loop(0, span, zrow, 0)
    pltpu.sync_copy(zbuf, acc.at[pl.ds(s * span, span)])
    plsc.subcore_barrier()

    bpw = 512

    @pl.when(wid < 20)
    def _():
        base = wid * bpw
        pltpu.sync_copy(seg_h.at[pl.ds(base, bpw)], idx_v)
        for v in range(bpw // 16):
            idx2_v[pl.ds(v * 16, 16)] = (idx_v[pl.ds(v * 16, 16)]
                                         + jnp.full((16,), 640, i32))
        pltpu.sync_copy(ro_h.at[pl.ds(base, bpw)], buf)
        pltpu.sync_copy(buf, acc.at[idx_v], add=True)
        pltpu.sync_copy(rs_h.at[pl.ds(base, bpw)], buf)
        pltpu.sync_copy(buf, acc.at[idx2_v], add=True)

    plsc.subcore_barrier()
    pltpu.sync_copy(acc.at[pl.ds(s * span, span)],
                    out_h.at[c, pl.ds(s * span, span)])


def _sc_a2m(ro, rs, seg):
    k = pl.kernel(
        _sc_a2m_body,
        out_type=jax.ShapeDtypeStruct((NC, _A2M_ROWS, EMB), f32),
        mesh=_sc_mesh,
        compiler_params=pltpu.CompilerParams(needs_layout_passes=False),
        scratch_types=[
            pltpu.VMEM((512,), i32),
            pltpu.VMEM((512,), i32),
            pltpu.VMEM((512, EMB), f32),
            pltpu.VMEM((_A2M_ROWS // NS, EMB), f32),
            pltpu.VMEM_SHARED((_A2M_ROWS, EMB), f32),
            pltpu.SemaphoreType.DMA,
        ],
    )
    return k(ro, rs, seg)


_EB = 2048
_EGRID = N_EDGES_PAD // _EB
_DB = _EB // 128


def _rbf_tiles(d2):
    d = jnp.sqrt(d2)
    scale = np.sqrt(2.0 / CUTOFF).astype(np.float32)
    inv = 1.0 / (d + 1e-6)
    return [scale * jnp.sin((k + 1) * np.pi * d / CUTOFF) * inv
            for k in range(NUM_RADIAL)]


def _rbf_mix_rows(tiles, zero_row, w6, a):
    del zero_row
    stack = jnp.concatenate([t[a:a + 1] for t in tiles], axis=0)
    return jax.lax.dot_general(stack, w6, (((0,), (0,)), ((), ())),
                               preferred_element_type=f32)


def _tc_atom_pre_body(h_ref, wa_ref, wb_ref, o1_ref, o2_ref):
    h = h_ref[...]
    o1_ref[...] = jax.lax.dot_general(h, wa_ref[...], (((1,), (0,)), ((), ())),
                                      preferred_element_type=f32)
    o2_ref[...] = jax.lax.dot_general(h, wb_ref[...], (((1,), (0,)), ((), ())),
                                      preferred_element_type=f32)


def _tc_atom_pre(h, wa, wb):
    grid = 10
    rb = N_ATOMS // grid
    return pl.pallas_call(
        _tc_atom_pre_body,
        grid=(grid,),
        in_specs=[
            pl.BlockSpec((rb, EMB), lambda s: (s, 0)),
            pl.BlockSpec((EMB, EMB), lambda s: (0, 0)),
            pl.BlockSpec((EMB, EMB), lambda s: (0, 0)),
        ],
        out_specs=[
            pl.BlockSpec((rb, EMB), lambda s: (s, 0)),
            pl.BlockSpec((rb, EMB), lambda s: (s, 0)),
        ],
        out_shape=[
            jax.ShapeDtypeStruct((N_ATOMS, EMB), f32),
            jax.ShapeDtypeStruct((N_ATOMS, EMB), f32),
        ],
    )(h, wa, wb)


def _tc_edge_init_body(gi_ref, gj_ref, d_ref, wr_ref, b_ref,
                       ow_ref, iw_ref, x_ref, t_ref, y_ref):
    tiles = _rbf_tiles(d_ref[...])
    zero_row = jnp.zeros((1, 128), f32)
    for a in range(_DB):
        rows = pl.ds(a * 128, 128)
        rbfe = _rbf_mix_rows(tiles, zero_row, wr_ref[...], a)
        xv = _swish(gi_ref[rows, :] + gj_ref[rows, :] + rbfe + b_ref[...])
        x_ref[rows, :] = xv
        t_ref[rows, :] = xv * _rbf_mix_rows(tiles, zero_row, ow_ref[...], a)
        y_ref[rows, :] = xv * _rbf_mix_rows(tiles, zero_row, iw_ref[...], a)


def _tc_edge_init(gi, gj, d2d, wr, b, ow, iw):
    espec = pl.BlockSpec((_EB, EMB), lambda s: (s, 0))
    dspec = pl.BlockSpec((_DB, 128), lambda s: (s, 0))
    wspec = pl.BlockSpec((NUM_RADIAL, EMB), lambda s: (0, 0))
    return pl.pallas_call(
        _tc_edge_init_body,
        grid=(_EGRID,),
        in_specs=[espec, espec, dspec, wspec,
                  pl.BlockSpec((1, EMB), lambda s: (0, 0)), wspec, wspec],
        out_specs=[espec, espec, espec],
        out_shape=[jax.ShapeDtypeStruct((N_EDGES_PAD, EMB), f32)] * 3,
    )(gi, gj, d2d, wr, b, ow, iw)


_MA_R = 16


def _tc_mabf_body(c_ref, w_ref, o0_ref, o1_ref, o2_ref):
    c = c_ref[...]
    polys = [jnp.ones_like(c), c]
    for _ in range(NUM_ABF - 2):
        polys.append(2.0 * c * polys[-1] - polys[-2])
    zero = jnp.zeros((1, 128), f32)
    outs = (o0_ref, o1_ref, o2_ref)
    for a in range(_MA_R):
        stack = jnp.concatenate(
            [polys[k][a:a + 1] for k in range(NUM_ABF)] + [zero],
            axis=0)
        for i in range(NUM_BLOCKS):
            outs[i][pl.ds(a * 128, 128), :] = jax.lax.dot_general(
                stack, w_ref[i], (((0,), (0,)), ((), ())),
                preferred_element_type=f32)


def _tc_mabf(cos2d, wabf8):
    grid = (N_TRI_PAD // 128) // _MA_R
    ospec = pl.BlockSpec((_MA_R * 128, EMB), lambda s: (s, 0))
    return pl.pallas_call(
        _tc_mabf_body,
        grid=(grid,),
        in_specs=[
            pl.BlockSpec((_MA_R, 128), lambda s: (s, 0)),
            pl.BlockSpec((NUM_BLOCKS, 8, EMB), lambda s: (0, 0, 0)),
        ],
        out_specs=[ospec, ospec, ospec],
        out_shape=[jax.ShapeDtypeStruct((N_TRI_PAD, EMB), f32)] * 3,
    )(cos2d, wabf8)


def _tc_mlp_body(ax_ref, lx_ref, d_ref, w1_ref, b1_ref, w2_ref,
                 b2_ref, ow_ref, iw_ref, xn_ref, t_ref, y_ref):
    u = _swish(jax.lax.dot_general(ax_ref[...], w1_ref[...],
                                   (((1,), (0,)), ((), ())),
                                   preferred_element_type=f32) + b1_ref[...])
    xn = lx_ref[...] + _swish(
        jax.lax.dot_general(u, w2_ref[...], (((1,), (0,)), ((), ())),
                            preferred_element_type=f32) + b2_ref[...])
    xn_ref[...] = xn
    tiles = _rbf_tiles(d_ref[...])
    zero_row = jnp.zeros((1, 128), f32)
    for a in range(_DB):
        rows = pl.ds(a * 128, 128)
        t_ref[rows, :] = xn[a * 128:(a + 1) * 128, :] * _rbf_mix_rows(
            tiles, zero_row, ow_ref[...], a)
        if y_ref is not None:
            y_ref[rows, :] = xn[a * 128:(a + 1) * 128, :] * _rbf_mix_rows(
                tiles, zero_row, iw_ref[...], a)


def _tc_mlp(aggx, lastx, d2d, w1, b1, w2, b2, ow, iw, want_y):
    espec = pl.BlockSpec((_EB, EMB), lambda s: (s, 0))
    dspec = pl.BlockSpec((_DB, 128), lambda s: (s, 0))
    mspec = pl.BlockSpec((EMB, EMB), lambda s: (0, 0))
    bspec = pl.BlockSpec((1, EMB), lambda s: (0, 0))
    wspec = pl.BlockSpec((NUM_RADIAL, EMB), lambda s: (0, 0))
    if want_y:
        body = _tc_mlp_body
        out_specs = [espec, espec, espec]
        out_shape = [jax.ShapeDtypeStruct((N_EDGES_PAD, EMB), f32)] * 3
    else:
        def body(ax, lx, d_, w1_, b1_, w2_, b2_, ow_, iw_, xn_, t_):
            _tc_mlp_body(ax, lx, d_, w1_, b1_, w2_, b2_, ow_, iw_,
                         xn_, t_, None)
        out_specs = [espec, espec]
        out_shape = [jax.ShapeDtypeStruct((N_EDGES_PAD, EMB), f32)] * 2
    return pl.pallas_call(
        body,
        grid=(_EGRID,),
        in_specs=[espec, espec, dspec, mspec, bspec, mspec, bspec,
                  wspec, wspec],
        out_specs=out_specs,
        out_shape=out_shape,
    )(aggx, lastx, d2d, w1, b1, w2, b2, ow, iw)


def _tc_atom_final_body(a0_ref, a1_ref, a2_ref, a3_ref, h_ref,
                        ow1_ref, ow2_ref, sbw_ref, sbb_ref,
                        ro_ref, rs_ref):
    arefs = (a0_ref, a1_ref, a2_ref, a3_ref)
    ro = None
    for i in range(NUM_BLOCKS + 1):
        a = arefs[i][0] + arefs[i][1]
        u = _swish(jax.lax.dot_general(a, ow1_ref[i], (((1,), (0,)), ((), ())),
                                       preferred_element_type=f32))
        v = jax.lax.dot_general(u, ow2_ref[i], (((1,), (0,)), ((), ())),
                                preferred_element_type=f32)
        ro = v if ro is None else ro + v
    ro_ref[...] = ro
    rs = _swish(jax.lax.dot_general(h_ref[...], sbw_ref[0],
                                    (((1,), (0,)), ((), ())),
                                    preferred_element_type=f32) + sbb_ref[0])
    for i in range(NUM_BLOCKS):
        rs = rs + _swish(
            jax.lax.dot_general(rs, sbw_ref[i + 1], (((1,), (0,)), ((), ())),
                                preferred_element_type=f32) + sbb_ref[i + 1])
    rs_ref[...] = rs


def _tc_atom_final(a_list, h, ow1, ow2, sbw, sbb):
    grid = 10
    rb = N_ATOMS // grid
    aspec = pl.BlockSpec((NC, rb, EMB), lambda s: (0, s, 0))
    nb1 = NUM_BLOCKS + 1
    return pl.pallas_call(
        _tc_atom_final_body,
        grid=(grid,),
        in_specs=[aspec, aspec, aspec, aspec,
                  pl.BlockSpec((rb, EMB), lambda s: (s, 0)),
                  pl.BlockSpec((nb1, EMB, EMB), lambda s: (0, 0, 0)),
                  pl.BlockSpec((nb1, EMB, EMB), lambda s: (0, 0, 0)),
                  pl.BlockSpec((nb1, EMB, EMB), lambda s: (0, 0, 0)),
                  pl.BlockSpec((nb1, 1, EMB), lambda s: (0, 0, 0))],
        out_specs=[pl.BlockSpec((rb, EMB), lambda s: (s, 0)),
                   pl.BlockSpec((rb, EMB), lambda s: (s, 0))],
        out_shape=[jax.ShapeDtypeStruct((N_ATOMS, EMB), f32)] * 2,
    )(*a_list, h, ow1, ow2, sbw, sbb)


def _tc_mol_body(m_ref, cm_ref, cs_ref, o_ref):
    res = m_ref[0, 0:N_MOL, :] + m_ref[1, 0:N_MOL, :]
    sing = m_ref[0, 640:640 + N_MOL, :] + m_ref[1, 640:640 + N_MOL, :]
    o_ref[...] = cm_ref[0, 0] * res + cs_ref[0, 0] * sing


def _tc_mol(mo, cm, cs):
    return pl.pallas_call(
        _tc_mol_body,
        in_specs=[pl.BlockSpec((NC, _A2M_ROWS, EMB), lambda: (0, 0, 0)),
                  pl.BlockSpec((1, 1), lambda: (0, 0)),
                  pl.BlockSpec((1, 1), lambda: (0, 0))],
        out_specs=pl.BlockSpec((N_MOL, EMB), lambda: (0, 0)),
        out_shape=jax.ShapeDtypeStruct((N_MOL, EMB), f32),
        grid=(),
    )(mo, cm, cs)


def kernel(Z, R, batch_seg, idx_i, idx_j, idx_kj, idx_ji, cosine_ijk, params):
    p = params
    ep = N_EDGES_PAD - N_EDGES
    Zp = jnp.pad(Z.astype(i32), (0, N_ATOMS_PAD - N_ATOMS))
    r128 = jnp.pad(R.astype(f32), ((0, 0), (0, EMB - 3)))
    idx_i_g = jnp.pad(idx_i.astype(i32), (0, ep))
    idx_j_g = jnp.pad(idx_j.astype(i32), (0, ep))
    idx_i_s = jnp.pad(idx_i.astype(i32), (0, ep),
                      constant_values=N_ATOMS)
    tp = N_TRI_PAD - N_TRIPLETS
    idx_kj = jnp.pad(idx_kj.astype(i32), (0, tp))
    idx_ji = jnp.pad(idx_ji.astype(i32), (0, tp),
                     constant_values=1 << 29)
    seg_p = jnp.pad(batch_seg.astype(i32), (0, N_ATOMS_PAD - N_ATOMS),
                    constant_values=N_MOL)

    emb_b = p["emb_b"].reshape(1, EMB)
    wabf8 = jnp.pad(p["int_W_abf"], ((0, 0), (0, 8 - NUM_ABF), (0, 0)))
    cos2d = jnp.pad(cosine_ijk.astype(f32),
                    (0, N_TRI_PAD - N_TRIPLETS)).reshape(N_TRI_PAD // 128, 128)

    hp = _sc_h_gather(p["emb_table"], Zp)
    h = hp[:N_ATOMS]
    hw1, hw2 = _tc_atom_pre(h, p["emb_W"][:EMB], p["emb_W"][EMB:2 * EMB])
    gi, gj = _sc_edge_gather(hw1, hw2, idx_i_g, idx_j_g)
    d2d = _sc_edge_d(r128, idx_i_g, idx_j_g).reshape(N_EDGES_PAD // 128, 128)
    wr_x = p["emb_W_rbf"] @ p["emb_W"][2 * EMB:]
    x, t0, y = _tc_edge_init(gi, gj, d2d, wr_x, emb_b,
                             p["out_W_rbf"][0], p["int_W_rbf"][0])
    ma = _tc_mabf(cos2d, wabf8)

    a_list = [_sc_e2a(t0, idx_i_s)]
    last = x
    for i in range(NUM_BLOCKS):
        aggx = _sc_triplet(y, ma[i], idx_kj, idx_ji, last)
        want_y = i < NUM_BLOCKS - 1
        ow = p["out_W_rbf"][i + 1]
        iw = p["int_W_rbf"][i + 1] if want_y else p["int_W_rbf"][i]
        outs = _tc_mlp(aggx, last, d2d, p["int_W1"][i],
                       p["int_b1"][i].reshape(1, EMB), p["int_W2"][i],
                       p["int_b2"][i].reshape(1, EMB), ow, iw, want_y)
        if want_y:
            xn, t_next, y = outs
        else:
            xn, t_next = outs
        a_list.append(_sc_e2a(t_next, idx_i_s))
        last = xn

    ro, rs = _tc_atom_final(a_list, h, p["out_W1"], p["out_W2"], p["sb_W"],
                            p["sb_b"].reshape(NUM_BLOCKS + 1, 1, EMB))
    ro_p = jnp.pad(ro, ((0, N_ATOMS_PAD - N_ATOMS), (0, 0)))
    rs_p = jnp.pad(rs, ((0, N_ATOMS_PAD - N_ATOMS), (0, 0)))
    mo = _sc_a2m(ro_p, rs_p, seg_p)
    return _tc_mol(mo, p["coef_mp"].reshape(1, 1), p["coef_sg"].reshape(1, 1))
